# Initial kernel scaffold; baseline (speedup 1.0000x reference)
import jax
import jax.numpy as jnp
from jax.experimental import pallas as pl


def kernel(hidden_states, start_pos, position_embeddings, attention_mask, wq, wk, wv, wo, q_norm_w, k_norm_w, ln1_w, ln2_w, gate_w, w_gate, w_up, w_down):
    raise NotImplementedError("write your pallas kernel here")


# speedup vs baseline: 1.0888x; 1.0888x over previous
import functools
import jax
import jax.numpy as jnp
from jax import lax
from jax.experimental import pallas as pl
from jax.experimental.pallas import tpu as pltpu
from jax.experimental.pallas import tpu_sc as plsc

B, S, D = 1, 2048, 2048
H, KV, DH = 16, 4, 128
E, TK, F = 8, 2, 768
EPS = 1e-6
SCALE = DH ** -0.5

BS1 = 256
BQ = 256
BK = 256
BS3 = 256
BM5 = 256


def _rms_in(x, w):
    v = jnp.mean(jnp.square(x), axis=-1, keepdims=True)
    return w * (x * lax.rsqrt(v + EPS))


def _rot_half(x):
    h = x.shape[-1] // 2
    return jnp.concatenate([-x[:, h:], x[:, :h]], axis=-1)


DK1 = 512
KC1 = D // DK1
QKVW = (H + 2 * KV) * DH


def _k1_body(x_ref, w_ref, cos_ref, sin_ref,
             ln1_ref, qn_ref, kn_ref, q_ref, k_ref, v_ref, acc_ref):
    kc = pl.program_id(1)
    x = x_ref[...]
    v = jnp.mean(jnp.square(x), axis=-1, keepdims=True)
    scale = lax.rsqrt(v + EPS)
    xs = x_ref[:, pl.ds(kc * DK1, DK1)]
    ws = ln1_ref[:, pl.ds(kc * DK1, DK1)]
    h = xs * scale * ws
    part = jnp.dot(h, w_ref[...], preferred_element_type=jnp.float32)

    @pl.when(kc == 0)
    def _():
        acc_ref[...] = part

    @pl.when(kc != 0)
    def _():
        acc_ref[...] = acc_ref[...] + part

    @pl.when(kc == KC1 - 1)
    def _():
        qkv = acc_ref[...]
        cos = cos_ref[...]
        sin = sin_ref[...]
        for hh in range(H):
            qh = _rms_in(qkv[:, hh * DH:(hh + 1) * DH], qn_ref[...])
            q_ref[hh, :, :] = qh * cos + _rot_half(qh) * sin
        for g in range(KV):
            kh = _rms_in(qkv[:, (H + g) * DH:(H + g + 1) * DH], kn_ref[...])
            k_ref[g, :, :] = kh * cos + _rot_half(kh) * sin
            v_ref[g, :, :] = qkv[:, (H + KV + g) * DH:(H + KV + g + 1) * DH]


def _k1(x, wqkv, cos, sin, ln1_w, qn_w, kn_w):
    n = S // BS1
    return pl.pallas_call(
        _k1_body,
        grid=(n, KC1),
        in_specs=[
            pl.BlockSpec((BS1, D), lambda i, kc: (i, 0)),
            pl.BlockSpec((DK1, QKVW), lambda i, kc: (kc, 0)),
            pl.BlockSpec((BS1, DH), lambda i, kc: (i, 0)),
            pl.BlockSpec((BS1, DH), lambda i, kc: (i, 0)),
            pl.BlockSpec((1, D), lambda i, kc: (0, 0)),
            pl.BlockSpec((1, DH), lambda i, kc: (0, 0)),
            pl.BlockSpec((1, DH), lambda i, kc: (0, 0)),
        ],
        out_specs=[
            pl.BlockSpec((H, BS1, DH), lambda i, kc: (0, i, 0)),
            pl.BlockSpec((KV, BS1, DH), lambda i, kc: (0, i, 0)),
            pl.BlockSpec((KV, BS1, DH), lambda i, kc: (0, i, 0)),
        ],
        out_shape=[
            jax.ShapeDtypeStruct((H, S, DH), jnp.float32),
            jax.ShapeDtypeStruct((KV, S, DH), jnp.float32),
            jax.ShapeDtypeStruct((KV, S, DH), jnp.float32),
        ],
        scratch_shapes=[pltpu.VMEM((BS1, QKVW), jnp.float32)],
    )(x, wqkv, cos, sin, ln1_w, qn_w, kn_w)


def _k2_body(q_ref, k_ref, v_ref, o_ref):
    iq = pl.program_id(1)
    q = q_ref[0] * SCALE
    row = iq * BQ + lax.broadcasted_iota(jnp.int32, (BQ, BK), 0)

    def step(j, carry):
        m, l, acc = carry
        kj = k_ref[0, pl.ds(j * BK, BK), :]
        vj = v_ref[0, pl.ds(j * BK, BK), :]
        s = lax.dot_general(q, kj, (((1,), (1,)), ((), ())),
                            preferred_element_type=jnp.float32)
        col = j * BK + lax.broadcasted_iota(jnp.int32, (BQ, BK), 1)
        s = jnp.where(col <= row, s, -1e30)
        mnew = jnp.maximum(m, jnp.max(s, axis=-1, keepdims=True))
        p = jnp.exp(s - mnew)
        corr = jnp.exp(m - mnew)
        l = l * corr + jnp.sum(p, axis=-1, keepdims=True)
        acc = acc * corr + jnp.dot(p, vj, preferred_element_type=jnp.float32)
        return m * 0 + mnew, l, acc

    m0 = jnp.full((BQ, 1), -1e30, jnp.float32)
    l0 = jnp.zeros((BQ, 1), jnp.float32)
    a0 = jnp.zeros((BQ, DH), jnp.float32)
    m, l, acc = lax.fori_loop(0, iq + 1, step, (m0, l0, a0))
    o_ref[0] = acc / l


def _k2(q, k, v):
    nq = S // BQ
    return pl.pallas_call(
        _k2_body,
        grid=(H, nq),
        in_specs=[
            pl.BlockSpec((1, BQ, DH), lambda h, i: (h, i, 0)),
            pl.BlockSpec((1, S, DH), lambda h, i: (h // (H // KV), 0, 0)),
            pl.BlockSpec((1, S, DH), lambda h, i: (h // (H // KV), 0, 0)),
        ],
        out_specs=pl.BlockSpec((1, BQ, DH), lambda h, i: (h, i, 0)),
        out_shape=jax.ShapeDtypeStruct((H, S, DH), jnp.float32),
    )(q, k, v)


HG3 = 4
KC3 = H // HG3


def _k3_body(x_ref, o_ref, wo_ref, ln2_ref, gw_ref,
             x2_ref, h2_ref, ti_ref, tw_ref, cnt_ref, acc_ref):
    kc = pl.program_id(1)
    s = jnp.dot(o_ref[0], wo_ref[pl.ds(0, DH), :],
                preferred_element_type=jnp.float32)
    for hh in range(1, HG3):
        s = s + jnp.dot(o_ref[hh], wo_ref[pl.ds(hh * DH, DH), :],
                        preferred_element_type=jnp.float32)

    @pl.when(kc == 0)
    def _():
        acc_ref[...] = x_ref[...] + s

    @pl.when(kc != 0)
    def _():
        acc_ref[...] = acc_ref[...] + s

    @pl.when(kc == KC3 - 1)
    def _():
        _k3_tail(acc_ref, ln2_ref, gw_ref, x2_ref, h2_ref,
                 ti_ref, tw_ref, cnt_ref)


def _k3_tail(acc_ref, ln2_ref, gw_ref, x2_ref, h2_ref, ti_ref, tw_ref, cnt_ref):
    acc = acc_ref[...]
    x2_ref[...] = acc
    h2 = _rms_in(acc, ln2_ref[...])
    h2_ref[...] = h2
    logits = jnp.dot(h2, gw_ref[...], preferred_element_type=jnp.float32)
    iot = lax.broadcasted_iota(jnp.int32, logits.shape, 1)
    m1 = jnp.max(logits, axis=-1, keepdims=True)
    i1 = jnp.min(jnp.where(logits == m1, iot, E), axis=-1, keepdims=True)
    l2m = jnp.where(iot == i1, -jnp.inf, logits)
    m2 = jnp.max(l2m, axis=-1, keepdims=True)
    i2 = jnp.min(jnp.where(l2m == m2, iot, E), axis=-1, keepdims=True)
    w1 = 1.0 / (1.0 + jnp.exp(m2 - m1))
    w2 = 1.0 - w1
    ti_ref[...] = jnp.concatenate([i1, i2], axis=1)
    tw_ref[...] = jnp.concatenate([w1, w2], axis=1)
    iot64 = lax.broadcasted_iota(jnp.int32, (BS3, 64), 1)
    oh = (iot64 == i1).astype(jnp.int32) + (iot64 == i2).astype(jnp.int32)
    cnt_ref[...] = jnp.sum(oh, axis=0, keepdims=True).reshape(1, 1, 64)


def _k3(x, o, wo, ln2_w, gate_w):
    n = S // BS3
    return pl.pallas_call(
        _k3_body,
        grid=(n, KC3),
        in_specs=[
            pl.BlockSpec((BS3, D), lambda i, kc: (i, 0)),
            pl.BlockSpec((HG3, BS3, DH), lambda i, kc: (kc, i, 0)),
            pl.BlockSpec((HG3 * DH, D), lambda i, kc: (kc, 0)),
            pl.BlockSpec((1, D), lambda i, kc: (0, 0)),
            pl.BlockSpec((D, E), lambda i, kc: (0, 0)),
        ],
        out_specs=[
            pl.BlockSpec((BS3, D), lambda i, kc: (i, 0)),
            pl.BlockSpec((BS3, D), lambda i, kc: (i, 0)),
            pl.BlockSpec((BS3, TK), lambda i, kc: (i, 0)),
            pl.BlockSpec((BS3, TK), lambda i, kc: (i, 0)),
            pl.BlockSpec((1, 1, 64), lambda i, kc: (i, 0, 0)),
        ],
        out_shape=[
            jax.ShapeDtypeStruct((S, D), jnp.float32),
            jax.ShapeDtypeStruct((S, D), jnp.float32),
            jax.ShapeDtypeStruct((S, TK), jnp.int32),
            jax.ShapeDtypeStruct((S, TK), jnp.float32),
            jax.ShapeDtypeStruct((S // BS3, 1, 64), jnp.int32),
        ],
        scratch_shapes=[pltpu.VMEM((BS3, D), jnp.float32)],
    )(x, o, wo, ln2_w, gate_w)


BLK = 128
NQ4 = 4
QTOK = S // NQ4
QPAIR = QTOK * TK
P = 5632
NB = P // BLK
NBP = 48
L = 16


def _extract(vec, lane):
    return jnp.sum(jnp.where(lax.iota(jnp.int32, L) == lane, vec, 0))


def _k4_kernel(ti_hbm, tw_hbm, h2_hbm, counts_hbm,
               hd_hbm, ws_hbm, pos_hbm, bexp_hbm,
               tiv, twv, posbuf, cmp_tok, cmp_w, cvm, zb, rows, bev, sem):
    c = lax.axis_index("c")
    s = lax.axis_index("s")
    e = s % E
    ql = s // E
    q = 2 * c + ql

    pltpu.sync_copy(counts_hbm.at[:], cvm)
    qoff = pl.multiple_of(q * QPAIR, QPAIR)
    pltpu.sync_copy(ti_hbm.at[pl.ds(qoff, QPAIR)], tiv)
    pltpu.sync_copy(tw_hbm.at[pl.ds(qoff, QPAIR)], twv)

    crow = [cvm[blk, 0, pl.ds(0, L)] for blk in range(2 * NQ4)]
    cq = {}
    cnt = {}
    for ee in range(E):
        for qq in range(NQ4):
            cval = _extract(crow[2 * qq], ee) + _extract(crow[2 * qq + 1], ee)
            cnt[(ee, qq)] = cval
            cq[(ee, qq)] = ((cval + L - 1) // L) * L
    base = {}
    endblk = []
    running = jnp.int32(0)
    for ee in range(E):
        tot = jnp.int32(0)
        for qq in range(NQ4):
            base[(ee, qq)] = running * BLK + tot
            tot = tot + cq[(ee, qq)]
        running = running + (tot + BLK - 1) // BLK
        endblk.append(running)

    my_base = jnp.int32(0)
    my_cnt = jnp.int32(0)
    my_cq = jnp.int32(0)
    for ee in range(E):
        for qq in range(NQ4):
            sel = jnp.logical_and(e == ee, q == qq)
            my_base = jnp.where(sel, base[(ee, qq)], my_base)
            my_cnt = jnp.where(sel, cnt[(ee, qq)], my_cnt)
            my_cq = jnp.where(sel, cq[(ee, qq)], my_cq)

    @pl.when(jnp.logical_and(c == 0, s == 0))
    def _():
        for ch in range(NBP // L):
            bv = lax.iota(jnp.int32, L) + ch * L
            acc = jnp.zeros((L,), jnp.int32)
            for ee in range(E - 1):
                acc = acc + (bv >= endblk[ee]).astype(jnp.int32)
            bev[pl.ds(ch * L, L)] = acc
        pltpu.sync_copy(bev, bexp_hbm.at[:])

    zv = jnp.zeros((L,), jnp.int32)
    for i in range(QPAIR // L):
        zb[pl.ds(i * L, L)] = zv
        cmp_tok[pl.ds(i * L, L)] = zv

    def pass2(i, cnt2):
        chunk = tiv[pl.ds(i * L, L)]
        mask = chunk == e
        mi = mask.astype(jnp.int32)
        within = plsc.cumsum(mi) - 1
        posv = my_base + cnt2 + within
        posbuf[pl.ds(i * L, L)] = jnp.where(mask, posv, 0)
        loc = cnt2 + within
        tok = (q * QPAIR + i * L + lax.iota(jnp.int32, L)) // TK
        plsc.store_scatter(cmp_tok, [loc], tok, mask=mask)
        plsc.store_scatter(cmp_w, [loc], twv[pl.ds(i * L, L)], mask=mask)
        return cnt2 + jnp.sum(mi)

    lax.fori_loop(0, QPAIR // L, pass2, jnp.int32(0))

    def gstep(j, carry):
        idxsl = cmp_tok.at[pl.ds(j * L, L)]
        pltpu.async_copy(h2_hbm.at[idxsl], rows, sem).wait()
        roff = pl.multiple_of(my_base + j * L, L)
        pltpu.sync_copy(rows, hd_hbm.at[pl.ds(roff, L)])
        pltpu.sync_copy(cmp_w.at[pl.ds(j * L, L)],
                        ws_hbm.at[pl.ds(roff, L)])
        return carry

    lax.fori_loop(0, my_cq // L, gstep, jnp.int32(0))

    pltpu.sync_copy(posbuf, pos_hbm.at[e, pl.ds(qoff, QPAIR)])


def _k4(ti_flat, tw_flat, h2, counts):
    mesh = plsc.VectorSubcoreMesh(core_axis_name="c", subcore_axis_name="s")
    kfn = pl.kernel(
        _k4_kernel,
        mesh=mesh,
        out_type=[
            jax.ShapeDtypeStruct((P, D), jnp.float32),
            jax.ShapeDtypeStruct((P,), jnp.float32),
            jax.ShapeDtypeStruct((E, S * TK), jnp.int32),
            jax.ShapeDtypeStruct((NBP,), jnp.int32),
        ],
        compiler_params=pltpu.CompilerParams(needs_layout_passes=False),
        scratch_types=[
            pltpu.VMEM((QPAIR,), jnp.int32),
            pltpu.VMEM((QPAIR,), jnp.float32),
            pltpu.VMEM((QPAIR,), jnp.int32),
            pltpu.VMEM((QPAIR,), jnp.int32),
            pltpu.VMEM((QPAIR,), jnp.float32),
            pltpu.VMEM((2 * NQ4, 1, 64), jnp.int32),
            pltpu.VMEM((QPAIR,), jnp.int32),
            pltpu.VMEM((L, D), jnp.float32),
            pltpu.VMEM((NBP,), jnp.int32),
            pltpu.SemaphoreType.DMA,
        ],
    )
    return kfn(ti_flat, tw_flat, h2, counts)


def _k5_body(bexp_ref, hd_ref, ws_ref, wg_ref, wu_ref, wd_ref, y_ref):
    hd = hd_ref[...]
    g = jnp.dot(hd, wg_ref[0], preferred_element_type=jnp.float32)
    u = jnp.dot(hd, wu_ref[0], preferred_element_type=jnp.float32)
    hh = (g * (1.0 / (1.0 + jnp.exp(-g)))) * u
    y = jnp.dot(hh, wd_ref[0], preferred_element_type=jnp.float32)
    y_ref[...] = y * ws_ref[...]


def _k5(hd, ws, bexp, w_gate, w_up, w_down):
    grid_spec = pltpu.PrefetchScalarGridSpec(
        num_scalar_prefetch=1,
        grid=(NB,),
        in_specs=[
            pl.BlockSpec((BLK, D), lambda b, be: (b, 0)),
            pl.BlockSpec((BLK, 1), lambda b, be: (b, 0)),
            pl.BlockSpec((1, D, F), lambda b, be: (be[b], 0, 0)),
            pl.BlockSpec((1, D, F), lambda b, be: (be[b], 0, 0)),
            pl.BlockSpec((1, F, D), lambda b, be: (be[b], 0, 0)),
        ],
        out_specs=pl.BlockSpec((BLK, D), lambda b, be: (b, 0)),
    )
    return pl.pallas_call(
        _k5_body,
        grid_spec=grid_spec,
        out_shape=jax.ShapeDtypeStruct((P, D), jnp.float32),
    )(bexp, hd, ws.reshape(P, 1), w_gate, w_up, w_down)


TPT = S // 32
CH6 = 8


def _k6_kernel(y_hbm, pos_hbm, x2_hbm, out_hbm, pidx, pparts, ybuf, xv, ov, sem):
    wid = lax.axis_index("c") * 16 + lax.axis_index("s")
    t0 = pl.multiple_of(wid * TPT, TPT)
    poff = pl.multiple_of(t0 * TK, TPT * TK)
    pltpu.sync_copy(pos_hbm.at[:, pl.ds(poff, TPT * TK)], pparts)
    npc = (TPT * TK) // L

    def sum_parts(i, carry):
        acc = pparts[0, pl.ds(i * L, L)]
        for ee in range(1, E):
            acc = acc + pparts[ee, pl.ds(i * L, L)]
        pidx[pl.ds(i * L, L)] = acc
        return carry

    lax.fori_loop(0, npc, sum_parts, jnp.int32(0))
    for ch in range(TPT // CH6):
        idxsl = pidx.at[pl.ds(ch * CH6 * TK, L)]
        pltpu.async_copy(y_hbm.at[idxsl], ybuf, sem).wait()
        pltpu.sync_copy(x2_hbm.at[pl.ds(pl.multiple_of(t0 + ch * CH6, CH6), CH6)], xv)

        def body(j, carry):
            sl = pl.ds(j * L, L)
            for tt in range(CH6):
                ov[tt, sl] = xv[tt, sl] + ybuf[2 * tt, sl] + ybuf[2 * tt + 1, sl]
            return carry

        lax.fori_loop(0, D // L, body, jnp.int32(0))
        pltpu.sync_copy(ov, out_hbm.at[pl.ds(pl.multiple_of(t0 + ch * CH6, CH6), CH6)])


def _k6(y, pos, x2):
    mesh = plsc.VectorSubcoreMesh(core_axis_name="c", subcore_axis_name="s")
    kfn = pl.kernel(
        _k6_kernel,
        mesh=mesh,
        out_type=jax.ShapeDtypeStruct((S, D), jnp.float32),
        compiler_params=pltpu.CompilerParams(needs_layout_passes=False),
        scratch_types=[
            pltpu.VMEM((TPT * TK,), jnp.int32),
            pltpu.VMEM((E, TPT * TK), jnp.int32),
            pltpu.VMEM((L, D), jnp.float32),
            pltpu.VMEM((CH6, D), jnp.float32),
            pltpu.VMEM((CH6, D), jnp.float32),
            pltpu.SemaphoreType.DMA,
        ],
    )
    return kfn(y, pos, x2)


def kernel(hidden_states, start_pos, position_embeddings, attention_mask,
           wq, wk, wv, wo, q_norm_w, k_norm_w, ln1_w, ln2_w,
           gate_w, w_gate, w_up, w_down):
    x = hidden_states.reshape(S, D)
    cos = position_embeddings[0]
    sin = position_embeddings[1]
    wqkv = jnp.concatenate([wq, wk, wv], axis=1)
    q, k, v = _k1(x, wqkv, cos, sin,
                  ln1_w.reshape(1, D), q_norm_w.reshape(1, DH),
                  k_norm_w.reshape(1, DH))
    o = _k2(q, k, v)
    x2, h2, ti, tw, counts = _k3(x, o, wo, ln2_w.reshape(1, D), gate_w)
    hd, ws, pos, bexp = _k4(ti.reshape(S * TK), tw.reshape(S * TK),
                            h2, counts)
    y = _k5(hd, ws, bexp, w_gate, w_up, w_down)
    out = _k6(y, pos, x2)
    return out.reshape(B, S, D)


# speedup vs baseline: 1.0924x; 1.0033x over previous
import functools
import jax
import jax.numpy as jnp
from jax import lax
from jax.experimental import pallas as pl
from jax.experimental.pallas import tpu as pltpu
from jax.experimental.pallas import tpu_sc as plsc

B, S, D = 1, 2048, 2048
H, KV, DH = 16, 4, 128
E, TK, F = 8, 2, 768
EPS = 1e-6
SCALE = DH ** -0.5

BS1 = 256
BQ = 256
BK = 256
BS3 = 256
BM5 = 256


def _rms_in(x, w):
    v = jnp.mean(jnp.square(x), axis=-1, keepdims=True)
    return w * (x * lax.rsqrt(v + EPS))


def _rot_half(x):
    h = x.shape[-1] // 2
    return jnp.concatenate([-x[:, h:], x[:, :h]], axis=-1)


DK1 = 512
KC1 = D // DK1
QKVW = (H + 2 * KV) * DH


def _k1_body(x_ref, w_ref, cos_ref, sin_ref,
             ln1_ref, qn_ref, kn_ref, q_ref, k_ref, v_ref, acc_ref):
    kc = pl.program_id(1)
    x = x_ref[...]
    v = jnp.mean(jnp.square(x), axis=-1, keepdims=True)
    scale = lax.rsqrt(v + EPS)
    xs = x_ref[:, pl.ds(kc * DK1, DK1)]
    ws = ln1_ref[:, pl.ds(kc * DK1, DK1)]
    h = xs * scale * ws
    part = jnp.dot(h, w_ref[...], preferred_element_type=jnp.float32)

    @pl.when(kc == 0)
    def _():
        acc_ref[...] = part

    @pl.when(kc != 0)
    def _():
        acc_ref[...] = acc_ref[...] + part

    @pl.when(kc == KC1 - 1)
    def _():
        qkv = acc_ref[...]
        cos = cos_ref[...]
        sin = sin_ref[...]
        for hh in range(H):
            qh = _rms_in(qkv[:, hh * DH:(hh + 1) * DH], qn_ref[...])
            q_ref[hh, :, :] = qh * cos + _rot_half(qh) * sin
        for g in range(KV):
            kh = _rms_in(qkv[:, (H + g) * DH:(H + g + 1) * DH], kn_ref[...])
            k_ref[g, :, :] = kh * cos + _rot_half(kh) * sin
            v_ref[g, :, :] = qkv[:, (H + KV + g) * DH:(H + KV + g + 1) * DH]


def _k1(x, wqkv, cos, sin, ln1_w, qn_w, kn_w):
    n = S // BS1
    return pl.pallas_call(
        _k1_body,
        grid=(n, KC1),
        in_specs=[
            pl.BlockSpec((BS1, D), lambda i, kc: (i, 0)),
            pl.BlockSpec((DK1, QKVW), lambda i, kc: (kc, 0)),
            pl.BlockSpec((BS1, DH), lambda i, kc: (i, 0)),
            pl.BlockSpec((BS1, DH), lambda i, kc: (i, 0)),
            pl.BlockSpec((1, D), lambda i, kc: (0, 0)),
            pl.BlockSpec((1, DH), lambda i, kc: (0, 0)),
            pl.BlockSpec((1, DH), lambda i, kc: (0, 0)),
        ],
        out_specs=[
            pl.BlockSpec((H, BS1, DH), lambda i, kc: (0, i, 0)),
            pl.BlockSpec((KV, BS1, DH), lambda i, kc: (0, i, 0)),
            pl.BlockSpec((KV, BS1, DH), lambda i, kc: (0, i, 0)),
        ],
        out_shape=[
            jax.ShapeDtypeStruct((H, S, DH), jnp.float32),
            jax.ShapeDtypeStruct((KV, S, DH), jnp.float32),
            jax.ShapeDtypeStruct((KV, S, DH), jnp.float32),
        ],
        scratch_shapes=[pltpu.VMEM((BS1, QKVW), jnp.float32)],
    )(x, wqkv, cos, sin, ln1_w, qn_w, kn_w)


def _k2_body(q_ref, k_ref, v_ref, o_ref):
    iq = pl.program_id(1)
    q = q_ref[0] * SCALE
    row = iq * BQ + lax.broadcasted_iota(jnp.int32, (BQ, BK), 0)

    def step(j, carry):
        m, l, acc = carry
        kj = k_ref[0, pl.ds(j * BK, BK), :]
        vj = v_ref[0, pl.ds(j * BK, BK), :]
        s = lax.dot_general(q, kj, (((1,), (1,)), ((), ())),
                            preferred_element_type=jnp.float32)
        col = j * BK + lax.broadcasted_iota(jnp.int32, (BQ, BK), 1)
        s = jnp.where(col <= row, s, -1e30)
        mnew = jnp.maximum(m, jnp.max(s, axis=-1, keepdims=True))
        p = jnp.exp(s - mnew)
        corr = jnp.exp(m - mnew)
        l = l * corr + jnp.sum(p, axis=-1, keepdims=True)
        acc = acc * corr + jnp.dot(p, vj, preferred_element_type=jnp.float32)
        return m * 0 + mnew, l, acc

    m0 = jnp.full((BQ, 1), -1e30, jnp.float32)
    l0 = jnp.zeros((BQ, 1), jnp.float32)
    a0 = jnp.zeros((BQ, DH), jnp.float32)
    m, l, acc = lax.fori_loop(0, iq + 1, step, (m0, l0, a0))
    o_ref[0] = acc / l


def _k2(q, k, v):
    nq = S // BQ
    return pl.pallas_call(
        _k2_body,
        grid=(H, nq),
        in_specs=[
            pl.BlockSpec((1, BQ, DH), lambda h, i: (h, i, 0)),
            pl.BlockSpec((1, S, DH), lambda h, i: (h // (H // KV), 0, 0)),
            pl.BlockSpec((1, S, DH), lambda h, i: (h // (H // KV), 0, 0)),
        ],
        out_specs=pl.BlockSpec((1, BQ, DH), lambda h, i: (h, i, 0)),
        out_shape=jax.ShapeDtypeStruct((H, S, DH), jnp.float32),
    )(q, k, v)


HG3 = 4
KC3 = H // HG3


def _k3_body(x_ref, o_ref, wo_ref, ln2_ref, gw_ref,
             x2_ref, h2_ref, ti_ref, tw_ref, cnt_ref, acc_ref):
    kc = pl.program_id(1)
    s = jnp.dot(o_ref[0], wo_ref[pl.ds(0, DH), :],
                preferred_element_type=jnp.float32)
    for hh in range(1, HG3):
        s = s + jnp.dot(o_ref[hh], wo_ref[pl.ds(hh * DH, DH), :],
                        preferred_element_type=jnp.float32)

    @pl.when(kc == 0)
    def _():
        acc_ref[...] = x_ref[...] + s

    @pl.when(kc != 0)
    def _():
        acc_ref[...] = acc_ref[...] + s

    @pl.when(kc == KC3 - 1)
    def _():
        _k3_tail(acc_ref, ln2_ref, gw_ref, x2_ref, h2_ref,
                 ti_ref, tw_ref, cnt_ref)


def _k3_tail(acc_ref, ln2_ref, gw_ref, x2_ref, h2_ref, ti_ref, tw_ref, cnt_ref):
    acc = acc_ref[...]
    x2_ref[...] = acc
    h2 = _rms_in(acc, ln2_ref[...])
    h2_ref[...] = h2
    logits = jnp.dot(h2, gw_ref[...], preferred_element_type=jnp.float32)
    iot = lax.broadcasted_iota(jnp.int32, logits.shape, 1)
    m1 = jnp.max(logits, axis=-1, keepdims=True)
    i1 = jnp.min(jnp.where(logits == m1, iot, E), axis=-1, keepdims=True)
    l2m = jnp.where(iot == i1, -jnp.inf, logits)
    m2 = jnp.max(l2m, axis=-1, keepdims=True)
    i2 = jnp.min(jnp.where(l2m == m2, iot, E), axis=-1, keepdims=True)
    w1 = 1.0 / (1.0 + jnp.exp(m2 - m1))
    w2 = 1.0 - w1
    ti_ref[...] = jnp.concatenate([i1, i2], axis=1)
    tw_ref[...] = jnp.concatenate([w1, w2], axis=1)
    iot64 = lax.broadcasted_iota(jnp.int32, (BS3, 64), 1)
    oh = (iot64 == i1).astype(jnp.int32) + (iot64 == i2).astype(jnp.int32)
    cnt_ref[...] = jnp.sum(oh, axis=0, keepdims=True).reshape(1, 1, 64)


def _k3(x, o, wo, ln2_w, gate_w):
    n = S // BS3
    return pl.pallas_call(
        _k3_body,
        grid=(n, KC3),
        in_specs=[
            pl.BlockSpec((BS3, D), lambda i, kc: (i, 0)),
            pl.BlockSpec((HG3, BS3, DH), lambda i, kc: (kc, i, 0)),
            pl.BlockSpec((HG3 * DH, D), lambda i, kc: (kc, 0)),
            pl.BlockSpec((1, D), lambda i, kc: (0, 0)),
            pl.BlockSpec((D, E), lambda i, kc: (0, 0)),
        ],
        out_specs=[
            pl.BlockSpec((BS3, D), lambda i, kc: (i, 0)),
            pl.BlockSpec((BS3, D), lambda i, kc: (i, 0)),
            pl.BlockSpec((BS3, TK), lambda i, kc: (i, 0)),
            pl.BlockSpec((BS3, TK), lambda i, kc: (i, 0)),
            pl.BlockSpec((1, 1, 64), lambda i, kc: (i, 0, 0)),
        ],
        out_shape=[
            jax.ShapeDtypeStruct((S, D), jnp.float32),
            jax.ShapeDtypeStruct((S, D), jnp.float32),
            jax.ShapeDtypeStruct((S, TK), jnp.int32),
            jax.ShapeDtypeStruct((S, TK), jnp.float32),
            jax.ShapeDtypeStruct((S // BS3, 1, 64), jnp.int32),
        ],
        scratch_shapes=[pltpu.VMEM((BS3, D), jnp.float32)],
    )(x, o, wo, ln2_w, gate_w)


BLK = 256
NQ4 = 4
QTOK = S // NQ4
QPAIR = QTOK * TK
P = 6656
NB = P // BLK
NBP = 32
L = 16


def _extract(vec, lane):
    return jnp.sum(jnp.where(lax.iota(jnp.int32, L) == lane, vec, 0))


def _k4_kernel(ti_hbm, tw_hbm, h2_hbm, counts_hbm,
               hd_hbm, ws_hbm, pos_hbm, bexp_hbm,
               tiv, twv, posbuf, cmp_tok, cmp_w, cvm, zb, rows, bev, sem):
    c = lax.axis_index("c")
    s = lax.axis_index("s")
    e = s % E
    ql = s // E
    q = 2 * c + ql

    pltpu.sync_copy(counts_hbm.at[:], cvm)
    qoff = pl.multiple_of(q * QPAIR, QPAIR)
    pltpu.sync_copy(ti_hbm.at[pl.ds(qoff, QPAIR)], tiv)
    pltpu.sync_copy(tw_hbm.at[pl.ds(qoff, QPAIR)], twv)

    crow = [cvm[blk, 0, pl.ds(0, L)] for blk in range(2 * NQ4)]
    cq = {}
    cnt = {}
    for ee in range(E):
        for qq in range(NQ4):
            cval = _extract(crow[2 * qq], ee) + _extract(crow[2 * qq + 1], ee)
            cnt[(ee, qq)] = cval
            cq[(ee, qq)] = ((cval + L - 1) // L) * L
    base = {}
    endblk = []
    running = jnp.int32(0)
    for ee in range(E):
        tot = jnp.int32(0)
        for qq in range(NQ4):
            base[(ee, qq)] = running * BLK + tot
            tot = tot + cq[(ee, qq)]
        running = running + (tot + BLK - 1) // BLK
        endblk.append(running)

    my_base = jnp.int32(0)
    my_cnt = jnp.int32(0)
    my_cq = jnp.int32(0)
    for ee in range(E):
        for qq in range(NQ4):
            sel = jnp.logical_and(e == ee, q == qq)
            my_base = jnp.where(sel, base[(ee, qq)], my_base)
            my_cnt = jnp.where(sel, cnt[(ee, qq)], my_cnt)
            my_cq = jnp.where(sel, cq[(ee, qq)], my_cq)

    @pl.when(jnp.logical_and(c == 0, s == 0))
    def _():
        for ch in range(NBP // L):
            bv = lax.iota(jnp.int32, L) + ch * L
            acc = jnp.zeros((L,), jnp.int32)
            for ee in range(E - 1):
                acc = acc + (bv >= endblk[ee]).astype(jnp.int32)
            bev[pl.ds(ch * L, L)] = acc
        pltpu.sync_copy(bev, bexp_hbm.at[:])

    zv = jnp.zeros((L,), jnp.int32)
    for i in range(QPAIR // L):
        zb[pl.ds(i * L, L)] = zv
        cmp_tok[pl.ds(i * L, L)] = zv

    def pass2(i, cnt2):
        chunk = tiv[pl.ds(i * L, L)]
        mask = chunk == e
        mi = mask.astype(jnp.int32)
        within = plsc.cumsum(mi) - 1
        posv = my_base + cnt2 + within
        posbuf[pl.ds(i * L, L)] = jnp.where(mask, posv, 0)
        loc = cnt2 + within
        tok = (q * QPAIR + i * L + lax.iota(jnp.int32, L)) // TK
        plsc.store_scatter(cmp_tok, [loc], tok, mask=mask)
        plsc.store_scatter(cmp_w, [loc], twv[pl.ds(i * L, L)], mask=mask)
        return cnt2 + jnp.sum(mi)

    lax.fori_loop(0, QPAIR // L, pass2, jnp.int32(0))

    def gstep(j, carry):
        idxsl = cmp_tok.at[pl.ds(j * L, L)]
        pltpu.async_copy(h2_hbm.at[idxsl], rows, sem).wait()
        roff = pl.multiple_of(my_base + j * L, L)
        pltpu.sync_copy(rows, hd_hbm.at[pl.ds(roff, L)])
        pltpu.sync_copy(cmp_w.at[pl.ds(j * L, L)],
                        ws_hbm.at[pl.ds(roff, L)])
        return carry

    lax.fori_loop(0, my_cq // L, gstep, jnp.int32(0))

    pltpu.sync_copy(posbuf, pos_hbm.at[e, pl.ds(qoff, QPAIR)])


def _k4(ti_flat, tw_flat, h2, counts):
    mesh = plsc.VectorSubcoreMesh(core_axis_name="c", subcore_axis_name="s")
    kfn = pl.kernel(
        _k4_kernel,
        mesh=mesh,
        out_type=[
            jax.ShapeDtypeStruct((P, D), jnp.float32),
            jax.ShapeDtypeStruct((P,), jnp.float32),
            jax.ShapeDtypeStruct((E, S * TK), jnp.int32),
            jax.ShapeDtypeStruct((NBP,), jnp.int32),
        ],
        compiler_params=pltpu.CompilerParams(needs_layout_passes=False),
        scratch_types=[
            pltpu.VMEM((QPAIR,), jnp.int32),
            pltpu.VMEM((QPAIR,), jnp.float32),
            pltpu.VMEM((QPAIR,), jnp.int32),
            pltpu.VMEM((QPAIR,), jnp.int32),
            pltpu.VMEM((QPAIR,), jnp.float32),
            pltpu.VMEM((2 * NQ4, 1, 64), jnp.int32),
            pltpu.VMEM((QPAIR,), jnp.int32),
            pltpu.VMEM((L, D), jnp.float32),
            pltpu.VMEM((NBP,), jnp.int32),
            pltpu.SemaphoreType.DMA,
        ],
    )
    return kfn(ti_flat, tw_flat, h2, counts)


def _k5_body(bexp_ref, hd_ref, ws_ref, wg_ref, wu_ref, wd_ref, y_ref):
    hd = hd_ref[...]
    g = jnp.dot(hd, wg_ref[0], preferred_element_type=jnp.float32)
    u = jnp.dot(hd, wu_ref[0], preferred_element_type=jnp.float32)
    hh = (g * (1.0 / (1.0 + jnp.exp(-g)))) * u
    y = jnp.dot(hh, wd_ref[0], preferred_element_type=jnp.float32)
    y_ref[...] = y * ws_ref[...]


def _k5(hd, ws, bexp, w_gate, w_up, w_down):
    grid_spec = pltpu.PrefetchScalarGridSpec(
        num_scalar_prefetch=1,
        grid=(NB,),
        in_specs=[
            pl.BlockSpec((BLK, D), lambda b, be: (b, 0)),
            pl.BlockSpec((BLK, 1), lambda b, be: (b, 0)),
            pl.BlockSpec((1, D, F), lambda b, be: (be[b], 0, 0)),
            pl.BlockSpec((1, D, F), lambda b, be: (be[b], 0, 0)),
            pl.BlockSpec((1, F, D), lambda b, be: (be[b], 0, 0)),
        ],
        out_specs=pl.BlockSpec((BLK, D), lambda b, be: (b, 0)),
    )
    return pl.pallas_call(
        _k5_body,
        grid_spec=grid_spec,
        out_shape=jax.ShapeDtypeStruct((P, D), jnp.float32),
    )(bexp, hd, ws.reshape(P, 1), w_gate, w_up, w_down)


TPT = S // 32
CH6 = 8


def _k6_kernel(y_hbm, pos_hbm, x2_hbm, out_hbm, pidx, pparts, ybuf, xv, ov, sem):
    wid = lax.axis_index("c") * 16 + lax.axis_index("s")
    t0 = pl.multiple_of(wid * TPT, TPT)
    poff = pl.multiple_of(t0 * TK, TPT * TK)
    pltpu.sync_copy(pos_hbm.at[:, pl.ds(poff, TPT * TK)], pparts)
    npc = (TPT * TK) // L

    def sum_parts(i, carry):
        acc = pparts[0, pl.ds(i * L, L)]
        for ee in range(1, E):
            acc = acc + pparts[ee, pl.ds(i * L, L)]
        pidx[pl.ds(i * L, L)] = acc
        return carry

    lax.fori_loop(0, npc, sum_parts, jnp.int32(0))
    for ch in range(TPT // CH6):
        idxsl = pidx.at[pl.ds(ch * CH6 * TK, L)]
        pltpu.async_copy(y_hbm.at[idxsl], ybuf, sem).wait()
        pltpu.sync_copy(x2_hbm.at[pl.ds(pl.multiple_of(t0 + ch * CH6, CH6), CH6)], xv)

        def body(j, carry):
            sl = pl.ds(j * L, L)
            for tt in range(CH6):
                ov[tt, sl] = xv[tt, sl] + ybuf[2 * tt, sl] + ybuf[2 * tt + 1, sl]
            return carry

        lax.fori_loop(0, D // L, body, jnp.int32(0))
        pltpu.sync_copy(ov, out_hbm.at[pl.ds(pl.multiple_of(t0 + ch * CH6, CH6), CH6)])


def _k6(y, pos, x2):
    mesh = plsc.VectorSubcoreMesh(core_axis_name="c", subcore_axis_name="s")
    kfn = pl.kernel(
        _k6_kernel,
        mesh=mesh,
        out_type=jax.ShapeDtypeStruct((S, D), jnp.float32),
        compiler_params=pltpu.CompilerParams(needs_layout_passes=False),
        scratch_types=[
            pltpu.VMEM((TPT * TK,), jnp.int32),
            pltpu.VMEM((E, TPT * TK), jnp.int32),
            pltpu.VMEM((L, D), jnp.float32),
            pltpu.VMEM((CH6, D), jnp.float32),
            pltpu.VMEM((CH6, D), jnp.float32),
            pltpu.SemaphoreType.DMA,
        ],
    )
    return kfn(y, pos, x2)


def kernel(hidden_states, start_pos, position_embeddings, attention_mask,
           wq, wk, wv, wo, q_norm_w, k_norm_w, ln1_w, ln2_w,
           gate_w, w_gate, w_up, w_down):
    x = hidden_states.reshape(S, D)
    cos = position_embeddings[0]
    sin = position_embeddings[1]
    wqkv = jnp.concatenate([wq, wk, wv], axis=1)
    q, k, v = _k1(x, wqkv, cos, sin,
                  ln1_w.reshape(1, D), q_norm_w.reshape(1, DH),
                  k_norm_w.reshape(1, DH))
    o = _k2(q, k, v)
    x2, h2, ti, tw, counts = _k3(x, o, wo, ln2_w.reshape(1, D), gate_w)
    hd, ws, pos, bexp = _k4(ti.reshape(S * TK), tw.reshape(S * TK),
                            h2, counts)
    y = _k5(hd, ws, bexp, w_gate, w_up, w_down)
    out = _k6(y, pos, x2)
    return out.reshape(B, S, D)


# speedup vs baseline: 1.2516x; 1.1457x over previous
import functools
import jax
import jax.numpy as jnp
from jax import lax
from jax.experimental import pallas as pl
from jax.experimental.pallas import tpu as pltpu
from jax.experimental.pallas import tpu_sc as plsc

B, S, D = 1, 2048, 2048
H, KV, DH = 16, 4, 128
E, TK, F = 8, 2, 768
EPS = 1e-6
SCALE = DH ** -0.5

BS1 = 256
BQ = 256
BK = 512
BS3 = 256
BM5 = 256


def _rms_in(x, w):
    v = jnp.mean(jnp.square(x), axis=-1, keepdims=True)
    return w * (x * lax.rsqrt(v + EPS))


def _rot_half(x):
    h = x.shape[-1] // 2
    return jnp.concatenate([-x[:, h:], x[:, :h]], axis=-1)


DK1 = 512
KC1 = D // DK1
QKVW = (H + 2 * KV) * DH


def _k1_body(x_ref, w_ref, cos_ref, sin_ref,
             ln1_ref, qn_ref, kn_ref, q_ref, k_ref, v_ref, acc_ref):
    kc = pl.program_id(1)
    x = x_ref[...]
    v = jnp.mean(jnp.square(x), axis=-1, keepdims=True)
    scale = lax.rsqrt(v + EPS)
    xs = x_ref[:, pl.ds(kc * DK1, DK1)]
    ws = ln1_ref[:, pl.ds(kc * DK1, DK1)]
    h = xs * scale * ws
    part = jnp.dot(h, w_ref[...], preferred_element_type=jnp.float32)

    @pl.when(kc == 0)
    def _():
        acc_ref[...] = part

    @pl.when(kc != 0)
    def _():
        acc_ref[...] = acc_ref[...] + part

    @pl.when(kc == KC1 - 1)
    def _():
        qkv = acc_ref[...]
        cos = cos_ref[...]
        sin = sin_ref[...]
        for hh in range(H):
            qh = _rms_in(qkv[:, hh * DH:(hh + 1) * DH], qn_ref[...])
            q_ref[hh, :, :] = qh * cos + _rot_half(qh) * sin
        for g in range(KV):
            kh = _rms_in(qkv[:, (H + g) * DH:(H + g + 1) * DH], kn_ref[...])
            k_ref[g, :, :] = kh * cos + _rot_half(kh) * sin
            v_ref[g, :, :] = qkv[:, (H + KV + g) * DH:(H + KV + g + 1) * DH]


def _k1(x, wqkv, cos, sin, ln1_w, qn_w, kn_w):
    n = S // BS1
    return pl.pallas_call(
        _k1_body,
        grid=(n, KC1),
        in_specs=[
            pl.BlockSpec((BS1, D), lambda i, kc: (i, 0)),
            pl.BlockSpec((DK1, QKVW), lambda i, kc: (kc, 0)),
            pl.BlockSpec((BS1, DH), lambda i, kc: (i, 0)),
            pl.BlockSpec((BS1, DH), lambda i, kc: (i, 0)),
            pl.BlockSpec((1, D), lambda i, kc: (0, 0)),
            pl.BlockSpec((1, DH), lambda i, kc: (0, 0)),
            pl.BlockSpec((1, DH), lambda i, kc: (0, 0)),
        ],
        out_specs=[
            pl.BlockSpec((H, BS1, DH), lambda i, kc: (0, i, 0)),
            pl.BlockSpec((KV, BS1, DH), lambda i, kc: (0, i, 0)),
            pl.BlockSpec((KV, BS1, DH), lambda i, kc: (0, i, 0)),
        ],
        out_shape=[
            jax.ShapeDtypeStruct((H, S, DH), jnp.float32),
            jax.ShapeDtypeStruct((KV, S, DH), jnp.float32),
            jax.ShapeDtypeStruct((KV, S, DH), jnp.float32),
        ],
        scratch_shapes=[pltpu.VMEM((BS1, QKVW), jnp.float32)],
    )(x, wqkv, cos, sin, ln1_w, qn_w, kn_w)


def _k2_body(q_ref, k_ref, v_ref, o_ref):
    iq = pl.program_id(1)
    q = q_ref[0] * SCALE
    row = iq * BQ + lax.broadcasted_iota(jnp.int32, (BQ, BK), 0)

    def step(j, carry):
        m, l, acc = carry
        kj = k_ref[0, pl.ds(j * BK, BK), :]
        vj = v_ref[0, pl.ds(j * BK, BK), :]
        s = lax.dot_general(q, kj, (((1,), (1,)), ((), ())),
                            preferred_element_type=jnp.float32)
        col = j * BK + lax.broadcasted_iota(jnp.int32, (BQ, BK), 1)
        s = jnp.where(col <= row, s, -1e30)
        mnew = jnp.maximum(m, jnp.max(s, axis=-1, keepdims=True))
        p = jnp.exp(s - mnew)
        corr = jnp.exp(m - mnew)
        l = l * corr + jnp.sum(p, axis=-1, keepdims=True)
        acc = acc * corr + jnp.dot(p, vj, preferred_element_type=jnp.float32)
        return m * 0 + mnew, l, acc

    m0 = jnp.full((BQ, 1), -1e30, jnp.float32)
    l0 = jnp.zeros((BQ, 1), jnp.float32)
    a0 = jnp.zeros((BQ, DH), jnp.float32)
    m, l, acc = lax.fori_loop(0, (iq * BQ) // BK + 1, step, (m0, l0, a0))
    o_ref[0] = acc / l


def _k2(q, k, v):
    nq = S // BQ
    return pl.pallas_call(
        _k2_body,
        grid=(H, nq),
        in_specs=[
            pl.BlockSpec((1, BQ, DH), lambda h, i: (h, i, 0)),
            pl.BlockSpec((1, S, DH), lambda h, i: (h // (H // KV), 0, 0)),
            pl.BlockSpec((1, S, DH), lambda h, i: (h // (H // KV), 0, 0)),
        ],
        out_specs=pl.BlockSpec((1, BQ, DH), lambda h, i: (h, i, 0)),
        out_shape=jax.ShapeDtypeStruct((H, S, DH), jnp.float32),
    )(q, k, v)


HG3 = 4
KC3 = H // HG3


def _k3_body(x_ref, o_ref, wo_ref, ln2_ref, gw_ref,
             x2_ref, h2_ref, ti_ref, tw_ref, cnt_ref, acc_ref):
    kc = pl.program_id(1)
    s = jnp.dot(o_ref[0], wo_ref[pl.ds(0, DH), :],
                preferred_element_type=jnp.float32)
    for hh in range(1, HG3):
        s = s + jnp.dot(o_ref[hh], wo_ref[pl.ds(hh * DH, DH), :],
                        preferred_element_type=jnp.float32)

    @pl.when(kc == 0)
    def _():
        acc_ref[...] = x_ref[...] + s

    @pl.when(kc != 0)
    def _():
        acc_ref[...] = acc_ref[...] + s

    @pl.when(kc == KC3 - 1)
    def _():
        _k3_tail(acc_ref, ln2_ref, gw_ref, x2_ref, h2_ref,
                 ti_ref, tw_ref, cnt_ref)


def _k3_tail(acc_ref, ln2_ref, gw_ref, x2_ref, h2_ref, ti_ref, tw_ref, cnt_ref):
    acc = acc_ref[...]
    x2_ref[...] = acc
    h2 = _rms_in(acc, ln2_ref[...])
    h2_ref[...] = h2
    logits = jnp.dot(h2, gw_ref[...], preferred_element_type=jnp.float32)
    iot = lax.broadcasted_iota(jnp.int32, logits.shape, 1)
    m1 = jnp.max(logits, axis=-1, keepdims=True)
    i1 = jnp.min(jnp.where(logits == m1, iot, E), axis=-1, keepdims=True)
    l2m = jnp.where(iot == i1, -jnp.inf, logits)
    m2 = jnp.max(l2m, axis=-1, keepdims=True)
    i2 = jnp.min(jnp.where(l2m == m2, iot, E), axis=-1, keepdims=True)
    w1 = 1.0 / (1.0 + jnp.exp(m2 - m1))
    w2 = 1.0 - w1
    ti_ref[...] = jnp.concatenate([i1, i2], axis=1)
    tw_ref[...] = jnp.concatenate([w1, w2], axis=1)
    iot64 = lax.broadcasted_iota(jnp.int32, (BS3, 64), 1)
    oh = (iot64 == i1).astype(jnp.int32) + (iot64 == i2).astype(jnp.int32)
    cnt_ref[...] = jnp.sum(oh, axis=0, keepdims=True).reshape(1, 1, 64)


def _k3(x, o, wo, ln2_w, gate_w):
    n = S // BS3
    return pl.pallas_call(
        _k3_body,
        grid=(n, KC3),
        in_specs=[
            pl.BlockSpec((BS3, D), lambda i, kc: (i, 0)),
            pl.BlockSpec((HG3, BS3, DH), lambda i, kc: (kc, i, 0)),
            pl.BlockSpec((HG3 * DH, D), lambda i, kc: (kc, 0)),
            pl.BlockSpec((1, D), lambda i, kc: (0, 0)),
            pl.BlockSpec((D, E), lambda i, kc: (0, 0)),
        ],
        out_specs=[
            pl.BlockSpec((BS3, D), lambda i, kc: (i, 0)),
            pl.BlockSpec((BS3, D), lambda i, kc: (i, 0)),
            pl.BlockSpec((BS3, TK), lambda i, kc: (i, 0)),
            pl.BlockSpec((BS3, TK), lambda i, kc: (i, 0)),
            pl.BlockSpec((1, 1, 64), lambda i, kc: (i, 0, 0)),
        ],
        out_shape=[
            jax.ShapeDtypeStruct((S, D), jnp.float32),
            jax.ShapeDtypeStruct((S, D), jnp.float32),
            jax.ShapeDtypeStruct((S, TK), jnp.int32),
            jax.ShapeDtypeStruct((S, TK), jnp.float32),
            jax.ShapeDtypeStruct((S // BS3, 1, 64), jnp.int32),
        ],
        scratch_shapes=[pltpu.VMEM((BS3, D), jnp.float32)],
    )(x, o, wo, ln2_w, gate_w)


BLK = 256
NQ4 = 4
QTOK = S // NQ4
QPAIR = QTOK * TK
P = 6656
NB = P // BLK
NBP = 32
L = 16


def _extract(vec, lane):
    return jnp.sum(jnp.where(lax.iota(jnp.int32, L) == lane, vec, 0))


def _k4_kernel(ti_hbm, tw_hbm, h2_hbm, counts_hbm,
               hd_hbm, ws_hbm, pos_hbm, bexp_hbm,
               tiv, twv, posbuf, cmp_tok, cmp_w, cvm, zb, rows, bev, sem):
    c = lax.axis_index("c")
    s = lax.axis_index("s")
    e = s % E
    ql = s // E
    q = 2 * c + ql

    pltpu.sync_copy(counts_hbm.at[:], cvm)
    qoff = pl.multiple_of(q * QPAIR, QPAIR)
    pltpu.sync_copy(ti_hbm.at[pl.ds(qoff, QPAIR)], tiv)
    pltpu.sync_copy(tw_hbm.at[pl.ds(qoff, QPAIR)], twv)

    crow = [cvm[blk, 0, pl.ds(0, L)] for blk in range(2 * NQ4)]
    cq = {}
    cnt = {}
    for ee in range(E):
        for qq in range(NQ4):
            cval = _extract(crow[2 * qq], ee) + _extract(crow[2 * qq + 1], ee)
            cnt[(ee, qq)] = cval
            cq[(ee, qq)] = ((cval + L - 1) // L) * L
    base = {}
    endblk = []
    running = jnp.int32(0)
    for ee in range(E):
        tot = jnp.int32(0)
        for qq in range(NQ4):
            base[(ee, qq)] = running * BLK + tot
            tot = tot + cq[(ee, qq)]
        running = running + (tot + BLK - 1) // BLK
        endblk.append(running)

    my_base = jnp.int32(0)
    my_cnt = jnp.int32(0)
    my_cq = jnp.int32(0)
    for ee in range(E):
        for qq in range(NQ4):
            sel = jnp.logical_and(e == ee, q == qq)
            my_base = jnp.where(sel, base[(ee, qq)], my_base)
            my_cnt = jnp.where(sel, cnt[(ee, qq)], my_cnt)
            my_cq = jnp.where(sel, cq[(ee, qq)], my_cq)

    @pl.when(jnp.logical_and(c == 0, s == 0))
    def _():
        for ch in range(NBP // L):
            bv = lax.iota(jnp.int32, L) + ch * L
            acc = jnp.zeros((L,), jnp.int32)
            for ee in range(E - 1):
                acc = acc + (bv >= endblk[ee]).astype(jnp.int32)
            bev[pl.ds(ch * L, L)] = acc
        pltpu.sync_copy(bev, bexp_hbm.at[:])

    zv = jnp.zeros((L,), jnp.int32)
    for i in range(QPAIR // L):
        zb[pl.ds(i * L, L)] = zv
        cmp_tok[pl.ds(i * L, L)] = zv

    def pass2(i, cnt2):
        chunk = tiv[pl.ds(i * L, L)]
        mask = chunk == e
        mi = mask.astype(jnp.int32)
        within = plsc.cumsum(mi) - 1
        posv = my_base + cnt2 + within
        posbuf[pl.ds(i * L, L)] = jnp.where(mask, posv, 0)
        loc = cnt2 + within
        tok = (q * QPAIR + i * L + lax.iota(jnp.int32, L)) // TK
        plsc.store_scatter(cmp_tok, [loc], tok, mask=mask)
        plsc.store_scatter(cmp_w, [loc], twv[pl.ds(i * L, L)], mask=mask)
        return cnt2 + jnp.sum(mi)

    lax.fori_loop(0, QPAIR // L, pass2, jnp.int32(0))

    def gstep(j, carry):
        idxsl = cmp_tok.at[pl.ds(j * L, L)]
        pltpu.async_copy(h2_hbm.at[idxsl], rows, sem).wait()
        roff = pl.multiple_of(my_base + j * L, L)
        pltpu.sync_copy(rows, hd_hbm.at[pl.ds(roff, L)])
        pltpu.sync_copy(cmp_w.at[pl.ds(j * L, L)],
                        ws_hbm.at[pl.ds(roff, L)])
        return carry

    lax.fori_loop(0, my_cq // L, gstep, jnp.int32(0))

    pltpu.sync_copy(posbuf, pos_hbm.at[e, pl.ds(qoff, QPAIR)])


def _k4(ti_flat, tw_flat, h2, counts):
    mesh = plsc.VectorSubcoreMesh(core_axis_name="c", subcore_axis_name="s")
    kfn = pl.kernel(
        _k4_kernel,
        mesh=mesh,
        out_type=[
            jax.ShapeDtypeStruct((P, D), jnp.float32),
            jax.ShapeDtypeStruct((P,), jnp.float32),
            jax.ShapeDtypeStruct((E, S * TK), jnp.int32),
            jax.ShapeDtypeStruct((NBP,), jnp.int32),
        ],
        compiler_params=pltpu.CompilerParams(needs_layout_passes=False),
        scratch_types=[
            pltpu.VMEM((QPAIR,), jnp.int32),
            pltpu.VMEM((QPAIR,), jnp.float32),
            pltpu.VMEM((QPAIR,), jnp.int32),
            pltpu.VMEM((QPAIR,), jnp.int32),
            pltpu.VMEM((QPAIR,), jnp.float32),
            pltpu.VMEM((2 * NQ4, 1, 64), jnp.int32),
            pltpu.VMEM((QPAIR,), jnp.int32),
            pltpu.VMEM((L, D), jnp.float32),
            pltpu.VMEM((NBP,), jnp.int32),
            pltpu.SemaphoreType.DMA,
        ],
    )
    return kfn(ti_flat, tw_flat, h2, counts)


def _k5_body(bexp_ref, hd_ref, ws_ref, wg_ref, wu_ref, wd_ref, y_ref):
    hd = hd_ref[...]
    g = jnp.dot(hd, wg_ref[0], preferred_element_type=jnp.float32)
    u = jnp.dot(hd, wu_ref[0], preferred_element_type=jnp.float32)
    hh = (g * (1.0 / (1.0 + jnp.exp(-g)))) * u
    y = jnp.dot(hh, wd_ref[0], preferred_element_type=jnp.float32)
    y_ref[...] = y * ws_ref[...]


def _k5(hd, ws, bexp, w_gate, w_up, w_down):
    grid_spec = pltpu.PrefetchScalarGridSpec(
        num_scalar_prefetch=1,
        grid=(NB,),
        in_specs=[
            pl.BlockSpec((BLK, D), lambda b, be: (b, 0)),
            pl.BlockSpec((BLK, 1), lambda b, be: (b, 0)),
            pl.BlockSpec((1, D, F), lambda b, be: (be[b], 0, 0)),
            pl.BlockSpec((1, D, F), lambda b, be: (be[b], 0, 0)),
            pl.BlockSpec((1, F, D), lambda b, be: (be[b], 0, 0)),
        ],
        out_specs=pl.BlockSpec((BLK, D), lambda b, be: (b, 0)),
    )
    return pl.pallas_call(
        _k5_body,
        grid_spec=grid_spec,
        out_shape=jax.ShapeDtypeStruct((P, D), jnp.float32),
    )(bexp, hd, ws.reshape(P, 1), w_gate, w_up, w_down)


TPT = S // 32
CH6 = 8


def _k6_kernel(y_hbm, pos_hbm, x2_hbm, out_hbm, pidx, pparts, ybuf, xv, ov, sem):
    wid = lax.axis_index("c") * 16 + lax.axis_index("s")
    t0 = pl.multiple_of(wid * TPT, TPT)
    poff = pl.multiple_of(t0 * TK, TPT * TK)
    pltpu.sync_copy(pos_hbm.at[:, pl.ds(poff, TPT * TK)], pparts)
    npc = (TPT * TK) // L

    def sum_parts(i, carry):
        acc = pparts[0, pl.ds(i * L, L)]
        for ee in range(1, E):
            acc = acc + pparts[ee, pl.ds(i * L, L)]
        pidx[pl.ds(i * L, L)] = acc
        return carry

    lax.fori_loop(0, npc, sum_parts, jnp.int32(0))
    for ch in range(TPT // CH6):
        idxsl = pidx.at[pl.ds(ch * CH6 * TK, L)]
        pltpu.async_copy(y_hbm.at[idxsl], ybuf, sem).wait()
        pltpu.sync_copy(x2_hbm.at[pl.ds(pl.multiple_of(t0 + ch * CH6, CH6), CH6)], xv)

        def body(j, carry):
            sl = pl.ds(j * L, L)
            for tt in range(CH6):
                ov[tt, sl] = xv[tt, sl] + ybuf[2 * tt, sl] + ybuf[2 * tt + 1, sl]
            return carry

        lax.fori_loop(0, D // L, body, jnp.int32(0))
        pltpu.sync_copy(ov, out_hbm.at[pl.ds(pl.multiple_of(t0 + ch * CH6, CH6), CH6)])


def _k6(y, pos, x2):
    mesh = plsc.VectorSubcoreMesh(core_axis_name="c", subcore_axis_name="s")
    kfn = pl.kernel(
        _k6_kernel,
        mesh=mesh,
        out_type=jax.ShapeDtypeStruct((S, D), jnp.float32),
        compiler_params=pltpu.CompilerParams(needs_layout_passes=False),
        scratch_types=[
            pltpu.VMEM((TPT * TK,), jnp.int32),
            pltpu.VMEM((E, TPT * TK), jnp.int32),
            pltpu.VMEM((L, D), jnp.float32),
            pltpu.VMEM((CH6, D), jnp.float32),
            pltpu.VMEM((CH6, D), jnp.float32),
            pltpu.SemaphoreType.DMA,
        ],
    )
    return kfn(y, pos, x2)


def kernel(hidden_states, start_pos, position_embeddings, attention_mask,
           wq, wk, wv, wo, q_norm_w, k_norm_w, ln1_w, ln2_w,
           gate_w, w_gate, w_up, w_down):
    x = hidden_states.reshape(S, D)
    cos = position_embeddings[0]
    sin = position_embeddings[1]
    wqkv = jnp.concatenate([wq, wk, wv], axis=1)
    q, k, v = _k1(x, wqkv, cos, sin,
                  ln1_w.reshape(1, D), q_norm_w.reshape(1, DH),
                  k_norm_w.reshape(1, DH))
    o = _k2(q, k, v)
    x2, h2, ti, tw, counts = _k3(x, o, wo, ln2_w.reshape(1, D), gate_w)
    hd, ws, pos, bexp = _k4(ti.reshape(S * TK), tw.reshape(S * TK),
                            h2, counts)
    y = _k5(hd, ws, bexp, w_gate, w_up, w_down)
    out = _k6(y, pos, x2)
    return out.reshape(B, S, D)


# speedup vs baseline: 1.3022x; 1.0404x over previous
import functools
import jax
import jax.numpy as jnp
from jax import lax
from jax.experimental import pallas as pl
from jax.experimental.pallas import tpu as pltpu
from jax.experimental.pallas import tpu_sc as plsc

B, S, D = 1, 2048, 2048
H, KV, DH = 16, 4, 128
E, TK, F = 8, 2, 768
EPS = 1e-6
SCALE = DH ** -0.5

BS1 = 256
BQ = 256
BK = 1024
BS3 = 256
BM5 = 256


def _rms_in(x, w):
    v = jnp.mean(jnp.square(x), axis=-1, keepdims=True)
    return w * (x * lax.rsqrt(v + EPS))


def _rot_half(x):
    h = x.shape[-1] // 2
    return jnp.concatenate([-x[:, h:], x[:, :h]], axis=-1)


DK1 = 512
KC1 = D // DK1
QKVW = (H + 2 * KV) * DH


def _k1_body(x_ref, w_ref, cos_ref, sin_ref,
             ln1_ref, qn_ref, kn_ref, q_ref, k_ref, v_ref, acc_ref):
    kc = pl.program_id(1)
    x = x_ref[...]
    v = jnp.mean(jnp.square(x), axis=-1, keepdims=True)
    scale = lax.rsqrt(v + EPS)
    xs = x_ref[:, pl.ds(kc * DK1, DK1)]
    ws = ln1_ref[:, pl.ds(kc * DK1, DK1)]
    h = xs * scale * ws
    part = jnp.dot(h, w_ref[...], preferred_element_type=jnp.float32)

    @pl.when(kc == 0)
    def _():
        acc_ref[...] = part

    @pl.when(kc != 0)
    def _():
        acc_ref[...] = acc_ref[...] + part

    @pl.when(kc == KC1 - 1)
    def _():
        qkv = acc_ref[...]
        cos = cos_ref[...]
        sin = sin_ref[...]
        for hh in range(H):
            qh = _rms_in(qkv[:, hh * DH:(hh + 1) * DH], qn_ref[...])
            q_ref[hh, :, :] = qh * cos + _rot_half(qh) * sin
        for g in range(KV):
            kh = _rms_in(qkv[:, (H + g) * DH:(H + g + 1) * DH], kn_ref[...])
            k_ref[g, :, :] = kh * cos + _rot_half(kh) * sin
            v_ref[g, :, :] = qkv[:, (H + KV + g) * DH:(H + KV + g + 1) * DH]


def _k1(x, wqkv, cos, sin, ln1_w, qn_w, kn_w):
    n = S // BS1
    return pl.pallas_call(
        _k1_body,
        grid=(n, KC1),
        in_specs=[
            pl.BlockSpec((BS1, D), lambda i, kc: (i, 0)),
            pl.BlockSpec((DK1, QKVW), lambda i, kc: (kc, 0)),
            pl.BlockSpec((BS1, DH), lambda i, kc: (i, 0)),
            pl.BlockSpec((BS1, DH), lambda i, kc: (i, 0)),
            pl.BlockSpec((1, D), lambda i, kc: (0, 0)),
            pl.BlockSpec((1, DH), lambda i, kc: (0, 0)),
            pl.BlockSpec((1, DH), lambda i, kc: (0, 0)),
        ],
        out_specs=[
            pl.BlockSpec((H, BS1, DH), lambda i, kc: (0, i, 0)),
            pl.BlockSpec((KV, BS1, DH), lambda i, kc: (0, i, 0)),
            pl.BlockSpec((KV, BS1, DH), lambda i, kc: (0, i, 0)),
        ],
        out_shape=[
            jax.ShapeDtypeStruct((H, S, DH), jnp.float32),
            jax.ShapeDtypeStruct((KV, S, DH), jnp.float32),
            jax.ShapeDtypeStruct((KV, S, DH), jnp.float32),
        ],
        scratch_shapes=[pltpu.VMEM((BS1, QKVW), jnp.float32)],
    )(x, wqkv, cos, sin, ln1_w, qn_w, kn_w)


def _k2_body(q_ref, k_ref, v_ref, o_ref):
    iq = pl.program_id(1)
    q = q_ref[0] * SCALE
    row = iq * BQ + lax.broadcasted_iota(jnp.int32, (BQ, BK), 0)

    def step(j, carry):
        m, l, acc = carry
        kj = k_ref[0, pl.ds(j * BK, BK), :]
        vj = v_ref[0, pl.ds(j * BK, BK), :]
        s = lax.dot_general(q, kj, (((1,), (1,)), ((), ())),
                            preferred_element_type=jnp.float32)
        col = j * BK + lax.broadcasted_iota(jnp.int32, (BQ, BK), 1)
        s = jnp.where(col <= row, s, -1e30)
        mnew = jnp.maximum(m, jnp.max(s, axis=-1, keepdims=True))
        p = jnp.exp(s - mnew)
        corr = jnp.exp(m - mnew)
        l = l * corr + jnp.sum(p, axis=-1, keepdims=True)
        acc = acc * corr + jnp.dot(p, vj, preferred_element_type=jnp.float32)
        return m * 0 + mnew, l, acc

    m0 = jnp.full((BQ, 1), -1e30, jnp.float32)
    l0 = jnp.zeros((BQ, 1), jnp.float32)
    a0 = jnp.zeros((BQ, DH), jnp.float32)
    m, l, acc = lax.fori_loop(0, (iq * BQ) // BK + 1, step, (m0, l0, a0))
    o_ref[0] = acc / l


def _k2(q, k, v):
    nq = S // BQ
    return pl.pallas_call(
        _k2_body,
        grid=(H, nq),
        in_specs=[
            pl.BlockSpec((1, BQ, DH), lambda h, i: (h, i, 0)),
            pl.BlockSpec((1, S, DH), lambda h, i: (h // (H // KV), 0, 0)),
            pl.BlockSpec((1, S, DH), lambda h, i: (h // (H // KV), 0, 0)),
        ],
        out_specs=pl.BlockSpec((1, BQ, DH), lambda h, i: (h, i, 0)),
        out_shape=jax.ShapeDtypeStruct((H, S, DH), jnp.float32),
    )(q, k, v)


HG3 = 4
KC3 = H // HG3


def _k3_body(x_ref, o_ref, wo_ref, ln2_ref, gw_ref,
             x2_ref, h2_ref, ti_ref, tw_ref, cnt_ref, acc_ref):
    kc = pl.program_id(1)
    s = jnp.dot(o_ref[0], wo_ref[pl.ds(0, DH), :],
                preferred_element_type=jnp.float32)
    for hh in range(1, HG3):
        s = s + jnp.dot(o_ref[hh], wo_ref[pl.ds(hh * DH, DH), :],
                        preferred_element_type=jnp.float32)

    @pl.when(kc == 0)
    def _():
        acc_ref[...] = x_ref[...] + s

    @pl.when(kc != 0)
    def _():
        acc_ref[...] = acc_ref[...] + s

    @pl.when(kc == KC3 - 1)
    def _():
        _k3_tail(acc_ref, ln2_ref, gw_ref, x2_ref, h2_ref,
                 ti_ref, tw_ref, cnt_ref)


def _k3_tail(acc_ref, ln2_ref, gw_ref, x2_ref, h2_ref, ti_ref, tw_ref, cnt_ref):
    acc = acc_ref[...]
    x2_ref[...] = acc
    h2 = _rms_in(acc, ln2_ref[...])
    h2_ref[...] = h2
    logits = jnp.dot(h2, gw_ref[...], preferred_element_type=jnp.float32)
    iot = lax.broadcasted_iota(jnp.int32, logits.shape, 1)
    m1 = jnp.max(logits, axis=-1, keepdims=True)
    i1 = jnp.min(jnp.where(logits == m1, iot, E), axis=-1, keepdims=True)
    l2m = jnp.where(iot == i1, -jnp.inf, logits)
    m2 = jnp.max(l2m, axis=-1, keepdims=True)
    i2 = jnp.min(jnp.where(l2m == m2, iot, E), axis=-1, keepdims=True)
    w1 = 1.0 / (1.0 + jnp.exp(m2 - m1))
    w2 = 1.0 - w1
    ti_ref[...] = jnp.concatenate([i1, i2], axis=1)
    tw_ref[...] = jnp.concatenate([w1, w2], axis=1)
    iot64 = lax.broadcasted_iota(jnp.int32, (BS3, 64), 1)
    oh = (iot64 == i1).astype(jnp.int32) + (iot64 == i2).astype(jnp.int32)
    cnt_ref[...] = jnp.sum(oh, axis=0, keepdims=True).reshape(1, 1, 64)


def _k3(x, o, wo, ln2_w, gate_w):
    n = S // BS3
    return pl.pallas_call(
        _k3_body,
        grid=(n, KC3),
        in_specs=[
            pl.BlockSpec((BS3, D), lambda i, kc: (i, 0)),
            pl.BlockSpec((HG3, BS3, DH), lambda i, kc: (kc, i, 0)),
            pl.BlockSpec((HG3 * DH, D), lambda i, kc: (kc, 0)),
            pl.BlockSpec((1, D), lambda i, kc: (0, 0)),
            pl.BlockSpec((D, E), lambda i, kc: (0, 0)),
        ],
        out_specs=[
            pl.BlockSpec((BS3, D), lambda i, kc: (i, 0)),
            pl.BlockSpec((BS3, D), lambda i, kc: (i, 0)),
            pl.BlockSpec((BS3, TK), lambda i, kc: (i, 0)),
            pl.BlockSpec((BS3, TK), lambda i, kc: (i, 0)),
            pl.BlockSpec((1, 1, 64), lambda i, kc: (i, 0, 0)),
        ],
        out_shape=[
            jax.ShapeDtypeStruct((S, D), jnp.float32),
            jax.ShapeDtypeStruct((S, D), jnp.float32),
            jax.ShapeDtypeStruct((S, TK), jnp.int32),
            jax.ShapeDtypeStruct((S, TK), jnp.float32),
            jax.ShapeDtypeStruct((S // BS3, 1, 64), jnp.int32),
        ],
        scratch_shapes=[pltpu.VMEM((BS3, D), jnp.float32)],
    )(x, o, wo, ln2_w, gate_w)


BLK = 256
NQ4 = 4
QTOK = S // NQ4
QPAIR = QTOK * TK
P = 6656
NB = P // BLK
NBP = 32
L = 16


def _extract(vec, lane):
    return jnp.sum(jnp.where(lax.iota(jnp.int32, L) == lane, vec, 0))


def _k4_kernel(ti_hbm, tw_hbm, h2_hbm, counts_hbm,
               hd_hbm, ws_hbm, pos_hbm, bexp_hbm,
               tiv, twv, posbuf, cmp_tok, cmp_w, cvm, zb, rows, bev, sem):
    c = lax.axis_index("c")
    s = lax.axis_index("s")
    e = s % E
    ql = s // E
    q = 2 * c + ql

    pltpu.sync_copy(counts_hbm.at[:], cvm)
    qoff = pl.multiple_of(q * QPAIR, QPAIR)
    pltpu.sync_copy(ti_hbm.at[pl.ds(qoff, QPAIR)], tiv)
    pltpu.sync_copy(tw_hbm.at[pl.ds(qoff, QPAIR)], twv)

    crow = [cvm[blk, 0, pl.ds(0, L)] for blk in range(2 * NQ4)]
    cq = {}
    cnt = {}
    for ee in range(E):
        for qq in range(NQ4):
            cval = _extract(crow[2 * qq], ee) + _extract(crow[2 * qq + 1], ee)
            cnt[(ee, qq)] = cval
            cq[(ee, qq)] = ((cval + L - 1) // L) * L
    base = {}
    endblk = []
    running = jnp.int32(0)
    for ee in range(E):
        tot = jnp.int32(0)
        for qq in range(NQ4):
            base[(ee, qq)] = running * BLK + tot
            tot = tot + cq[(ee, qq)]
        running = running + (tot + BLK - 1) // BLK
        endblk.append(running)

    my_base = jnp.int32(0)
    my_cnt = jnp.int32(0)
    my_cq = jnp.int32(0)
    for ee in range(E):
        for qq in range(NQ4):
            sel = jnp.logical_and(e == ee, q == qq)
            my_base = jnp.where(sel, base[(ee, qq)], my_base)
            my_cnt = jnp.where(sel, cnt[(ee, qq)], my_cnt)
            my_cq = jnp.where(sel, cq[(ee, qq)], my_cq)

    @pl.when(jnp.logical_and(c == 0, s == 0))
    def _():
        for ch in range(NBP // L):
            bv = lax.iota(jnp.int32, L) + ch * L
            acc = jnp.zeros((L,), jnp.int32)
            for ee in range(E - 1):
                acc = acc + (bv >= endblk[ee]).astype(jnp.int32)
            bev[pl.ds(ch * L, L)] = acc
        pltpu.sync_copy(bev, bexp_hbm.at[:])

    zv = jnp.zeros((L,), jnp.int32)
    for i in range(QPAIR // L):
        zb[pl.ds(i * L, L)] = zv
        cmp_tok[pl.ds(i * L, L)] = zv

    def pass2(i, cnt2):
        chunk = tiv[pl.ds(i * L, L)]
        mask = chunk == e
        mi = mask.astype(jnp.int32)
        within = plsc.cumsum(mi) - 1
        posv = my_base + cnt2 + within
        posbuf[pl.ds(i * L, L)] = jnp.where(mask, posv, 0)
        loc = cnt2 + within
        tok = (q * QPAIR + i * L + lax.iota(jnp.int32, L)) // TK
        plsc.store_scatter(cmp_tok, [loc], tok, mask=mask)
        plsc.store_scatter(cmp_w, [loc], twv[pl.ds(i * L, L)], mask=mask)
        return cnt2 + jnp.sum(mi)

    lax.fori_loop(0, QPAIR // L, pass2, jnp.int32(0))

    def gstep(j, carry):
        idxsl = cmp_tok.at[pl.ds(j * L, L)]
        pltpu.async_copy(h2_hbm.at[idxsl], rows, sem).wait()
        roff = pl.multiple_of(my_base + j * L, L)
        pltpu.sync_copy(rows, hd_hbm.at[pl.ds(roff, L)])
        pltpu.sync_copy(cmp_w.at[pl.ds(j * L, L)],
                        ws_hbm.at[pl.ds(roff, L)])
        return carry

    lax.fori_loop(0, my_cq // L, gstep, jnp.int32(0))

    pltpu.sync_copy(posbuf, pos_hbm.at[e, pl.ds(qoff, QPAIR)])


def _k4(ti_flat, tw_flat, h2, counts):
    mesh = plsc.VectorSubcoreMesh(core_axis_name="c", subcore_axis_name="s")
    kfn = pl.kernel(
        _k4_kernel,
        mesh=mesh,
        out_type=[
            jax.ShapeDtypeStruct((P, D), jnp.float32),
            jax.ShapeDtypeStruct((P,), jnp.float32),
            jax.ShapeDtypeStruct((E, S * TK), jnp.int32),
            jax.ShapeDtypeStruct((NBP,), jnp.int32),
        ],
        compiler_params=pltpu.CompilerParams(needs_layout_passes=False),
        scratch_types=[
            pltpu.VMEM((QPAIR,), jnp.int32),
            pltpu.VMEM((QPAIR,), jnp.float32),
            pltpu.VMEM((QPAIR,), jnp.int32),
            pltpu.VMEM((QPAIR,), jnp.int32),
            pltpu.VMEM((QPAIR,), jnp.float32),
            pltpu.VMEM((2 * NQ4, 1, 64), jnp.int32),
            pltpu.VMEM((QPAIR,), jnp.int32),
            pltpu.VMEM((L, D), jnp.float32),
            pltpu.VMEM((NBP,), jnp.int32),
            pltpu.SemaphoreType.DMA,
        ],
    )
    return kfn(ti_flat, tw_flat, h2, counts)


def _k5_body(bexp_ref, hd_ref, ws_ref, wg_ref, wu_ref, wd_ref, y_ref):
    hd = hd_ref[...]
    g = jnp.dot(hd, wg_ref[0], preferred_element_type=jnp.float32)
    u = jnp.dot(hd, wu_ref[0], preferred_element_type=jnp.float32)
    hh = (g * (1.0 / (1.0 + jnp.exp(-g)))) * u
    y = jnp.dot(hh, wd_ref[0], preferred_element_type=jnp.float32)
    y_ref[...] = y * ws_ref[...]


def _k5(hd, ws, bexp, w_gate, w_up, w_down):
    grid_spec = pltpu.PrefetchScalarGridSpec(
        num_scalar_prefetch=1,
        grid=(NB,),
        in_specs=[
            pl.BlockSpec((BLK, D), lambda b, be: (b, 0)),
            pl.BlockSpec((BLK, 1), lambda b, be: (b, 0)),
            pl.BlockSpec((1, D, F), lambda b, be: (be[b], 0, 0)),
            pl.BlockSpec((1, D, F), lambda b, be: (be[b], 0, 0)),
            pl.BlockSpec((1, F, D), lambda b, be: (be[b], 0, 0)),
        ],
        out_specs=pl.BlockSpec((BLK, D), lambda b, be: (b, 0)),
    )
    return pl.pallas_call(
        _k5_body,
        grid_spec=grid_spec,
        out_shape=jax.ShapeDtypeStruct((P, D), jnp.float32),
    )(bexp, hd, ws.reshape(P, 1), w_gate, w_up, w_down)


TPT = S // 32
CH6 = 8


def _k6_kernel(y_hbm, pos_hbm, x2_hbm, out_hbm, pidx, pparts, ybuf, xv, ov, sem):
    wid = lax.axis_index("c") * 16 + lax.axis_index("s")
    t0 = pl.multiple_of(wid * TPT, TPT)
    poff = pl.multiple_of(t0 * TK, TPT * TK)
    pltpu.sync_copy(pos_hbm.at[:, pl.ds(poff, TPT * TK)], pparts)
    npc = (TPT * TK) // L

    def sum_parts(i, carry):
        acc = pparts[0, pl.ds(i * L, L)]
        for ee in range(1, E):
            acc = acc + pparts[ee, pl.ds(i * L, L)]
        pidx[pl.ds(i * L, L)] = acc
        return carry

    lax.fori_loop(0, npc, sum_parts, jnp.int32(0))
    for ch in range(TPT // CH6):
        idxsl = pidx.at[pl.ds(ch * CH6 * TK, L)]
        pltpu.async_copy(y_hbm.at[idxsl], ybuf, sem).wait()
        pltpu.sync_copy(x2_hbm.at[pl.ds(pl.multiple_of(t0 + ch * CH6, CH6), CH6)], xv)

        def body(j, carry):
            sl = pl.ds(j * L, L)
            for tt in range(CH6):
                ov[tt, sl] = xv[tt, sl] + ybuf[2 * tt, sl] + ybuf[2 * tt + 1, sl]
            return carry

        lax.fori_loop(0, D // L, body, jnp.int32(0))
        pltpu.sync_copy(ov, out_hbm.at[pl.ds(pl.multiple_of(t0 + ch * CH6, CH6), CH6)])


def _k6(y, pos, x2):
    mesh = plsc.VectorSubcoreMesh(core_axis_name="c", subcore_axis_name="s")
    kfn = pl.kernel(
        _k6_kernel,
        mesh=mesh,
        out_type=jax.ShapeDtypeStruct((S, D), jnp.float32),
        compiler_params=pltpu.CompilerParams(needs_layout_passes=False),
        scratch_types=[
            pltpu.VMEM((TPT * TK,), jnp.int32),
            pltpu.VMEM((E, TPT * TK), jnp.int32),
            pltpu.VMEM((L, D), jnp.float32),
            pltpu.VMEM((CH6, D), jnp.float32),
            pltpu.VMEM((CH6, D), jnp.float32),
            pltpu.SemaphoreType.DMA,
        ],
    )
    return kfn(y, pos, x2)


def kernel(hidden_states, start_pos, position_embeddings, attention_mask,
           wq, wk, wv, wo, q_norm_w, k_norm_w, ln1_w, ln2_w,
           gate_w, w_gate, w_up, w_down):
    x = hidden_states.reshape(S, D)
    cos = position_embeddings[0]
    sin = position_embeddings[1]
    wqkv = jnp.concatenate([wq, wk, wv], axis=1)
    q, k, v = _k1(x, wqkv, cos, sin,
                  ln1_w.reshape(1, D), q_norm_w.reshape(1, DH),
                  k_norm_w.reshape(1, DH))
    o = _k2(q, k, v)
    x2, h2, ti, tw, counts = _k3(x, o, wo, ln2_w.reshape(1, D), gate_w)
    hd, ws, pos, bexp = _k4(ti.reshape(S * TK), tw.reshape(S * TK),
                            h2, counts)
    y = _k5(hd, ws, bexp, w_gate, w_up, w_down)
    out = _k6(y, pos, x2)
    return out.reshape(B, S, D)


# speedup vs baseline: 1.3757x; 1.0565x over previous
import functools
import jax
import jax.numpy as jnp
from jax import lax
from jax.experimental import pallas as pl
from jax.experimental.pallas import tpu as pltpu
from jax.experimental.pallas import tpu_sc as plsc

B, S, D = 1, 2048, 2048
H, KV, DH = 16, 4, 128
E, TK, F = 8, 2, 768
EPS = 1e-6
SCALE = DH ** -0.5

BS1 = 256
BQ = 512
BK = 1024
BS3 = 256
BM5 = 256


def _rms_in(x, w):
    v = jnp.mean(jnp.square(x), axis=-1, keepdims=True)
    return w * (x * lax.rsqrt(v + EPS))


def _rot_half(x):
    h = x.shape[-1] // 2
    return jnp.concatenate([-x[:, h:], x[:, :h]], axis=-1)


DK1 = 512
KC1 = D // DK1
QKVW = (H + 2 * KV) * DH


def _k1_body(x_ref, w_ref, cos_ref, sin_ref,
             ln1_ref, qn_ref, kn_ref, q_ref, k_ref, v_ref, acc_ref):
    kc = pl.program_id(1)
    x = x_ref[...]
    v = jnp.mean(jnp.square(x), axis=-1, keepdims=True)
    scale = lax.rsqrt(v + EPS)
    xs = x_ref[:, pl.ds(kc * DK1, DK1)]
    ws = ln1_ref[:, pl.ds(kc * DK1, DK1)]
    h = xs * scale * ws
    part = jnp.dot(h, w_ref[...], preferred_element_type=jnp.float32)

    @pl.when(kc == 0)
    def _():
        acc_ref[...] = part

    @pl.when(kc != 0)
    def _():
        acc_ref[...] = acc_ref[...] + part

    @pl.when(kc == KC1 - 1)
    def _():
        qkv = acc_ref[...]
        cos = cos_ref[...]
        sin = sin_ref[...]
        for hh in range(H):
            qh = _rms_in(qkv[:, hh * DH:(hh + 1) * DH], qn_ref[...])
            q_ref[hh, :, :] = qh * cos + _rot_half(qh) * sin
        for g in range(KV):
            kh = _rms_in(qkv[:, (H + g) * DH:(H + g + 1) * DH], kn_ref[...])
            k_ref[g, :, :] = kh * cos + _rot_half(kh) * sin
            v_ref[g, :, :] = qkv[:, (H + KV + g) * DH:(H + KV + g + 1) * DH]


def _k1(x, wqkv, cos, sin, ln1_w, qn_w, kn_w):
    n = S // BS1
    return pl.pallas_call(
        _k1_body,
        grid=(n, KC1),
        in_specs=[
            pl.BlockSpec((BS1, D), lambda i, kc: (i, 0)),
            pl.BlockSpec((DK1, QKVW), lambda i, kc: (kc, 0)),
            pl.BlockSpec((BS1, DH), lambda i, kc: (i, 0)),
            pl.BlockSpec((BS1, DH), lambda i, kc: (i, 0)),
            pl.BlockSpec((1, D), lambda i, kc: (0, 0)),
            pl.BlockSpec((1, DH), lambda i, kc: (0, 0)),
            pl.BlockSpec((1, DH), lambda i, kc: (0, 0)),
        ],
        out_specs=[
            pl.BlockSpec((H, BS1, DH), lambda i, kc: (0, i, 0)),
            pl.BlockSpec((KV, BS1, DH), lambda i, kc: (0, i, 0)),
            pl.BlockSpec((KV, BS1, DH), lambda i, kc: (0, i, 0)),
        ],
        out_shape=[
            jax.ShapeDtypeStruct((H, S, DH), jnp.float32),
            jax.ShapeDtypeStruct((KV, S, DH), jnp.float32),
            jax.ShapeDtypeStruct((KV, S, DH), jnp.float32),
        ],
        scratch_shapes=[pltpu.VMEM((BS1, QKVW), jnp.float32)],
    )(x, wqkv, cos, sin, ln1_w, qn_w, kn_w)


def _k2_body(q_ref, k_ref, v_ref, o_ref):
    iq = pl.program_id(1)
    q = q_ref[0] * SCALE
    row = iq * BQ + lax.broadcasted_iota(jnp.int32, (BQ, BK), 0)

    def step(j, carry):
        m, l, acc = carry
        kj = k_ref[0, pl.ds(j * BK, BK), :]
        vj = v_ref[0, pl.ds(j * BK, BK), :]
        s = lax.dot_general(q, kj, (((1,), (1,)), ((), ())),
                            preferred_element_type=jnp.float32)
        col = j * BK + lax.broadcasted_iota(jnp.int32, (BQ, BK), 1)
        s = jnp.where(col <= row, s, -1e30)
        mnew = jnp.maximum(m, jnp.max(s, axis=-1, keepdims=True))
        p = jnp.exp(s - mnew)
        corr = jnp.exp(m - mnew)
        l = l * corr + jnp.sum(p, axis=-1, keepdims=True)
        acc = acc * corr + jnp.dot(p, vj, preferred_element_type=jnp.float32)
        return m * 0 + mnew, l, acc

    m0 = jnp.full((BQ, 1), -1e30, jnp.float32)
    l0 = jnp.zeros((BQ, 1), jnp.float32)
    a0 = jnp.zeros((BQ, DH), jnp.float32)
    m, l, acc = lax.fori_loop(0, (iq * BQ) // BK + 1, step, (m0, l0, a0))
    o_ref[0] = acc / l


def _k2(q, k, v):
    nq = S // BQ
    return pl.pallas_call(
        _k2_body,
        grid=(H, nq),
        in_specs=[
            pl.BlockSpec((1, BQ, DH), lambda h, i: (h, i, 0)),
            pl.BlockSpec((1, S, DH), lambda h, i: (h // (H // KV), 0, 0)),
            pl.BlockSpec((1, S, DH), lambda h, i: (h // (H // KV), 0, 0)),
        ],
        out_specs=pl.BlockSpec((1, BQ, DH), lambda h, i: (h, i, 0)),
        out_shape=jax.ShapeDtypeStruct((H, S, DH), jnp.float32),
    )(q, k, v)


HG3 = 4
KC3 = H // HG3


def _k3_body(x_ref, o_ref, wo_ref, ln2_ref, gw_ref,
             x2_ref, h2_ref, ti_ref, tw_ref, cnt_ref, acc_ref):
    kc = pl.program_id(1)
    s = jnp.dot(o_ref[0], wo_ref[pl.ds(0, DH), :],
                preferred_element_type=jnp.float32)
    for hh in range(1, HG3):
        s = s + jnp.dot(o_ref[hh], wo_ref[pl.ds(hh * DH, DH), :],
                        preferred_element_type=jnp.float32)

    @pl.when(kc == 0)
    def _():
        acc_ref[...] = x_ref[...] + s

    @pl.when(kc != 0)
    def _():
        acc_ref[...] = acc_ref[...] + s

    @pl.when(kc == KC3 - 1)
    def _():
        _k3_tail(acc_ref, ln2_ref, gw_ref, x2_ref, h2_ref,
                 ti_ref, tw_ref, cnt_ref)


def _k3_tail(acc_ref, ln2_ref, gw_ref, x2_ref, h2_ref, ti_ref, tw_ref, cnt_ref):
    acc = acc_ref[...]
    x2_ref[...] = acc
    h2 = _rms_in(acc, ln2_ref[...])
    h2_ref[...] = h2
    logits = jnp.dot(h2, gw_ref[...], preferred_element_type=jnp.float32)
    iot = lax.broadcasted_iota(jnp.int32, logits.shape, 1)
    m1 = jnp.max(logits, axis=-1, keepdims=True)
    i1 = jnp.min(jnp.where(logits == m1, iot, E), axis=-1, keepdims=True)
    l2m = jnp.where(iot == i1, -jnp.inf, logits)
    m2 = jnp.max(l2m, axis=-1, keepdims=True)
    i2 = jnp.min(jnp.where(l2m == m2, iot, E), axis=-1, keepdims=True)
    w1 = 1.0 / (1.0 + jnp.exp(m2 - m1))
    w2 = 1.0 - w1
    ti_ref[...] = jnp.concatenate([i1, i2], axis=1)
    tw_ref[...] = jnp.concatenate([w1, w2], axis=1)
    iot64 = lax.broadcasted_iota(jnp.int32, (BS3, 64), 1)
    oh = (iot64 == i1).astype(jnp.int32) + (iot64 == i2).astype(jnp.int32)
    cnt_ref[...] = jnp.sum(oh, axis=0, keepdims=True).reshape(1, 1, 64)


def _k3(x, o, wo, ln2_w, gate_w):
    n = S // BS3
    return pl.pallas_call(
        _k3_body,
        grid=(n, KC3),
        in_specs=[
            pl.BlockSpec((BS3, D), lambda i, kc: (i, 0)),
            pl.BlockSpec((HG3, BS3, DH), lambda i, kc: (kc, i, 0)),
            pl.BlockSpec((HG3 * DH, D), lambda i, kc: (kc, 0)),
            pl.BlockSpec((1, D), lambda i, kc: (0, 0)),
            pl.BlockSpec((D, E), lambda i, kc: (0, 0)),
        ],
        out_specs=[
            pl.BlockSpec((BS3, D), lambda i, kc: (i, 0)),
            pl.BlockSpec((BS3, D), lambda i, kc: (i, 0)),
            pl.BlockSpec((BS3, TK), lambda i, kc: (i, 0)),
            pl.BlockSpec((BS3, TK), lambda i, kc: (i, 0)),
            pl.BlockSpec((1, 1, 64), lambda i, kc: (i, 0, 0)),
        ],
        out_shape=[
            jax.ShapeDtypeStruct((S, D), jnp.float32),
            jax.ShapeDtypeStruct((S, D), jnp.float32),
            jax.ShapeDtypeStruct((S, TK), jnp.int32),
            jax.ShapeDtypeStruct((S, TK), jnp.float32),
            jax.ShapeDtypeStruct((S // BS3, 1, 64), jnp.int32),
        ],
        scratch_shapes=[pltpu.VMEM((BS3, D), jnp.float32)],
    )(x, o, wo, ln2_w, gate_w)


BLK = 256
NQ4 = 4
QTOK = S // NQ4
QPAIR = QTOK * TK
P = 6656
NB = P // BLK
NBP = 32
L = 16


def _extract(vec, lane):
    return jnp.sum(jnp.where(lax.iota(jnp.int32, L) == lane, vec, 0))


def _k4_kernel(ti_hbm, tw_hbm, h2_hbm, counts_hbm,
               hd_hbm, ws_hbm, pos_hbm, bexp_hbm,
               tiv, twv, posbuf, cmp_tok, cmp_w, cvm, zb, rows, bev, sem):
    c = lax.axis_index("c")
    s = lax.axis_index("s")
    e = s % E
    ql = s // E
    q = 2 * c + ql

    pltpu.sync_copy(counts_hbm.at[:], cvm)
    qoff = pl.multiple_of(q * QPAIR, QPAIR)
    pltpu.sync_copy(ti_hbm.at[pl.ds(qoff, QPAIR)], tiv)
    pltpu.sync_copy(tw_hbm.at[pl.ds(qoff, QPAIR)], twv)

    crow = [cvm[blk, 0, pl.ds(0, L)] for blk in range(2 * NQ4)]
    cq = {}
    cnt = {}
    for ee in range(E):
        for qq in range(NQ4):
            cval = _extract(crow[2 * qq], ee) + _extract(crow[2 * qq + 1], ee)
            cnt[(ee, qq)] = cval
            cq[(ee, qq)] = ((cval + L - 1) // L) * L
    base = {}
    endblk = []
    running = jnp.int32(0)
    for ee in range(E):
        tot = jnp.int32(0)
        for qq in range(NQ4):
            base[(ee, qq)] = running * BLK + tot
            tot = tot + cq[(ee, qq)]
        running = running + (tot + BLK - 1) // BLK
        endblk.append(running)

    my_base = jnp.int32(0)
    my_cnt = jnp.int32(0)
    my_cq = jnp.int32(0)
    for ee in range(E):
        for qq in range(NQ4):
            sel = jnp.logical_and(e == ee, q == qq)
            my_base = jnp.where(sel, base[(ee, qq)], my_base)
            my_cnt = jnp.where(sel, cnt[(ee, qq)], my_cnt)
            my_cq = jnp.where(sel, cq[(ee, qq)], my_cq)

    @pl.when(jnp.logical_and(c == 0, s == 0))
    def _():
        for ch in range(NBP // L):
            bv = lax.iota(jnp.int32, L) + ch * L
            acc = jnp.zeros((L,), jnp.int32)
            for ee in range(E - 1):
                acc = acc + (bv >= endblk[ee]).astype(jnp.int32)
            bev[pl.ds(ch * L, L)] = acc
        pltpu.sync_copy(bev, bexp_hbm.at[:])

    zv = jnp.zeros((L,), jnp.int32)
    for i in range(QPAIR // L):
        zb[pl.ds(i * L, L)] = zv
        cmp_tok[pl.ds(i * L, L)] = zv

    def pass2(i, cnt2):
        chunk = tiv[pl.ds(i * L, L)]
        mask = chunk == e
        mi = mask.astype(jnp.int32)
        within = plsc.cumsum(mi) - 1
        posv = my_base + cnt2 + within
        posbuf[pl.ds(i * L, L)] = jnp.where(mask, posv, 0)
        loc = cnt2 + within
        tok = (q * QPAIR + i * L + lax.iota(jnp.int32, L)) // TK
        plsc.store_scatter(cmp_tok, [loc], tok, mask=mask)
        plsc.store_scatter(cmp_w, [loc], twv[pl.ds(i * L, L)], mask=mask)
        return cnt2 + jnp.sum(mi)

    lax.fori_loop(0, QPAIR // L, pass2, jnp.int32(0))

    def gstep(j, carry):
        idxsl = cmp_tok.at[pl.ds(j * L, L)]
        pltpu.async_copy(h2_hbm.at[idxsl], rows, sem).wait()
        roff = pl.multiple_of(my_base + j * L, L)
        pltpu.sync_copy(rows, hd_hbm.at[pl.ds(roff, L)])
        pltpu.sync_copy(cmp_w.at[pl.ds(j * L, L)],
                        ws_hbm.at[pl.ds(roff, L)])
        return carry

    lax.fori_loop(0, my_cq // L, gstep, jnp.int32(0))

    pltpu.sync_copy(posbuf, pos_hbm.at[e, pl.ds(qoff, QPAIR)])


def _k4(ti_flat, tw_flat, h2, counts):
    mesh = plsc.VectorSubcoreMesh(core_axis_name="c", subcore_axis_name="s")
    kfn = pl.kernel(
        _k4_kernel,
        mesh=mesh,
        out_type=[
            jax.ShapeDtypeStruct((P, D), jnp.float32),
            jax.ShapeDtypeStruct((P,), jnp.float32),
            jax.ShapeDtypeStruct((E, S * TK), jnp.int32),
            jax.ShapeDtypeStruct((NBP,), jnp.int32),
        ],
        compiler_params=pltpu.CompilerParams(needs_layout_passes=False),
        scratch_types=[
            pltpu.VMEM((QPAIR,), jnp.int32),
            pltpu.VMEM((QPAIR,), jnp.float32),
            pltpu.VMEM((QPAIR,), jnp.int32),
            pltpu.VMEM((QPAIR,), jnp.int32),
            pltpu.VMEM((QPAIR,), jnp.float32),
            pltpu.VMEM((2 * NQ4, 1, 64), jnp.int32),
            pltpu.VMEM((QPAIR,), jnp.int32),
            pltpu.VMEM((L, D), jnp.float32),
            pltpu.VMEM((NBP,), jnp.int32),
            pltpu.SemaphoreType.DMA,
        ],
    )
    return kfn(ti_flat, tw_flat, h2, counts)


def _k5_body(bexp_ref, hd_ref, ws_ref, wg_ref, wu_ref, wd_ref, y_ref):
    hd = hd_ref[...]
    g = jnp.dot(hd, wg_ref[0], preferred_element_type=jnp.float32)
    u = jnp.dot(hd, wu_ref[0], preferred_element_type=jnp.float32)
    hh = (g * (1.0 / (1.0 + jnp.exp(-g)))) * u
    y = jnp.dot(hh, wd_ref[0], preferred_element_type=jnp.float32)
    y_ref[...] = y * ws_ref[...]


def _k5(hd, ws, bexp, w_gate, w_up, w_down):
    grid_spec = pltpu.PrefetchScalarGridSpec(
        num_scalar_prefetch=1,
        grid=(NB,),
        in_specs=[
            pl.BlockSpec((BLK, D), lambda b, be: (b, 0)),
            pl.BlockSpec((BLK, 1), lambda b, be: (b, 0)),
            pl.BlockSpec((1, D, F), lambda b, be: (be[b], 0, 0)),
            pl.BlockSpec((1, D, F), lambda b, be: (be[b], 0, 0)),
            pl.BlockSpec((1, F, D), lambda b, be: (be[b], 0, 0)),
        ],
        out_specs=pl.BlockSpec((BLK, D), lambda b, be: (b, 0)),
    )
    return pl.pallas_call(
        _k5_body,
        grid_spec=grid_spec,
        out_shape=jax.ShapeDtypeStruct((P, D), jnp.float32),
    )(bexp, hd, ws.reshape(P, 1), w_gate, w_up, w_down)


TPT = S // 32
CH6 = 8


def _k6_kernel(y_hbm, pos_hbm, x2_hbm, out_hbm, pidx, pparts, ybuf, xv, ov, sem):
    wid = lax.axis_index("c") * 16 + lax.axis_index("s")
    t0 = pl.multiple_of(wid * TPT, TPT)
    poff = pl.multiple_of(t0 * TK, TPT * TK)
    pltpu.sync_copy(pos_hbm.at[:, pl.ds(poff, TPT * TK)], pparts)
    npc = (TPT * TK) // L

    def sum_parts(i, carry):
        acc = pparts[0, pl.ds(i * L, L)]
        for ee in range(1, E):
            acc = acc + pparts[ee, pl.ds(i * L, L)]
        pidx[pl.ds(i * L, L)] = acc
        return carry

    lax.fori_loop(0, npc, sum_parts, jnp.int32(0))
    for ch in range(TPT // CH6):
        idxsl = pidx.at[pl.ds(ch * CH6 * TK, L)]
        pltpu.async_copy(y_hbm.at[idxsl], ybuf, sem).wait()
        pltpu.sync_copy(x2_hbm.at[pl.ds(pl.multiple_of(t0 + ch * CH6, CH6), CH6)], xv)

        def body(j, carry):
            sl = pl.ds(j * L, L)
            for tt in range(CH6):
                ov[tt, sl] = xv[tt, sl] + ybuf[2 * tt, sl] + ybuf[2 * tt + 1, sl]
            return carry

        lax.fori_loop(0, D // L, body, jnp.int32(0))
        pltpu.sync_copy(ov, out_hbm.at[pl.ds(pl.multiple_of(t0 + ch * CH6, CH6), CH6)])


def _k6(y, pos, x2):
    mesh = plsc.VectorSubcoreMesh(core_axis_name="c", subcore_axis_name="s")
    kfn = pl.kernel(
        _k6_kernel,
        mesh=mesh,
        out_type=jax.ShapeDtypeStruct((S, D), jnp.float32),
        compiler_params=pltpu.CompilerParams(needs_layout_passes=False),
        scratch_types=[
            pltpu.VMEM((TPT * TK,), jnp.int32),
            pltpu.VMEM((E, TPT * TK), jnp.int32),
            pltpu.VMEM((L, D), jnp.float32),
            pltpu.VMEM((CH6, D), jnp.float32),
            pltpu.VMEM((CH6, D), jnp.float32),
            pltpu.SemaphoreType.DMA,
        ],
    )
    return kfn(y, pos, x2)


def kernel(hidden_states, start_pos, position_embeddings, attention_mask,
           wq, wk, wv, wo, q_norm_w, k_norm_w, ln1_w, ln2_w,
           gate_w, w_gate, w_up, w_down):
    x = hidden_states.reshape(S, D)
    cos = position_embeddings[0]
    sin = position_embeddings[1]
    wqkv = jnp.concatenate([wq, wk, wv], axis=1)
    q, k, v = _k1(x, wqkv, cos, sin,
                  ln1_w.reshape(1, D), q_norm_w.reshape(1, DH),
                  k_norm_w.reshape(1, DH))
    o = _k2(q, k, v)
    x2, h2, ti, tw, counts = _k3(x, o, wo, ln2_w.reshape(1, D), gate_w)
    hd, ws, pos, bexp = _k4(ti.reshape(S * TK), tw.reshape(S * TK),
                            h2, counts)
    y = _k5(hd, ws, bexp, w_gate, w_up, w_down)
    out = _k6(y, pos, x2)
    return out.reshape(B, S, D)


# speedup vs baseline: 1.4026x; 1.0195x over previous
import functools
import jax
import jax.numpy as jnp
from jax import lax
from jax.experimental import pallas as pl
from jax.experimental.pallas import tpu as pltpu
from jax.experimental.pallas import tpu_sc as plsc

B, S, D = 1, 2048, 2048
H, KV, DH = 16, 4, 128
E, TK, F = 8, 2, 768
EPS = 1e-6
SCALE = DH ** -0.5

BS1 = 256
BQ = 1024
BK = 1024
BS3 = 256
BM5 = 256


def _rms_in(x, w):
    v = jnp.mean(jnp.square(x), axis=-1, keepdims=True)
    return w * (x * lax.rsqrt(v + EPS))


def _rot_half(x):
    h = x.shape[-1] // 2
    return jnp.concatenate([-x[:, h:], x[:, :h]], axis=-1)


DK1 = 512
KC1 = D // DK1
QKVW = (H + 2 * KV) * DH


def _k1_body(x_ref, w_ref, cos_ref, sin_ref,
             ln1_ref, qn_ref, kn_ref, q_ref, k_ref, v_ref, acc_ref):
    kc = pl.program_id(1)
    x = x_ref[...]
    v = jnp.mean(jnp.square(x), axis=-1, keepdims=True)
    scale = lax.rsqrt(v + EPS)
    xs = x_ref[:, pl.ds(kc * DK1, DK1)]
    ws = ln1_ref[:, pl.ds(kc * DK1, DK1)]
    h = xs * scale * ws
    part = jnp.dot(h, w_ref[...], preferred_element_type=jnp.float32)

    @pl.when(kc == 0)
    def _():
        acc_ref[...] = part

    @pl.when(kc != 0)
    def _():
        acc_ref[...] = acc_ref[...] + part

    @pl.when(kc == KC1 - 1)
    def _():
        qkv = acc_ref[...]
        cos = cos_ref[...]
        sin = sin_ref[...]
        for hh in range(H):
            qh = _rms_in(qkv[:, hh * DH:(hh + 1) * DH], qn_ref[...])
            q_ref[hh, :, :] = qh * cos + _rot_half(qh) * sin
        for g in range(KV):
            kh = _rms_in(qkv[:, (H + g) * DH:(H + g + 1) * DH], kn_ref[...])
            k_ref[g, :, :] = kh * cos + _rot_half(kh) * sin
            v_ref[g, :, :] = qkv[:, (H + KV + g) * DH:(H + KV + g + 1) * DH]


def _k1(x, wqkv, cos, sin, ln1_w, qn_w, kn_w):
    n = S // BS1
    return pl.pallas_call(
        _k1_body,
        grid=(n, KC1),
        in_specs=[
            pl.BlockSpec((BS1, D), lambda i, kc: (i, 0)),
            pl.BlockSpec((DK1, QKVW), lambda i, kc: (kc, 0)),
            pl.BlockSpec((BS1, DH), lambda i, kc: (i, 0)),
            pl.BlockSpec((BS1, DH), lambda i, kc: (i, 0)),
            pl.BlockSpec((1, D), lambda i, kc: (0, 0)),
            pl.BlockSpec((1, DH), lambda i, kc: (0, 0)),
            pl.BlockSpec((1, DH), lambda i, kc: (0, 0)),
        ],
        out_specs=[
            pl.BlockSpec((H, BS1, DH), lambda i, kc: (0, i, 0)),
            pl.BlockSpec((KV, BS1, DH), lambda i, kc: (0, i, 0)),
            pl.BlockSpec((KV, BS1, DH), lambda i, kc: (0, i, 0)),
        ],
        out_shape=[
            jax.ShapeDtypeStruct((H, S, DH), jnp.float32),
            jax.ShapeDtypeStruct((KV, S, DH), jnp.float32),
            jax.ShapeDtypeStruct((KV, S, DH), jnp.float32),
        ],
        scratch_shapes=[pltpu.VMEM((BS1, QKVW), jnp.float32)],
    )(x, wqkv, cos, sin, ln1_w, qn_w, kn_w)


def _k2_body(q_ref, k_ref, v_ref, o_ref):
    iq = pl.program_id(1)
    q = q_ref[0] * SCALE
    row = iq * BQ + lax.broadcasted_iota(jnp.int32, (BQ, BK), 0)

    def step(j, carry):
        m, l, acc = carry
        kj = k_ref[0, pl.ds(j * BK, BK), :]
        vj = v_ref[0, pl.ds(j * BK, BK), :]
        s = lax.dot_general(q, kj, (((1,), (1,)), ((), ())),
                            preferred_element_type=jnp.float32)
        col = j * BK + lax.broadcasted_iota(jnp.int32, (BQ, BK), 1)
        s = jnp.where(col <= row, s, -1e30)
        mnew = jnp.maximum(m, jnp.max(s, axis=-1, keepdims=True))
        p = jnp.exp(s - mnew)
        corr = jnp.exp(m - mnew)
        l = l * corr + jnp.sum(p, axis=-1, keepdims=True)
        acc = acc * corr + jnp.dot(p, vj, preferred_element_type=jnp.float32)
        return m * 0 + mnew, l, acc

    m0 = jnp.full((BQ, 1), -1e30, jnp.float32)
    l0 = jnp.zeros((BQ, 1), jnp.float32)
    a0 = jnp.zeros((BQ, DH), jnp.float32)
    m, l, acc = lax.fori_loop(0, (iq * BQ) // BK + 1, step, (m0, l0, a0))
    o_ref[0] = acc / l


def _k2(q, k, v):
    nq = S // BQ
    return pl.pallas_call(
        _k2_body,
        grid=(H, nq),
        in_specs=[
            pl.BlockSpec((1, BQ, DH), lambda h, i: (h, i, 0)),
            pl.BlockSpec((1, S, DH), lambda h, i: (h // (H // KV), 0, 0)),
            pl.BlockSpec((1, S, DH), lambda h, i: (h // (H // KV), 0, 0)),
        ],
        out_specs=pl.BlockSpec((1, BQ, DH), lambda h, i: (h, i, 0)),
        out_shape=jax.ShapeDtypeStruct((H, S, DH), jnp.float32),
    )(q, k, v)


HG3 = 4
KC3 = H // HG3


def _k3_body(x_ref, o_ref, wo_ref, ln2_ref, gw_ref,
             x2_ref, h2_ref, ti_ref, tw_ref, cnt_ref, acc_ref):
    kc = pl.program_id(1)
    s = jnp.dot(o_ref[0], wo_ref[pl.ds(0, DH), :],
                preferred_element_type=jnp.float32)
    for hh in range(1, HG3):
        s = s + jnp.dot(o_ref[hh], wo_ref[pl.ds(hh * DH, DH), :],
                        preferred_element_type=jnp.float32)

    @pl.when(kc == 0)
    def _():
        acc_ref[...] = x_ref[...] + s

    @pl.when(kc != 0)
    def _():
        acc_ref[...] = acc_ref[...] + s

    @pl.when(kc == KC3 - 1)
    def _():
        _k3_tail(acc_ref, ln2_ref, gw_ref, x2_ref, h2_ref,
                 ti_ref, tw_ref, cnt_ref)


def _k3_tail(acc_ref, ln2_ref, gw_ref, x2_ref, h2_ref, ti_ref, tw_ref, cnt_ref):
    acc = acc_ref[...]
    x2_ref[...] = acc
    h2 = _rms_in(acc, ln2_ref[...])
    h2_ref[...] = h2
    logits = jnp.dot(h2, gw_ref[...], preferred_element_type=jnp.float32)
    iot = lax.broadcasted_iota(jnp.int32, logits.shape, 1)
    m1 = jnp.max(logits, axis=-1, keepdims=True)
    i1 = jnp.min(jnp.where(logits == m1, iot, E), axis=-1, keepdims=True)
    l2m = jnp.where(iot == i1, -jnp.inf, logits)
    m2 = jnp.max(l2m, axis=-1, keepdims=True)
    i2 = jnp.min(jnp.where(l2m == m2, iot, E), axis=-1, keepdims=True)
    w1 = 1.0 / (1.0 + jnp.exp(m2 - m1))
    w2 = 1.0 - w1
    ti_ref[...] = jnp.concatenate([i1, i2], axis=1)
    tw_ref[...] = jnp.concatenate([w1, w2], axis=1)
    iot64 = lax.broadcasted_iota(jnp.int32, (BS3, 64), 1)
    oh = (iot64 == i1).astype(jnp.int32) + (iot64 == i2).astype(jnp.int32)
    cnt_ref[...] = jnp.sum(oh, axis=0, keepdims=True).reshape(1, 1, 64)


def _k3(x, o, wo, ln2_w, gate_w):
    n = S // BS3
    return pl.pallas_call(
        _k3_body,
        grid=(n, KC3),
        in_specs=[
            pl.BlockSpec((BS3, D), lambda i, kc: (i, 0)),
            pl.BlockSpec((HG3, BS3, DH), lambda i, kc: (kc, i, 0)),
            pl.BlockSpec((HG3 * DH, D), lambda i, kc: (kc, 0)),
            pl.BlockSpec((1, D), lambda i, kc: (0, 0)),
            pl.BlockSpec((D, E), lambda i, kc: (0, 0)),
        ],
        out_specs=[
            pl.BlockSpec((BS3, D), lambda i, kc: (i, 0)),
            pl.BlockSpec((BS3, D), lambda i, kc: (i, 0)),
            pl.BlockSpec((BS3, TK), lambda i, kc: (i, 0)),
            pl.BlockSpec((BS3, TK), lambda i, kc: (i, 0)),
            pl.BlockSpec((1, 1, 64), lambda i, kc: (i, 0, 0)),
        ],
        out_shape=[
            jax.ShapeDtypeStruct((S, D), jnp.float32),
            jax.ShapeDtypeStruct((S, D), jnp.float32),
            jax.ShapeDtypeStruct((S, TK), jnp.int32),
            jax.ShapeDtypeStruct((S, TK), jnp.float32),
            jax.ShapeDtypeStruct((S // BS3, 1, 64), jnp.int32),
        ],
        scratch_shapes=[pltpu.VMEM((BS3, D), jnp.float32)],
    )(x, o, wo, ln2_w, gate_w)


BLK = 256
NQ4 = 4
QTOK = S // NQ4
QPAIR = QTOK * TK
P = 6656
NB = P // BLK
NBP = 32
L = 16


def _extract(vec, lane):
    return jnp.sum(jnp.where(lax.iota(jnp.int32, L) == lane, vec, 0))


def _k4_kernel(ti_hbm, tw_hbm, h2_hbm, counts_hbm,
               hd_hbm, ws_hbm, pos_hbm, bexp_hbm,
               tiv, twv, posbuf, cmp_tok, cmp_w, cvm, zb, rows, bev, sem):
    c = lax.axis_index("c")
    s = lax.axis_index("s")
    e = s % E
    ql = s // E
    q = 2 * c + ql

    pltpu.sync_copy(counts_hbm.at[:], cvm)
    qoff = pl.multiple_of(q * QPAIR, QPAIR)
    pltpu.sync_copy(ti_hbm.at[pl.ds(qoff, QPAIR)], tiv)
    pltpu.sync_copy(tw_hbm.at[pl.ds(qoff, QPAIR)], twv)

    crow = [cvm[blk, 0, pl.ds(0, L)] for blk in range(2 * NQ4)]
    cq = {}
    cnt = {}
    for ee in range(E):
        for qq in range(NQ4):
            cval = _extract(crow[2 * qq], ee) + _extract(crow[2 * qq + 1], ee)
            cnt[(ee, qq)] = cval
            cq[(ee, qq)] = ((cval + L - 1) // L) * L
    base = {}
    endblk = []
    running = jnp.int32(0)
    for ee in range(E):
        tot = jnp.int32(0)
        for qq in range(NQ4):
            base[(ee, qq)] = running * BLK + tot
            tot = tot + cq[(ee, qq)]
        running = running + (tot + BLK - 1) // BLK
        endblk.append(running)

    my_base = jnp.int32(0)
    my_cnt = jnp.int32(0)
    my_cq = jnp.int32(0)
    for ee in range(E):
        for qq in range(NQ4):
            sel = jnp.logical_and(e == ee, q == qq)
            my_base = jnp.where(sel, base[(ee, qq)], my_base)
            my_cnt = jnp.where(sel, cnt[(ee, qq)], my_cnt)
            my_cq = jnp.where(sel, cq[(ee, qq)], my_cq)

    @pl.when(jnp.logical_and(c == 0, s == 0))
    def _():
        for ch in range(NBP // L):
            bv = lax.iota(jnp.int32, L) + ch * L
            acc = jnp.zeros((L,), jnp.int32)
            for ee in range(E - 1):
                acc = acc + (bv >= endblk[ee]).astype(jnp.int32)
            bev[pl.ds(ch * L, L)] = acc
        pltpu.sync_copy(bev, bexp_hbm.at[:])

    zv = jnp.zeros((L,), jnp.int32)
    for i in range(QPAIR // L):
        zb[pl.ds(i * L, L)] = zv
        cmp_tok[pl.ds(i * L, L)] = zv

    def pass2(i, cnt2):
        chunk = tiv[pl.ds(i * L, L)]
        mask = chunk == e
        mi = mask.astype(jnp.int32)
        within = plsc.cumsum(mi) - 1
        posv = my_base + cnt2 + within
        posbuf[pl.ds(i * L, L)] = jnp.where(mask, posv, 0)
        loc = cnt2 + within
        tok = (q * QPAIR + i * L + lax.iota(jnp.int32, L)) // TK
        plsc.store_scatter(cmp_tok, [loc], tok, mask=mask)
        plsc.store_scatter(cmp_w, [loc], twv[pl.ds(i * L, L)], mask=mask)
        return cnt2 + jnp.sum(mi)

    lax.fori_loop(0, QPAIR // L, pass2, jnp.int32(0))

    def gstep(j, carry):
        idxsl = cmp_tok.at[pl.ds(j * L, L)]
        pltpu.async_copy(h2_hbm.at[idxsl], rows, sem).wait()
        roff = pl.multiple_of(my_base + j * L, L)
        pltpu.sync_copy(rows, hd_hbm.at[pl.ds(roff, L)])
        pltpu.sync_copy(cmp_w.at[pl.ds(j * L, L)],
                        ws_hbm.at[pl.ds(roff, L)])
        return carry

    lax.fori_loop(0, my_cq // L, gstep, jnp.int32(0))

    pltpu.sync_copy(posbuf, pos_hbm.at[e, pl.ds(qoff, QPAIR)])


def _k4(ti_flat, tw_flat, h2, counts):
    mesh = plsc.VectorSubcoreMesh(core_axis_name="c", subcore_axis_name="s")
    kfn = pl.kernel(
        _k4_kernel,
        mesh=mesh,
        out_type=[
            jax.ShapeDtypeStruct((P, D), jnp.float32),
            jax.ShapeDtypeStruct((P,), jnp.float32),
            jax.ShapeDtypeStruct((E, S * TK), jnp.int32),
            jax.ShapeDtypeStruct((NBP,), jnp.int32),
        ],
        compiler_params=pltpu.CompilerParams(needs_layout_passes=False),
        scratch_types=[
            pltpu.VMEM((QPAIR,), jnp.int32),
            pltpu.VMEM((QPAIR,), jnp.float32),
            pltpu.VMEM((QPAIR,), jnp.int32),
            pltpu.VMEM((QPAIR,), jnp.int32),
            pltpu.VMEM((QPAIR,), jnp.float32),
            pltpu.VMEM((2 * NQ4, 1, 64), jnp.int32),
            pltpu.VMEM((QPAIR,), jnp.int32),
            pltpu.VMEM((L, D), jnp.float32),
            pltpu.VMEM((NBP,), jnp.int32),
            pltpu.SemaphoreType.DMA,
        ],
    )
    return kfn(ti_flat, tw_flat, h2, counts)


def _k5_body(bexp_ref, hd_ref, ws_ref, wg_ref, wu_ref, wd_ref, y_ref):
    hd = hd_ref[...]
    g = jnp.dot(hd, wg_ref[0], preferred_element_type=jnp.float32)
    u = jnp.dot(hd, wu_ref[0], preferred_element_type=jnp.float32)
    hh = (g * (1.0 / (1.0 + jnp.exp(-g)))) * u
    y = jnp.dot(hh, wd_ref[0], preferred_element_type=jnp.float32)
    y_ref[...] = y * ws_ref[...]


def _k5(hd, ws, bexp, w_gate, w_up, w_down):
    grid_spec = pltpu.PrefetchScalarGridSpec(
        num_scalar_prefetch=1,
        grid=(NB,),
        in_specs=[
            pl.BlockSpec((BLK, D), lambda b, be: (b, 0)),
            pl.BlockSpec((BLK, 1), lambda b, be: (b, 0)),
            pl.BlockSpec((1, D, F), lambda b, be: (be[b], 0, 0)),
            pl.BlockSpec((1, D, F), lambda b, be: (be[b], 0, 0)),
            pl.BlockSpec((1, F, D), lambda b, be: (be[b], 0, 0)),
        ],
        out_specs=pl.BlockSpec((BLK, D), lambda b, be: (b, 0)),
    )
    return pl.pallas_call(
        _k5_body,
        grid_spec=grid_spec,
        out_shape=jax.ShapeDtypeStruct((P, D), jnp.float32),
    )(bexp, hd, ws.reshape(P, 1), w_gate, w_up, w_down)


TPT = S // 32
CH6 = 8


def _k6_kernel(y_hbm, pos_hbm, x2_hbm, out_hbm, pidx, pparts, ybuf, xv, ov, sem):
    wid = lax.axis_index("c") * 16 + lax.axis_index("s")
    t0 = pl.multiple_of(wid * TPT, TPT)
    poff = pl.multiple_of(t0 * TK, TPT * TK)
    pltpu.sync_copy(pos_hbm.at[:, pl.ds(poff, TPT * TK)], pparts)
    npc = (TPT * TK) // L

    def sum_parts(i, carry):
        acc = pparts[0, pl.ds(i * L, L)]
        for ee in range(1, E):
            acc = acc + pparts[ee, pl.ds(i * L, L)]
        pidx[pl.ds(i * L, L)] = acc
        return carry

    lax.fori_loop(0, npc, sum_parts, jnp.int32(0))
    for ch in range(TPT // CH6):
        idxsl = pidx.at[pl.ds(ch * CH6 * TK, L)]
        pltpu.async_copy(y_hbm.at[idxsl], ybuf, sem).wait()
        pltpu.sync_copy(x2_hbm.at[pl.ds(pl.multiple_of(t0 + ch * CH6, CH6), CH6)], xv)

        def body(j, carry):
            sl = pl.ds(j * L, L)
            for tt in range(CH6):
                ov[tt, sl] = xv[tt, sl] + ybuf[2 * tt, sl] + ybuf[2 * tt + 1, sl]
            return carry

        lax.fori_loop(0, D // L, body, jnp.int32(0))
        pltpu.sync_copy(ov, out_hbm.at[pl.ds(pl.multiple_of(t0 + ch * CH6, CH6), CH6)])


def _k6(y, pos, x2):
    mesh = plsc.VectorSubcoreMesh(core_axis_name="c", subcore_axis_name="s")
    kfn = pl.kernel(
        _k6_kernel,
        mesh=mesh,
        out_type=jax.ShapeDtypeStruct((S, D), jnp.float32),
        compiler_params=pltpu.CompilerParams(needs_layout_passes=False),
        scratch_types=[
            pltpu.VMEM((TPT * TK,), jnp.int32),
            pltpu.VMEM((E, TPT * TK), jnp.int32),
            pltpu.VMEM((L, D), jnp.float32),
            pltpu.VMEM((CH6, D), jnp.float32),
            pltpu.VMEM((CH6, D), jnp.float32),
            pltpu.SemaphoreType.DMA,
        ],
    )
    return kfn(y, pos, x2)


def kernel(hidden_states, start_pos, position_embeddings, attention_mask,
           wq, wk, wv, wo, q_norm_w, k_norm_w, ln1_w, ln2_w,
           gate_w, w_gate, w_up, w_down):
    x = hidden_states.reshape(S, D)
    cos = position_embeddings[0]
    sin = position_embeddings[1]
    wqkv = jnp.concatenate([wq, wk, wv], axis=1)
    q, k, v = _k1(x, wqkv, cos, sin,
                  ln1_w.reshape(1, D), q_norm_w.reshape(1, DH),
                  k_norm_w.reshape(1, DH))
    o = _k2(q, k, v)
    x2, h2, ti, tw, counts = _k3(x, o, wo, ln2_w.reshape(1, D), gate_w)
    hd, ws, pos, bexp = _k4(ti.reshape(S * TK), tw.reshape(S * TK),
                            h2, counts)
    y = _k5(hd, ws, bexp, w_gate, w_up, w_down)
    out = _k6(y, pos, x2)
    return out.reshape(B, S, D)


# speedup vs baseline: 1.4246x; 1.0157x over previous
import functools
import jax
import jax.numpy as jnp
from jax import lax
from jax.experimental import pallas as pl
from jax.experimental.pallas import tpu as pltpu
from jax.experimental.pallas import tpu_sc as plsc

B, S, D = 1, 2048, 2048
H, KV, DH = 16, 4, 128
E, TK, F = 8, 2, 768
EPS = 1e-6
SCALE = DH ** -0.5

BS1 = 256
BQ = 1024
BK = 1024
BS3 = 256
BM5 = 256


def _rms_in(x, w):
    v = jnp.mean(jnp.square(x), axis=-1, keepdims=True)
    return w * (x * lax.rsqrt(v + EPS))


def _rot_half(x):
    h = x.shape[-1] // 2
    return jnp.concatenate([-x[:, h:], x[:, :h]], axis=-1)


DK1 = 512
KC1 = D // DK1
QKVW = (H + 2 * KV) * DH


def _k1_body(x_ref, w_ref, qkv_ref, acc_ref, ssq_ref):
    kc = pl.program_id(1)
    xs = x_ref[...]
    part = jnp.dot(xs, w_ref[...], preferred_element_type=jnp.float32)
    ssq = jnp.sum(xs * xs, axis=-1, keepdims=True)

    @pl.when(kc == 0)
    def _():
        acc_ref[...] = part
        ssq_ref[...] = ssq

    @pl.when(kc != 0)
    def _():
        acc_ref[...] = acc_ref[...] + part
        ssq_ref[...] = ssq_ref[...] + ssq

    @pl.when(kc == KC1 - 1)
    def _():
        scale = lax.rsqrt(ssq_ref[...] / D + EPS)
        qkv_ref[...] = acc_ref[...] * scale


def _k1(x, wqkv):
    n = S // BS1
    return pl.pallas_call(
        _k1_body,
        grid=(n, KC1),
        in_specs=[
            pl.BlockSpec((BS1, DK1), lambda i, kc: (i, kc)),
            pl.BlockSpec((DK1, QKVW), lambda i, kc: (kc, 0)),
        ],
        out_specs=pl.BlockSpec((BS1, QKVW), lambda i, kc: (i, 0)),
        out_shape=jax.ShapeDtypeStruct((S, QKVW), jnp.float32),
        scratch_shapes=[pltpu.VMEM((BS1, QKVW), jnp.float32),
                        pltpu.VMEM((BS1, 1), jnp.float32)],
    )(x, wqkv)


def _k2_body(q_ref, k_ref, v_ref, cos_ref, sin_ref, qn_ref, kn_ref,
             o_ref, kr_ref):
    h = pl.program_id(0)
    iq = pl.program_id(1)

    @pl.when(jnp.logical_and(iq == 0, h % (H // KV) == 0))
    def _():
        km = _rms_in(k_ref[...], kn_ref[...])
        kr_ref[...] = km * cos_ref[...] + _rot_half(km) * sin_ref[...]

    qm = _rms_in(q_ref[...], qn_ref[...])
    cosq = cos_ref[pl.ds(iq * BQ, BQ), :]
    sinq = sin_ref[pl.ds(iq * BQ, BQ), :]
    q = (qm * cosq + _rot_half(qm) * sinq) * SCALE
    row = iq * BQ + lax.broadcasted_iota(jnp.int32, (BQ, BK), 0)

    def step(j, carry):
        m, l, acc = carry
        kj = kr_ref[pl.ds(j * BK, BK), :]
        vj = v_ref[pl.ds(j * BK, BK), :]
        s = lax.dot_general(q, kj, (((1,), (1,)), ((), ())),
                            preferred_element_type=jnp.float32)
        col = j * BK + lax.broadcasted_iota(jnp.int32, (BQ, BK), 1)
        s = jnp.where(col <= row, s, -1e30)
        mnew = jnp.maximum(m, jnp.max(s, axis=-1, keepdims=True))
        p = jnp.exp(s - mnew)
        corr = jnp.exp(m - mnew)
        l = l * corr + jnp.sum(p, axis=-1, keepdims=True)
        acc = acc * corr + jnp.dot(p, vj, preferred_element_type=jnp.float32)
        return m * 0 + mnew, l, acc

    m0 = jnp.full((BQ, 1), -1e30, jnp.float32)
    l0 = jnp.zeros((BQ, 1), jnp.float32)
    a0 = jnp.zeros((BQ, DH), jnp.float32)
    m, l, acc = lax.fori_loop(0, (iq * BQ) // BK + 1, step, (m0, l0, a0))
    o_ref[0] = acc / l


def _k2(qkv, cos, sin, qn_w, kn_w):
    nq = S // BQ
    grp = H // KV
    return pl.pallas_call(
        _k2_body,
        grid=(H, nq),
        in_specs=[
            pl.BlockSpec((BQ, DH), lambda h, i: (i, h)),
            pl.BlockSpec((S, DH), lambda h, i: (0, H + h // grp)),
            pl.BlockSpec((S, DH), lambda h, i: (0, H + KV + h // grp)),
            pl.BlockSpec((S, DH), lambda h, i: (0, 0)),
            pl.BlockSpec((S, DH), lambda h, i: (0, 0)),
            pl.BlockSpec((1, DH), lambda h, i: (0, 0)),
            pl.BlockSpec((1, DH), lambda h, i: (0, 0)),
        ],
        out_specs=pl.BlockSpec((1, BQ, DH), lambda h, i: (h, i, 0)),
        out_shape=jax.ShapeDtypeStruct((H, S, DH), jnp.float32),
        scratch_shapes=[pltpu.VMEM((S, DH), jnp.float32)],
    )(qkv, qkv, qkv, cos, sin, qn_w, kn_w)


HG3 = 4
KC3 = H // HG3


def _k3_body(x_ref, o_ref, wo_ref, ln2_ref, gw_ref,
             x2_ref, h2_ref, ti_ref, tw_ref, cnt_ref, acc_ref):
    kc = pl.program_id(1)
    s = jnp.dot(o_ref[0], wo_ref[pl.ds(0, DH), :],
                preferred_element_type=jnp.float32)
    for hh in range(1, HG3):
        s = s + jnp.dot(o_ref[hh], wo_ref[pl.ds(hh * DH, DH), :],
                        preferred_element_type=jnp.float32)

    @pl.when(kc == 0)
    def _():
        acc_ref[...] = x_ref[...] + s

    @pl.when(kc != 0)
    def _():
        acc_ref[...] = acc_ref[...] + s

    @pl.when(kc == KC3 - 1)
    def _():
        _k3_tail(acc_ref, ln2_ref, gw_ref, x2_ref, h2_ref,
                 ti_ref, tw_ref, cnt_ref)


def _k3_tail(acc_ref, ln2_ref, gw_ref, x2_ref, h2_ref, ti_ref, tw_ref, cnt_ref):
    acc = acc_ref[...]
    x2_ref[...] = acc
    h2 = _rms_in(acc, ln2_ref[...])
    h2_ref[...] = h2
    logits = jnp.dot(h2, gw_ref[...], preferred_element_type=jnp.float32)
    iot = lax.broadcasted_iota(jnp.int32, logits.shape, 1)
    m1 = jnp.max(logits, axis=-1, keepdims=True)
    i1 = jnp.min(jnp.where(logits == m1, iot, E), axis=-1, keepdims=True)
    l2m = jnp.where(iot == i1, -jnp.inf, logits)
    m2 = jnp.max(l2m, axis=-1, keepdims=True)
    i2 = jnp.min(jnp.where(l2m == m2, iot, E), axis=-1, keepdims=True)
    w1 = 1.0 / (1.0 + jnp.exp(m2 - m1))
    w2 = 1.0 - w1
    ti_ref[...] = jnp.concatenate([i1, i2], axis=1)
    tw_ref[...] = jnp.concatenate([w1, w2], axis=1)
    iot64 = lax.broadcasted_iota(jnp.int32, (BS3, 64), 1)
    oh = (iot64 == i1).astype(jnp.int32) + (iot64 == i2).astype(jnp.int32)
    cnt_ref[...] = jnp.sum(oh, axis=0, keepdims=True).reshape(1, 1, 64)


def _k3(x, o, wo, ln2_w, gate_w):
    n = S // BS3
    return pl.pallas_call(
        _k3_body,
        grid=(n, KC3),
        in_specs=[
            pl.BlockSpec((BS3, D), lambda i, kc: (i, 0)),
            pl.BlockSpec((HG3, BS3, DH), lambda i, kc: (kc, i, 0)),
            pl.BlockSpec((HG3 * DH, D), lambda i, kc: (kc, 0)),
            pl.BlockSpec((1, D), lambda i, kc: (0, 0)),
            pl.BlockSpec((D, E), lambda i, kc: (0, 0)),
        ],
        out_specs=[
            pl.BlockSpec((BS3, D), lambda i, kc: (i, 0)),
            pl.BlockSpec((BS3, D), lambda i, kc: (i, 0)),
            pl.BlockSpec((BS3, TK), lambda i, kc: (i, 0)),
            pl.BlockSpec((BS3, TK), lambda i, kc: (i, 0)),
            pl.BlockSpec((1, 1, 64), lambda i, kc: (i, 0, 0)),
        ],
        out_shape=[
            jax.ShapeDtypeStruct((S, D), jnp.float32),
            jax.ShapeDtypeStruct((S, D), jnp.float32),
            jax.ShapeDtypeStruct((S, TK), jnp.int32),
            jax.ShapeDtypeStruct((S, TK), jnp.float32),
            jax.ShapeDtypeStruct((S // BS3, 1, 64), jnp.int32),
        ],
        scratch_shapes=[pltpu.VMEM((BS3, D), jnp.float32)],
    )(x, o, wo, ln2_w, gate_w)


BLK = 256
NQ4 = 4
QTOK = S // NQ4
QPAIR = QTOK * TK
P = 6656
NB = P // BLK
NBP = 32
L = 16


def _extract(vec, lane):
    return jnp.sum(jnp.where(lax.iota(jnp.int32, L) == lane, vec, 0))


def _k4_kernel(ti_hbm, tw_hbm, h2_hbm, counts_hbm,
               hd_hbm, ws_hbm, pos_hbm, bexp_hbm,
               tiv, twv, posbuf, cmp_tok, cmp_w, cvm, zb, rows, bev, sem):
    c = lax.axis_index("c")
    s = lax.axis_index("s")
    e = s % E
    ql = s // E
    q = 2 * c + ql

    pltpu.sync_copy(counts_hbm.at[:], cvm)
    qoff = pl.multiple_of(q * QPAIR, QPAIR)
    pltpu.sync_copy(ti_hbm.at[pl.ds(qoff, QPAIR)], tiv)
    pltpu.sync_copy(tw_hbm.at[pl.ds(qoff, QPAIR)], twv)

    crow = [cvm[blk, 0, pl.ds(0, L)] for blk in range(2 * NQ4)]
    cq = {}
    cnt = {}
    for ee in range(E):
        for qq in range(NQ4):
            cval = _extract(crow[2 * qq], ee) + _extract(crow[2 * qq + 1], ee)
            cnt[(ee, qq)] = cval
            cq[(ee, qq)] = ((cval + L - 1) // L) * L
    base = {}
    endblk = []
    running = jnp.int32(0)
    for ee in range(E):
        tot = jnp.int32(0)
        for qq in range(NQ4):
            base[(ee, qq)] = running * BLK + tot
            tot = tot + cq[(ee, qq)]
        running = running + (tot + BLK - 1) // BLK
        endblk.append(running)

    my_base = jnp.int32(0)
    my_cnt = jnp.int32(0)
    my_cq = jnp.int32(0)
    for ee in range(E):
        for qq in range(NQ4):
            sel = jnp.logical_and(e == ee, q == qq)
            my_base = jnp.where(sel, base[(ee, qq)], my_base)
            my_cnt = jnp.where(sel, cnt[(ee, qq)], my_cnt)
            my_cq = jnp.where(sel, cq[(ee, qq)], my_cq)

    @pl.when(jnp.logical_and(c == 0, s == 0))
    def _():
        for ch in range(NBP // L):
            bv = lax.iota(jnp.int32, L) + ch * L
            acc = jnp.zeros((L,), jnp.int32)
            for ee in range(E - 1):
                acc = acc + (bv >= endblk[ee]).astype(jnp.int32)
            bev[pl.ds(ch * L, L)] = acc
        pltpu.sync_copy(bev, bexp_hbm.at[:])

    zv = jnp.zeros((L,), jnp.int32)
    for i in range(QPAIR // L):
        zb[pl.ds(i * L, L)] = zv
        cmp_tok[pl.ds(i * L, L)] = zv

    def pass2(i, cnt2):
        chunk = tiv[pl.ds(i * L, L)]
        mask = chunk == e
        mi = mask.astype(jnp.int32)
        within = plsc.cumsum(mi) - 1
        posv = my_base + cnt2 + within
        posbuf[pl.ds(i * L, L)] = jnp.where(mask, posv, 0)
        loc = cnt2 + within
        tok = (q * QPAIR + i * L + lax.iota(jnp.int32, L)) // TK
        plsc.store_scatter(cmp_tok, [loc], tok, mask=mask)
        plsc.store_scatter(cmp_w, [loc], twv[pl.ds(i * L, L)], mask=mask)
        return cnt2 + jnp.sum(mi)

    lax.fori_loop(0, QPAIR // L, pass2, jnp.int32(0))

    def gstep(j, carry):
        idxsl = cmp_tok.at[pl.ds(j * L, L)]
        pltpu.async_copy(h2_hbm.at[idxsl], rows, sem).wait()
        roff = pl.multiple_of(my_base + j * L, L)
        pltpu.sync_copy(rows, hd_hbm.at[pl.ds(roff, L)])
        pltpu.sync_copy(cmp_w.at[pl.ds(j * L, L)],
                        ws_hbm.at[pl.ds(roff, L)])
        return carry

    lax.fori_loop(0, my_cq // L, gstep, jnp.int32(0))

    pltpu.sync_copy(posbuf, pos_hbm.at[e, pl.ds(qoff, QPAIR)])


def _k4(ti_flat, tw_flat, h2, counts):
    mesh = plsc.VectorSubcoreMesh(core_axis_name="c", subcore_axis_name="s")
    kfn = pl.kernel(
        _k4_kernel,
        mesh=mesh,
        out_type=[
            jax.ShapeDtypeStruct((P, D), jnp.float32),
            jax.ShapeDtypeStruct((P,), jnp.float32),
            jax.ShapeDtypeStruct((E, S * TK), jnp.int32),
            jax.ShapeDtypeStruct((NBP,), jnp.int32),
        ],
        compiler_params=pltpu.CompilerParams(needs_layout_passes=False),
        scratch_types=[
            pltpu.VMEM((QPAIR,), jnp.int32),
            pltpu.VMEM((QPAIR,), jnp.float32),
            pltpu.VMEM((QPAIR,), jnp.int32),
            pltpu.VMEM((QPAIR,), jnp.int32),
            pltpu.VMEM((QPAIR,), jnp.float32),
            pltpu.VMEM((2 * NQ4, 1, 64), jnp.int32),
            pltpu.VMEM((QPAIR,), jnp.int32),
            pltpu.VMEM((L, D), jnp.float32),
            pltpu.VMEM((NBP,), jnp.int32),
            pltpu.SemaphoreType.DMA,
        ],
    )
    return kfn(ti_flat, tw_flat, h2, counts)


def _k5_body(bexp_ref, hd_ref, ws_ref, wg_ref, wu_ref, wd_ref, y_ref):
    hd = hd_ref[...]
    g = jnp.dot(hd, wg_ref[0], preferred_element_type=jnp.float32)
    u = jnp.dot(hd, wu_ref[0], preferred_element_type=jnp.float32)
    hh = (g * (1.0 / (1.0 + jnp.exp(-g)))) * u
    y = jnp.dot(hh, wd_ref[0], preferred_element_type=jnp.float32)
    y_ref[...] = y * ws_ref[...]


def _k5(hd, ws, bexp, w_gate, w_up, w_down):
    grid_spec = pltpu.PrefetchScalarGridSpec(
        num_scalar_prefetch=1,
        grid=(NB,),
        in_specs=[
            pl.BlockSpec((BLK, D), lambda b, be: (b, 0)),
            pl.BlockSpec((BLK, 1), lambda b, be: (b, 0)),
            pl.BlockSpec((1, D, F), lambda b, be: (be[b], 0, 0)),
            pl.BlockSpec((1, D, F), lambda b, be: (be[b], 0, 0)),
            pl.BlockSpec((1, F, D), lambda b, be: (be[b], 0, 0)),
        ],
        out_specs=pl.BlockSpec((BLK, D), lambda b, be: (b, 0)),
    )
    return pl.pallas_call(
        _k5_body,
        grid_spec=grid_spec,
        out_shape=jax.ShapeDtypeStruct((P, D), jnp.float32),
    )(bexp, hd, ws.reshape(P, 1), w_gate, w_up, w_down)


TPT = S // 32
CH6 = 8


def _k6_kernel(y_hbm, pos_hbm, x2_hbm, out_hbm, pidx, pparts, ybuf, xv, ov, sem):
    wid = lax.axis_index("c") * 16 + lax.axis_index("s")
    t0 = pl.multiple_of(wid * TPT, TPT)
    poff = pl.multiple_of(t0 * TK, TPT * TK)
    pltpu.sync_copy(pos_hbm.at[:, pl.ds(poff, TPT * TK)], pparts)
    npc = (TPT * TK) // L

    def sum_parts(i, carry):
        acc = pparts[0, pl.ds(i * L, L)]
        for ee in range(1, E):
            acc = acc + pparts[ee, pl.ds(i * L, L)]
        pidx[pl.ds(i * L, L)] = acc
        return carry

    lax.fori_loop(0, npc, sum_parts, jnp.int32(0))
    for ch in range(TPT // CH6):
        idxsl = pidx.at[pl.ds(ch * CH6 * TK, L)]
        pltpu.async_copy(y_hbm.at[idxsl], ybuf, sem).wait()
        pltpu.sync_copy(x2_hbm.at[pl.ds(pl.multiple_of(t0 + ch * CH6, CH6), CH6)], xv)

        def body(j, carry):
            sl = pl.ds(j * L, L)
            for tt in range(CH6):
                ov[tt, sl] = xv[tt, sl] + ybuf[2 * tt, sl] + ybuf[2 * tt + 1, sl]
            return carry

        lax.fori_loop(0, D // L, body, jnp.int32(0))
        pltpu.sync_copy(ov, out_hbm.at[pl.ds(pl.multiple_of(t0 + ch * CH6, CH6), CH6)])


def _k6(y, pos, x2):
    mesh = plsc.VectorSubcoreMesh(core_axis_name="c", subcore_axis_name="s")
    kfn = pl.kernel(
        _k6_kernel,
        mesh=mesh,
        out_type=jax.ShapeDtypeStruct((S, D), jnp.float32),
        compiler_params=pltpu.CompilerParams(needs_layout_passes=False),
        scratch_types=[
            pltpu.VMEM((TPT * TK,), jnp.int32),
            pltpu.VMEM((E, TPT * TK), jnp.int32),
            pltpu.VMEM((L, D), jnp.float32),
            pltpu.VMEM((CH6, D), jnp.float32),
            pltpu.VMEM((CH6, D), jnp.float32),
            pltpu.SemaphoreType.DMA,
        ],
    )
    return kfn(y, pos, x2)


def kernel(hidden_states, start_pos, position_embeddings, attention_mask,
           wq, wk, wv, wo, q_norm_w, k_norm_w, ln1_w, ln2_w,
           gate_w, w_gate, w_up, w_down):
    x = hidden_states.reshape(S, D)
    cos = position_embeddings[0]
    sin = position_embeddings[1]
    wqkv = ln1_w[:, None] * jnp.concatenate([wq, wk, wv], axis=1)
    qkv = _k1(x, wqkv)
    o = _k2(qkv, cos, sin, q_norm_w.reshape(1, DH), k_norm_w.reshape(1, DH))
    x2, h2, ti, tw, counts = _k3(x, o, wo, ln2_w.reshape(1, D), gate_w)
    hd, ws, pos, bexp = _k4(ti.reshape(S * TK), tw.reshape(S * TK),
                            h2, counts)
    y = _k5(hd, ws, bexp, w_gate, w_up, w_down)
    out = _k6(y, pos, x2)
    return out.reshape(B, S, D)


# speedup vs baseline: 1.4419x; 1.0121x over previous
import functools
import jax
import jax.numpy as jnp
from jax import lax
from jax.experimental import pallas as pl
from jax.experimental.pallas import tpu as pltpu
from jax.experimental.pallas import tpu_sc as plsc

B, S, D = 1, 2048, 2048
H, KV, DH = 16, 4, 128
E, TK, F = 8, 2, 768
EPS = 1e-6
SCALE = DH ** -0.5

BS1 = 256
BQ = 1024
BK = 1024
BS3 = 256
BM5 = 256


def _rms_in(x, w):
    v = jnp.mean(jnp.square(x), axis=-1, keepdims=True)
    return w * (x * lax.rsqrt(v + EPS))


def _rot_half(x):
    h = x.shape[-1] // 2
    return jnp.concatenate([-x[:, h:], x[:, :h]], axis=-1)


DK1 = 512
KC1 = D // DK1
QKVW = (H + 2 * KV) * DH


def _k1_body(x_ref, w_ref, qkv_ref, acc_ref, ssq_ref):
    kc = pl.program_id(1)
    xs = x_ref[...]
    part = jnp.dot(xs, w_ref[...], preferred_element_type=jnp.float32)
    ssq = jnp.sum(xs * xs, axis=-1, keepdims=True)

    @pl.when(kc == 0)
    def _():
        acc_ref[...] = part
        ssq_ref[...] = ssq

    @pl.when(kc != 0)
    def _():
        acc_ref[...] = acc_ref[...] + part
        ssq_ref[...] = ssq_ref[...] + ssq

    @pl.when(kc == KC1 - 1)
    def _():
        scale = lax.rsqrt(ssq_ref[...] / D + EPS)
        qkv_ref[...] = acc_ref[...] * scale


def _k1(x, wqkv):
    n = S // BS1
    return pl.pallas_call(
        _k1_body,
        grid=(n, KC1),
        in_specs=[
            pl.BlockSpec((BS1, DK1), lambda i, kc: (i, kc)),
            pl.BlockSpec((DK1, QKVW), lambda i, kc: (kc, 0)),
        ],
        out_specs=pl.BlockSpec((BS1, QKVW), lambda i, kc: (i, 0)),
        out_shape=jax.ShapeDtypeStruct((S, QKVW), jnp.float32),
        scratch_shapes=[pltpu.VMEM((BS1, QKVW), jnp.float32),
                        pltpu.VMEM((BS1, 1), jnp.float32)],
    )(x, wqkv)


def _k2_body(q_ref, k_ref, v_ref, cos_ref, sin_ref, qn_ref, kn_ref,
             o_ref, kr_ref):
    h = pl.program_id(0)
    iq = pl.program_id(1)

    @pl.when(jnp.logical_and(iq == 0, h % (H // KV) == 0))
    def _():
        km = _rms_in(k_ref[...], kn_ref[...])
        kr_ref[...] = km * cos_ref[...] + _rot_half(km) * sin_ref[...]

    qm = _rms_in(q_ref[...], qn_ref[...])
    cosq = cos_ref[pl.ds(iq * BQ, BQ), :]
    sinq = sin_ref[pl.ds(iq * BQ, BQ), :]
    q = (qm * cosq + _rot_half(qm) * sinq) * SCALE
    row = iq * BQ + lax.broadcasted_iota(jnp.int32, (BQ, BK), 0)

    def step(j, carry):
        m, l, acc = carry
        kj = kr_ref[pl.ds(j * BK, BK), :]
        vj = v_ref[pl.ds(j * BK, BK), :]
        s = lax.dot_general(q, kj, (((1,), (1,)), ((), ())),
                            preferred_element_type=jnp.float32)
        col = j * BK + lax.broadcasted_iota(jnp.int32, (BQ, BK), 1)
        s = jnp.where(col <= row, s, -1e30)
        mnew = jnp.maximum(m, jnp.max(s, axis=-1, keepdims=True))
        p = jnp.exp(s - mnew)
        corr = jnp.exp(m - mnew)
        l = l * corr + jnp.sum(p, axis=-1, keepdims=True)
        acc = acc * corr + jnp.dot(p, vj, preferred_element_type=jnp.float32)
        return m * 0 + mnew, l, acc

    m0 = jnp.full((BQ, 1), -1e30, jnp.float32)
    l0 = jnp.zeros((BQ, 1), jnp.float32)
    a0 = jnp.zeros((BQ, DH), jnp.float32)
    m, l, acc = lax.fori_loop(0, (iq * BQ) // BK + 1, step, (m0, l0, a0))
    o_ref[...] = acc / l


def _k2(qkv, cos, sin, qn_w, kn_w):
    nq = S // BQ
    grp = H // KV
    return pl.pallas_call(
        _k2_body,
        grid=(H, nq),
        in_specs=[
            pl.BlockSpec((BQ, DH), lambda h, i: (i, h)),
            pl.BlockSpec((S, DH), lambda h, i: (0, H + h // grp)),
            pl.BlockSpec((S, DH), lambda h, i: (0, H + KV + h // grp)),
            pl.BlockSpec((S, DH), lambda h, i: (0, 0)),
            pl.BlockSpec((S, DH), lambda h, i: (0, 0)),
            pl.BlockSpec((1, DH), lambda h, i: (0, 0)),
            pl.BlockSpec((1, DH), lambda h, i: (0, 0)),
        ],
        out_specs=pl.BlockSpec((BQ, DH), lambda h, i: (i, h)),
        out_shape=jax.ShapeDtypeStruct((S, H * DH), jnp.float32),
        scratch_shapes=[pltpu.VMEM((S, DH), jnp.float32)],
    )(qkv, qkv, qkv, cos, sin, qn_w, kn_w)


DKO = 512
KC3 = (H * DH) // DKO


def _k3_body(x_ref, o_ref, wo_ref, ln2_ref, gw_ref,
             x2_ref, h2_ref, ti_ref, tw_ref, cnt_ref, acc_ref):
    kc = pl.program_id(1)
    s = jnp.dot(o_ref[...], wo_ref[...], preferred_element_type=jnp.float32)

    @pl.when(kc == 0)
    def _():
        acc_ref[...] = x_ref[...] + s

    @pl.when(kc != 0)
    def _():
        acc_ref[...] = acc_ref[...] + s

    @pl.when(kc == KC3 - 1)
    def _():
        _k3_tail(acc_ref, ln2_ref, gw_ref, x2_ref, h2_ref,
                 ti_ref, tw_ref, cnt_ref)


def _k3_tail(acc_ref, ln2_ref, gw_ref, x2_ref, h2_ref, ti_ref, tw_ref, cnt_ref):
    acc = acc_ref[...]
    x2_ref[...] = acc
    h2 = _rms_in(acc, ln2_ref[...])
    h2_ref[...] = h2
    logits = jnp.dot(h2, gw_ref[...], preferred_element_type=jnp.float32)
    iot = lax.broadcasted_iota(jnp.int32, logits.shape, 1)
    m1 = jnp.max(logits, axis=-1, keepdims=True)
    i1 = jnp.min(jnp.where(logits == m1, iot, E), axis=-1, keepdims=True)
    l2m = jnp.where(iot == i1, -jnp.inf, logits)
    m2 = jnp.max(l2m, axis=-1, keepdims=True)
    i2 = jnp.min(jnp.where(l2m == m2, iot, E), axis=-1, keepdims=True)
    w1 = 1.0 / (1.0 + jnp.exp(m2 - m1))
    w2 = 1.0 - w1
    ti_ref[...] = jnp.concatenate([i1, i2], axis=1)
    tw_ref[...] = jnp.concatenate([w1, w2], axis=1)
    iot64 = lax.broadcasted_iota(jnp.int32, (BS3, 64), 1)
    oh = (iot64 == i1).astype(jnp.int32) + (iot64 == i2).astype(jnp.int32)
    cnt_ref[...] = jnp.sum(oh, axis=0, keepdims=True).reshape(1, 1, 64)


def _k3(x, o, wo, ln2_w, gate_w):
    n = S // BS3
    return pl.pallas_call(
        _k3_body,
        grid=(n, KC3),
        in_specs=[
            pl.BlockSpec((BS3, D), lambda i, kc: (i, 0)),
            pl.BlockSpec((BS3, DKO), lambda i, kc: (i, kc)),
            pl.BlockSpec((DKO, D), lambda i, kc: (kc, 0)),
            pl.BlockSpec((1, D), lambda i, kc: (0, 0)),
            pl.BlockSpec((D, E), lambda i, kc: (0, 0)),
        ],
        out_specs=[
            pl.BlockSpec((BS3, D), lambda i, kc: (i, 0)),
            pl.BlockSpec((BS3, D), lambda i, kc: (i, 0)),
            pl.BlockSpec((BS3, TK), lambda i, kc: (i, 0)),
            pl.BlockSpec((BS3, TK), lambda i, kc: (i, 0)),
            pl.BlockSpec((1, 1, 64), lambda i, kc: (i, 0, 0)),
        ],
        out_shape=[
            jax.ShapeDtypeStruct((S, D), jnp.float32),
            jax.ShapeDtypeStruct((S, D), jnp.float32),
            jax.ShapeDtypeStruct((S, TK), jnp.int32),
            jax.ShapeDtypeStruct((S, TK), jnp.float32),
            jax.ShapeDtypeStruct((S // BS3, 1, 64), jnp.int32),
        ],
        scratch_shapes=[pltpu.VMEM((BS3, D), jnp.float32)],
    )(x, o, wo, ln2_w, gate_w)


BLK = 256
NQ4 = 4
QTOK = S // NQ4
QPAIR = QTOK * TK
P = 6656
NB = P // BLK
NBP = 32
L = 16


def _extract(vec, lane):
    return jnp.sum(jnp.where(lax.iota(jnp.int32, L) == lane, vec, 0))


def _k4_kernel(ti_hbm, tw_hbm, h2_hbm, counts_hbm,
               hd_hbm, ws_hbm, pos_hbm, bexp_hbm,
               tiv, twv, posbuf, cmp_tok, cmp_w, cvm, zb, rows, bev, sem):
    c = lax.axis_index("c")
    s = lax.axis_index("s")
    e = s % E
    ql = s // E
    q = 2 * c + ql

    pltpu.sync_copy(counts_hbm.at[:], cvm)
    qoff = pl.multiple_of(q * QPAIR, QPAIR)
    pltpu.sync_copy(ti_hbm.at[pl.ds(qoff, QPAIR)], tiv)
    pltpu.sync_copy(tw_hbm.at[pl.ds(qoff, QPAIR)], twv)

    crow = [cvm[blk, 0, pl.ds(0, L)] for blk in range(2 * NQ4)]
    cq = {}
    cnt = {}
    for ee in range(E):
        for qq in range(NQ4):
            cval = _extract(crow[2 * qq], ee) + _extract(crow[2 * qq + 1], ee)
            cnt[(ee, qq)] = cval
            cq[(ee, qq)] = ((cval + L - 1) // L) * L
    base = {}
    endblk = []
    running = jnp.int32(0)
    for ee in range(E):
        tot = jnp.int32(0)
        for qq in range(NQ4):
            base[(ee, qq)] = running * BLK + tot
            tot = tot + cq[(ee, qq)]
        running = running + (tot + BLK - 1) // BLK
        endblk.append(running)

    my_base = jnp.int32(0)
    my_cnt = jnp.int32(0)
    my_cq = jnp.int32(0)
    for ee in range(E):
        for qq in range(NQ4):
            sel = jnp.logical_and(e == ee, q == qq)
            my_base = jnp.where(sel, base[(ee, qq)], my_base)
            my_cnt = jnp.where(sel, cnt[(ee, qq)], my_cnt)
            my_cq = jnp.where(sel, cq[(ee, qq)], my_cq)

    @pl.when(jnp.logical_and(c == 0, s == 0))
    def _():
        for ch in range(NBP // L):
            bv = lax.iota(jnp.int32, L) + ch * L
            acc = jnp.zeros((L,), jnp.int32)
            for ee in range(E - 1):
                acc = acc + (bv >= endblk[ee]).astype(jnp.int32)
            bev[pl.ds(ch * L, L)] = acc
        pltpu.sync_copy(bev, bexp_hbm.at[:])

    zv = jnp.zeros((L,), jnp.int32)
    for i in range(QPAIR // L):
        zb[pl.ds(i * L, L)] = zv
        cmp_tok[pl.ds(i * L, L)] = zv

    def pass2(i, cnt2):
        chunk = tiv[pl.ds(i * L, L)]
        mask = chunk == e
        mi = mask.astype(jnp.int32)
        within = plsc.cumsum(mi) - 1
        posv = my_base + cnt2 + within
        posbuf[pl.ds(i * L, L)] = jnp.where(mask, posv, 0)
        loc = cnt2 + within
        tok = (q * QPAIR + i * L + lax.iota(jnp.int32, L)) // TK
        plsc.store_scatter(cmp_tok, [loc], tok, mask=mask)
        plsc.store_scatter(cmp_w, [loc], twv[pl.ds(i * L, L)], mask=mask)
        return cnt2 + jnp.sum(mi)

    lax.fori_loop(0, QPAIR // L, pass2, jnp.int32(0))

    def gstep(j, carry):
        idxsl = cmp_tok.at[pl.ds(j * L, L)]
        pltpu.async_copy(h2_hbm.at[idxsl], rows, sem).wait()
        roff = pl.multiple_of(my_base + j * L, L)
        pltpu.sync_copy(rows, hd_hbm.at[pl.ds(roff, L)])
        pltpu.sync_copy(cmp_w.at[pl.ds(j * L, L)],
                        ws_hbm.at[pl.ds(roff, L)])
        return carry

    lax.fori_loop(0, my_cq // L, gstep, jnp.int32(0))

    pltpu.sync_copy(posbuf, pos_hbm.at[e, pl.ds(qoff, QPAIR)])


def _k4(ti_flat, tw_flat, h2, counts):
    mesh = plsc.VectorSubcoreMesh(core_axis_name="c", subcore_axis_name="s")
    kfn = pl.kernel(
        _k4_kernel,
        mesh=mesh,
        out_type=[
            jax.ShapeDtypeStruct((P, D), jnp.float32),
            jax.ShapeDtypeStruct((P,), jnp.float32),
            jax.ShapeDtypeStruct((E, S * TK), jnp.int32),
            jax.ShapeDtypeStruct((NBP,), jnp.int32),
        ],
        compiler_params=pltpu.CompilerParams(needs_layout_passes=False),
        scratch_types=[
            pltpu.VMEM((QPAIR,), jnp.int32),
            pltpu.VMEM((QPAIR,), jnp.float32),
            pltpu.VMEM((QPAIR,), jnp.int32),
            pltpu.VMEM((QPAIR,), jnp.int32),
            pltpu.VMEM((QPAIR,), jnp.float32),
            pltpu.VMEM((2 * NQ4, 1, 64), jnp.int32),
            pltpu.VMEM((QPAIR,), jnp.int32),
            pltpu.VMEM((L, D), jnp.float32),
            pltpu.VMEM((NBP,), jnp.int32),
            pltpu.SemaphoreType.DMA,
        ],
    )
    return kfn(ti_flat, tw_flat, h2, counts)


def _k5_body(bexp_ref, hd_ref, ws_ref, wg_ref, wu_ref, wd_ref, y_ref):
    hd = hd_ref[...]
    g = jnp.dot(hd, wg_ref[0], preferred_element_type=jnp.float32)
    u = jnp.dot(hd, wu_ref[0], preferred_element_type=jnp.float32)
    hh = (g * (1.0 / (1.0 + jnp.exp(-g)))) * u
    y = jnp.dot(hh, wd_ref[0], preferred_element_type=jnp.float32)
    y_ref[...] = y * ws_ref[...]


def _k5(hd, ws, bexp, w_gate, w_up, w_down):
    grid_spec = pltpu.PrefetchScalarGridSpec(
        num_scalar_prefetch=1,
        grid=(NB,),
        in_specs=[
            pl.BlockSpec((BLK, D), lambda b, be: (b, 0)),
            pl.BlockSpec((BLK, 1), lambda b, be: (b, 0)),
            pl.BlockSpec((1, D, F), lambda b, be: (be[b], 0, 0)),
            pl.BlockSpec((1, D, F), lambda b, be: (be[b], 0, 0)),
            pl.BlockSpec((1, F, D), lambda b, be: (be[b], 0, 0)),
        ],
        out_specs=pl.BlockSpec((BLK, D), lambda b, be: (b, 0)),
    )
    return pl.pallas_call(
        _k5_body,
        grid_spec=grid_spec,
        out_shape=jax.ShapeDtypeStruct((P, D), jnp.float32),
    )(bexp, hd, ws.reshape(P, 1), w_gate, w_up, w_down)


TPT = S // 32
CH6 = 8


def _k6_kernel(y_hbm, pos_hbm, x2_hbm, out_hbm, pidx, pparts, ybuf, xv, ov, sem):
    wid = lax.axis_index("c") * 16 + lax.axis_index("s")
    t0 = pl.multiple_of(wid * TPT, TPT)
    poff = pl.multiple_of(t0 * TK, TPT * TK)
    pltpu.sync_copy(pos_hbm.at[:, pl.ds(poff, TPT * TK)], pparts)
    npc = (TPT * TK) // L

    def sum_parts(i, carry):
        acc = pparts[0, pl.ds(i * L, L)]
        for ee in range(1, E):
            acc = acc + pparts[ee, pl.ds(i * L, L)]
        pidx[pl.ds(i * L, L)] = acc
        return carry

    lax.fori_loop(0, npc, sum_parts, jnp.int32(0))
    for ch in range(TPT // CH6):
        idxsl = pidx.at[pl.ds(ch * CH6 * TK, L)]
        pltpu.async_copy(y_hbm.at[idxsl], ybuf, sem).wait()
        pltpu.sync_copy(x2_hbm.at[pl.ds(pl.multiple_of(t0 + ch * CH6, CH6), CH6)], xv)

        def body(j, carry):
            sl = pl.ds(j * L, L)
            for tt in range(CH6):
                ov[tt, sl] = xv[tt, sl] + ybuf[2 * tt, sl] + ybuf[2 * tt + 1, sl]
            return carry

        lax.fori_loop(0, D // L, body, jnp.int32(0))
        pltpu.sync_copy(ov, out_hbm.at[pl.ds(pl.multiple_of(t0 + ch * CH6, CH6), CH6)])


def _k6(y, pos, x2):
    mesh = plsc.VectorSubcoreMesh(core_axis_name="c", subcore_axis_name="s")
    kfn = pl.kernel(
        _k6_kernel,
        mesh=mesh,
        out_type=jax.ShapeDtypeStruct((S, D), jnp.float32),
        compiler_params=pltpu.CompilerParams(needs_layout_passes=False),
        scratch_types=[
            pltpu.VMEM((TPT * TK,), jnp.int32),
            pltpu.VMEM((E, TPT * TK), jnp.int32),
            pltpu.VMEM((L, D), jnp.float32),
            pltpu.VMEM((CH6, D), jnp.float32),
            pltpu.VMEM((CH6, D), jnp.float32),
            pltpu.SemaphoreType.DMA,
        ],
    )
    return kfn(y, pos, x2)


def kernel(hidden_states, start_pos, position_embeddings, attention_mask,
           wq, wk, wv, wo, q_norm_w, k_norm_w, ln1_w, ln2_w,
           gate_w, w_gate, w_up, w_down):
    x = hidden_states.reshape(S, D)
    cos = position_embeddings[0]
    sin = position_embeddings[1]
    wqkv = ln1_w[:, None] * jnp.concatenate([wq, wk, wv], axis=1)
    qkv = _k1(x, wqkv)
    o = _k2(qkv, cos, sin, q_norm_w.reshape(1, DH), k_norm_w.reshape(1, DH))
    x2, h2, ti, tw, counts = _k3(x, o, wo, ln2_w.reshape(1, D), gate_w)
    hd, ws, pos, bexp = _k4(ti.reshape(S * TK), tw.reshape(S * TK),
                            h2, counts)
    y = _k5(hd, ws, bexp, w_gate, w_up, w_down)
    out = _k6(y, pos, x2)
    return out.reshape(B, S, D)


# speedup vs baseline: 1.4601x; 1.0126x over previous
import functools
import jax
import jax.numpy as jnp
from jax import lax
from jax.experimental import pallas as pl
from jax.experimental.pallas import tpu as pltpu
from jax.experimental.pallas import tpu_sc as plsc

B, S, D = 1, 2048, 2048
H, KV, DH = 16, 4, 128
E, TK, F = 8, 2, 768
EPS = 1e-6
SCALE = DH ** -0.5

BS1 = 256
BQ = 1024
BK = 1024
BS3 = 256
BM5 = 256


def _rms_in(x, w):
    v = jnp.mean(jnp.square(x), axis=-1, keepdims=True)
    return w * (x * lax.rsqrt(v + EPS))


def _rot_half(x):
    h = x.shape[-1] // 2
    return jnp.concatenate([-x[:, h:], x[:, :h]], axis=-1)


DK1 = 512
KC1 = D // DK1
QKVW = (H + 2 * KV) * DH


def _k1_body(x_ref, w_ref, qkv_ref, acc_ref, ssq_ref):
    kc = pl.program_id(1)
    xs = x_ref[...]
    part = jnp.dot(xs, w_ref[...], preferred_element_type=jnp.float32)
    ssq = jnp.sum(xs * xs, axis=-1, keepdims=True)

    @pl.when(kc == 0)
    def _():
        acc_ref[...] = part
        ssq_ref[...] = ssq

    @pl.when(kc != 0)
    def _():
        acc_ref[...] = acc_ref[...] + part
        ssq_ref[...] = ssq_ref[...] + ssq

    @pl.when(kc == KC1 - 1)
    def _():
        scale = lax.rsqrt(ssq_ref[...] / D + EPS)
        qkv_ref[...] = acc_ref[...] * scale


def _k1(x, wqkv):
    n = S // BS1
    return pl.pallas_call(
        _k1_body,
        grid=(n, KC1),
        in_specs=[
            pl.BlockSpec((BS1, DK1), lambda i, kc: (i, kc)),
            pl.BlockSpec((DK1, QKVW), lambda i, kc: (kc, 0)),
        ],
        out_specs=pl.BlockSpec((BS1, QKVW), lambda i, kc: (i, 0)),
        out_shape=jax.ShapeDtypeStruct((S, QKVW), jnp.float32),
        scratch_shapes=[pltpu.VMEM((BS1, QKVW), jnp.float32),
                        pltpu.VMEM((BS1, 1), jnp.float32)],
    )(x, wqkv)


def _k2_body(q_ref, k_ref, v_ref, cos_ref, sin_ref, qn_ref, kn_ref,
             o_ref, kr_ref):
    h = pl.program_id(0)
    iq = pl.program_id(1)

    @pl.when(jnp.logical_and(iq == 0, h % (H // KV) == 0))
    def _():
        km = _rms_in(k_ref[...], kn_ref[...])
        kr_ref[...] = km * cos_ref[...] + _rot_half(km) * sin_ref[...]

    qm = _rms_in(q_ref[...], qn_ref[...])
    cosq = cos_ref[pl.ds(iq * BQ, BQ), :]
    sinq = sin_ref[pl.ds(iq * BQ, BQ), :]
    q = (qm * cosq + _rot_half(qm) * sinq) * SCALE
    row = iq * BQ + lax.broadcasted_iota(jnp.int32, (BQ, BK), 0)

    def step(j, carry):
        m, l, acc = carry
        kj = kr_ref[pl.ds(j * BK, BK), :]
        vj = v_ref[pl.ds(j * BK, BK), :]
        s = lax.dot_general(q, kj, (((1,), (1,)), ((), ())),
                            preferred_element_type=jnp.float32)
        col = j * BK + lax.broadcasted_iota(jnp.int32, (BQ, BK), 1)
        s = jnp.where(col <= row, s, -1e30)
        mnew = jnp.maximum(m, jnp.max(s, axis=-1, keepdims=True))
        p = jnp.exp(s - mnew)
        corr = jnp.exp(m - mnew)
        l = l * corr + jnp.sum(p, axis=-1, keepdims=True)
        acc = acc * corr + jnp.dot(p, vj, preferred_element_type=jnp.float32)
        return m * 0 + mnew, l, acc

    m0 = jnp.full((BQ, 1), -1e30, jnp.float32)
    l0 = jnp.zeros((BQ, 1), jnp.float32)
    a0 = jnp.zeros((BQ, DH), jnp.float32)
    m, l, acc = lax.fori_loop(0, (iq * BQ) // BK + 1, step, (m0, l0, a0))
    o_ref[...] = acc / l


def _k2(qkv, cos, sin, qn_w, kn_w):
    nq = S // BQ
    grp = H // KV
    return pl.pallas_call(
        _k2_body,
        grid=(H, nq),
        in_specs=[
            pl.BlockSpec((BQ, DH), lambda h, i: (i, h)),
            pl.BlockSpec((S, DH), lambda h, i: (0, H + h // grp)),
            pl.BlockSpec((S, DH), lambda h, i: (0, H + KV + h // grp)),
            pl.BlockSpec((S, DH), lambda h, i: (0, 0)),
            pl.BlockSpec((S, DH), lambda h, i: (0, 0)),
            pl.BlockSpec((1, DH), lambda h, i: (0, 0)),
            pl.BlockSpec((1, DH), lambda h, i: (0, 0)),
        ],
        out_specs=pl.BlockSpec((BQ, DH), lambda h, i: (i, h)),
        out_shape=jax.ShapeDtypeStruct((S, H * DH), jnp.float32),
        scratch_shapes=[pltpu.VMEM((S, DH), jnp.float32)],
    )(qkv, qkv, qkv, cos, sin, qn_w, kn_w)


DKO = 512
KC3 = (H * DH) // DKO


def _k3_body(x_ref, o_ref, wo_ref, ln2_ref, gw_ref,
             x2_ref, h2_ref, ti_ref, tw_ref, cnt_ref, acc_ref):
    kc = pl.program_id(1)
    s = jnp.dot(o_ref[...], wo_ref[...], preferred_element_type=jnp.float32)

    @pl.when(kc == 0)
    def _():
        acc_ref[...] = x_ref[...] + s

    @pl.when(kc != 0)
    def _():
        acc_ref[...] = acc_ref[...] + s

    @pl.when(kc == KC3 - 1)
    def _():
        _k3_tail(acc_ref, ln2_ref, gw_ref, x2_ref, h2_ref,
                 ti_ref, tw_ref, cnt_ref)


def _k3_tail(acc_ref, ln2_ref, gw_ref, x2_ref, h2_ref, ti_ref, tw_ref, cnt_ref):
    acc = acc_ref[...]
    x2_ref[...] = acc
    h2 = _rms_in(acc, ln2_ref[...])
    h2_ref[...] = h2
    logits = jnp.dot(h2, gw_ref[...], preferred_element_type=jnp.float32)
    iot = lax.broadcasted_iota(jnp.int32, logits.shape, 1)
    m1 = jnp.max(logits, axis=-1, keepdims=True)
    i1 = jnp.min(jnp.where(logits == m1, iot, E), axis=-1, keepdims=True)
    l2m = jnp.where(iot == i1, -jnp.inf, logits)
    m2 = jnp.max(l2m, axis=-1, keepdims=True)
    i2 = jnp.min(jnp.where(l2m == m2, iot, E), axis=-1, keepdims=True)
    w1 = 1.0 / (1.0 + jnp.exp(m2 - m1))
    w2 = 1.0 - w1
    ti_ref[...] = jnp.concatenate([i1, i2], axis=1)
    tw_ref[...] = jnp.concatenate([w1, w2], axis=1)
    iot64 = lax.broadcasted_iota(jnp.int32, (BS3, 64), 1)
    oh = (iot64 == i1).astype(jnp.int32) + (iot64 == i2).astype(jnp.int32)
    cnt_ref[...] = jnp.sum(oh, axis=0, keepdims=True).reshape(1, 1, 64)


def _k3(x, o, wo, ln2_w, gate_w):
    n = S // BS3
    return pl.pallas_call(
        _k3_body,
        grid=(n, KC3),
        in_specs=[
            pl.BlockSpec((BS3, D), lambda i, kc: (i, 0)),
            pl.BlockSpec((BS3, DKO), lambda i, kc: (i, kc)),
            pl.BlockSpec((DKO, D), lambda i, kc: (kc, 0)),
            pl.BlockSpec((1, D), lambda i, kc: (0, 0)),
            pl.BlockSpec((D, E), lambda i, kc: (0, 0)),
        ],
        out_specs=[
            pl.BlockSpec((BS3, D), lambda i, kc: (i, 0)),
            pl.BlockSpec((BS3, D), lambda i, kc: (i, 0)),
            pl.BlockSpec((BS3, TK), lambda i, kc: (i, 0)),
            pl.BlockSpec((BS3, TK), lambda i, kc: (i, 0)),
            pl.BlockSpec((1, 1, 64), lambda i, kc: (i, 0, 0)),
        ],
        out_shape=[
            jax.ShapeDtypeStruct((S, D), jnp.float32),
            jax.ShapeDtypeStruct((S, D), jnp.float32),
            jax.ShapeDtypeStruct((S, TK), jnp.int32),
            jax.ShapeDtypeStruct((S, TK), jnp.float32),
            jax.ShapeDtypeStruct((S // BS3, 1, 64), jnp.int32),
        ],
        scratch_shapes=[pltpu.VMEM((BS3, D), jnp.float32)],
    )(x, o, wo, ln2_w, gate_w)


BLK = 256
NQ4 = 4
QTOK = S // NQ4
QPAIR = QTOK * TK
P = 6656
NB = P // BLK
NBP = 32
L = 16


def _extract(vec, lane):
    return jnp.sum(jnp.where(lax.iota(jnp.int32, L) == lane, vec, 0))


def _k4_kernel(ti_hbm, tw_hbm, h2_hbm, counts_hbm,
               hd_hbm, ws_hbm, pos_hbm, bexp_hbm,
               tiv, twv, posbuf, cmp_tok, cmp_w, cvm, zb, rows, bev, sem):
    c = lax.axis_index("c")
    s = lax.axis_index("s")
    e = s % E
    ql = s // E
    q = 2 * c + ql

    pltpu.sync_copy(counts_hbm.at[:], cvm)
    qoff = pl.multiple_of(q * QPAIR, QPAIR)
    pltpu.sync_copy(ti_hbm.at[pl.ds(qoff, QPAIR)], tiv)
    pltpu.sync_copy(tw_hbm.at[pl.ds(qoff, QPAIR)], twv)

    crow = [cvm[blk, 0, pl.ds(0, L)] for blk in range(2 * NQ4)]
    cq = {}
    cnt = {}
    for ee in range(E):
        for qq in range(NQ4):
            cval = _extract(crow[2 * qq], ee) + _extract(crow[2 * qq + 1], ee)
            cnt[(ee, qq)] = cval
            cq[(ee, qq)] = ((cval + L - 1) // L) * L
    base = {}
    endblk = []
    running = jnp.int32(0)
    for ee in range(E):
        tot = jnp.int32(0)
        for qq in range(NQ4):
            base[(ee, qq)] = running * BLK + tot
            tot = tot + cq[(ee, qq)]
        running = running + (tot + BLK - 1) // BLK
        endblk.append(running)

    my_base = jnp.int32(0)
    my_cnt = jnp.int32(0)
    my_cq = jnp.int32(0)
    for ee in range(E):
        for qq in range(NQ4):
            sel = jnp.logical_and(e == ee, q == qq)
            my_base = jnp.where(sel, base[(ee, qq)], my_base)
            my_cnt = jnp.where(sel, cnt[(ee, qq)], my_cnt)
            my_cq = jnp.where(sel, cq[(ee, qq)], my_cq)

    @pl.when(jnp.logical_and(c == 0, s == 0))
    def _():
        for ch in range(NBP // L):
            bv = lax.iota(jnp.int32, L) + ch * L
            acc = jnp.zeros((L,), jnp.int32)
            for ee in range(E - 1):
                acc = acc + (bv >= endblk[ee]).astype(jnp.int32)
            bev[pl.ds(ch * L, L)] = acc
        pltpu.sync_copy(bev, bexp_hbm.at[:])

    zv = jnp.zeros((L,), jnp.int32)
    for i in range(QPAIR // L):
        zb[pl.ds(i * L, L)] = zv
        cmp_tok[pl.ds(i * L, L)] = zv

    def pass2(i, cnt2):
        chunk = tiv[pl.ds(i * L, L)]
        mask = chunk == e
        mi = mask.astype(jnp.int32)
        within = plsc.cumsum(mi) - 1
        posv = my_base + cnt2 + within
        posbuf[pl.ds(i * L, L)] = jnp.where(mask, posv, 0)
        loc = cnt2 + within
        tok = (q * QPAIR + i * L + lax.iota(jnp.int32, L)) // TK
        plsc.store_scatter(cmp_tok, [loc], tok, mask=mask)
        plsc.store_scatter(cmp_w, [loc], twv[pl.ds(i * L, L)], mask=mask)
        return cnt2 + jnp.sum(mi)

    lax.fori_loop(0, QPAIR // L, pass2, jnp.int32(0))

    def gstep(j, carry):
        idxsl = cmp_tok.at[pl.ds(j * L, L)]
        pltpu.async_copy(h2_hbm.at[idxsl], rows, sem).wait()
        roff = pl.multiple_of(my_base + j * L, L)
        pltpu.sync_copy(rows, hd_hbm.at[pl.ds(roff, L)])
        pltpu.sync_copy(cmp_w.at[pl.ds(j * L, L)],
                        ws_hbm.at[pl.ds(roff, L)])
        return carry

    lax.fori_loop(0, my_cq // L, gstep, jnp.int32(0))

    pltpu.sync_copy(posbuf, pos_hbm.at[e, pl.ds(qoff, QPAIR)])


def _k4(ti_flat, tw_flat, h2, counts):
    mesh = plsc.VectorSubcoreMesh(core_axis_name="c", subcore_axis_name="s")
    kfn = pl.kernel(
        _k4_kernel,
        mesh=mesh,
        out_type=[
            jax.ShapeDtypeStruct((P, D), jnp.float32),
            jax.ShapeDtypeStruct((P,), jnp.float32),
            jax.ShapeDtypeStruct((E, S * TK), jnp.int32),
            jax.ShapeDtypeStruct((NBP,), jnp.int32),
        ],
        compiler_params=pltpu.CompilerParams(needs_layout_passes=False),
        scratch_types=[
            pltpu.VMEM((QPAIR,), jnp.int32),
            pltpu.VMEM((QPAIR,), jnp.float32),
            pltpu.VMEM((QPAIR,), jnp.int32),
            pltpu.VMEM((QPAIR,), jnp.int32),
            pltpu.VMEM((QPAIR,), jnp.float32),
            pltpu.VMEM((2 * NQ4, 1, 64), jnp.int32),
            pltpu.VMEM((QPAIR,), jnp.int32),
            pltpu.VMEM((L, D), jnp.float32),
            pltpu.VMEM((NBP,), jnp.int32),
            pltpu.SemaphoreType.DMA,
        ],
    )
    return kfn(ti_flat, tw_flat, h2, counts)


def _k5_body(bexp_ref, hd_ref, ws_ref, wg_ref, wu_ref, wd_ref, y_ref):
    hd = hd_ref[...]
    g = jnp.dot(hd, wg_ref[0], preferred_element_type=jnp.float32)
    u = jnp.dot(hd, wu_ref[0], preferred_element_type=jnp.float32)
    hh = (g * (1.0 / (1.0 + jnp.exp(-g)))) * u
    y = jnp.dot(hh, wd_ref[0], preferred_element_type=jnp.float32)
    y_ref[...] = y * ws_ref[...]


def _k5(hd, ws, bexp, w_gate, w_up, w_down):
    grid_spec = pltpu.PrefetchScalarGridSpec(
        num_scalar_prefetch=1,
        grid=(NB,),
        in_specs=[
            pl.BlockSpec((BLK, D), lambda b, be: (b, 0)),
            pl.BlockSpec((BLK, 1), lambda b, be: (b, 0)),
            pl.BlockSpec((1, D, F), lambda b, be: (be[b], 0, 0)),
            pl.BlockSpec((1, D, F), lambda b, be: (be[b], 0, 0)),
            pl.BlockSpec((1, F, D), lambda b, be: (be[b], 0, 0)),
        ],
        out_specs=pl.BlockSpec((BLK, D), lambda b, be: (b, 0)),
    )
    return pl.pallas_call(
        _k5_body,
        grid_spec=grid_spec,
        out_shape=jax.ShapeDtypeStruct((P, D), jnp.float32),
    )(bexp, hd, ws.reshape(P, 1), w_gate, w_up, w_down)


TPT = S // 32
CH6 = 8


def _k6_kernel(y_hbm, pos_hbm, x2_hbm, out_hbm,
               pidx, pparts, yb0, yb1, xv, ov, sem0, sem1):
    wid = lax.axis_index("c") * 16 + lax.axis_index("s")
    t0 = pl.multiple_of(wid * TPT, TPT)
    poff = pl.multiple_of(t0 * TK, TPT * TK)
    pltpu.sync_copy(pos_hbm.at[:, pl.ds(poff, TPT * TK)], pparts)
    npc = (TPT * TK) // L

    def sum_parts(i, carry):
        acc = pparts[0, pl.ds(i * L, L)]
        for ee in range(1, E):
            acc = acc + pparts[ee, pl.ds(i * L, L)]
        pidx[pl.ds(i * L, L)] = acc
        return carry

    lax.fori_loop(0, npc, sum_parts, jnp.int32(0))

    nch = TPT // CH6
    ybufs = [yb0, yb1]
    sems = [sem0, sem1]
    cps = [None, None]
    for ch in range(nch + 1):
        if ch < nch:
            idxsl = pidx.at[pl.ds(ch * CH6 * TK, L)]
            cps[ch % 2] = pltpu.async_copy(y_hbm.at[idxsl], ybufs[ch % 2],
                                           sems[ch % 2])
        if ch > 0:
            p = ch - 1
            yb = ybufs[p % 2]
            pltpu.sync_copy(
                x2_hbm.at[pl.ds(pl.multiple_of(t0 + p * CH6, CH6), CH6)], xv)
            cps[p % 2].wait()

            def body(j, carry):
                sl = pl.ds(j * L, L)
                for tt in range(CH6):
                    ov[tt, sl] = (xv[tt, sl] + yb[2 * tt, sl]
                                  + yb[2 * tt + 1, sl])
                return carry

            lax.fori_loop(0, D // L, body, jnp.int32(0))
            pltpu.sync_copy(
                ov, out_hbm.at[pl.ds(pl.multiple_of(t0 + p * CH6, CH6), CH6)])


def _k6(y, pos, x2):
    mesh = plsc.VectorSubcoreMesh(core_axis_name="c", subcore_axis_name="s")
    kfn = pl.kernel(
        _k6_kernel,
        mesh=mesh,
        out_type=jax.ShapeDtypeStruct((S, D), jnp.float32),
        compiler_params=pltpu.CompilerParams(needs_layout_passes=False),
        scratch_types=[
            pltpu.VMEM((TPT * TK,), jnp.int32),
            pltpu.VMEM((E, TPT * TK), jnp.int32),
            pltpu.VMEM((L, D), jnp.float32),
            pltpu.VMEM((L, D), jnp.float32),
            pltpu.VMEM((CH6, D), jnp.float32),
            pltpu.VMEM((CH6, D), jnp.float32),
            pltpu.SemaphoreType.DMA,
            pltpu.SemaphoreType.DMA,
        ],
    )
    return kfn(y, pos, x2)


def kernel(hidden_states, start_pos, position_embeddings, attention_mask,
           wq, wk, wv, wo, q_norm_w, k_norm_w, ln1_w, ln2_w,
           gate_w, w_gate, w_up, w_down):
    x = hidden_states.reshape(S, D)
    cos = position_embeddings[0]
    sin = position_embeddings[1]
    wqkv = ln1_w[:, None] * jnp.concatenate([wq, wk, wv], axis=1)
    qkv = _k1(x, wqkv)
    o = _k2(qkv, cos, sin, q_norm_w.reshape(1, DH), k_norm_w.reshape(1, DH))
    x2, h2, ti, tw, counts = _k3(x, o, wo, ln2_w.reshape(1, D), gate_w)
    hd, ws, pos, bexp = _k4(ti.reshape(S * TK), tw.reshape(S * TK),
                            h2, counts)
    y = _k5(hd, ws, bexp, w_gate, w_up, w_down)
    out = _k6(y, pos, x2)
    return out.reshape(B, S, D)


# speedup vs baseline: 1.4619x; 1.0012x over previous
import jax
import jax.numpy as jnp
from jax import lax
from jax.experimental import pallas as pl
from jax.experimental.pallas import tpu as pltpu
from jax.experimental.pallas import tpu_sc as plsc

B, S, D = 1, 2048, 2048
H, KV, DH = 16, 4, 128
E, TK, F = 8, 2, 768
EPS = 1e-6
SCALE = DH ** -0.5

BS1 = 256
BQ = 1024
BK = 1024
BS3 = 256


def _rms_in(x, w):
    v = jnp.mean(jnp.square(x), axis=-1, keepdims=True)
    return w * (x * lax.rsqrt(v + EPS))


def _rot_half(x):
    h = x.shape[-1] // 2
    return jnp.concatenate([-x[:, h:], x[:, :h]], axis=-1)


DK1 = 512
KC1 = D // DK1
QKVW = (H + 2 * KV) * DH


def _k1_body(x_ref, w_ref, qkv_ref, acc_ref, ssq_ref):
    kc = pl.program_id(1)
    xs = x_ref[...]
    part = jnp.dot(xs, w_ref[...], preferred_element_type=jnp.float32)
    ssq = jnp.sum(xs * xs, axis=-1, keepdims=True)

    @pl.when(kc == 0)
    def _():
        acc_ref[...] = part
        ssq_ref[...] = ssq

    @pl.when(kc != 0)
    def _():
        acc_ref[...] = acc_ref[...] + part
        ssq_ref[...] = ssq_ref[...] + ssq

    @pl.when(kc == KC1 - 1)
    def _():
        scale = lax.rsqrt(ssq_ref[...] / D + EPS)
        qkv_ref[...] = acc_ref[...] * scale


def _k1(x, wqkv):
    n = S // BS1
    return pl.pallas_call(
        _k1_body,
        grid=(n, KC1),
        in_specs=[
            pl.BlockSpec((BS1, DK1), lambda i, kc: (i, kc)),
            pl.BlockSpec((DK1, QKVW), lambda i, kc: (kc, 0)),
        ],
        out_specs=pl.BlockSpec((BS1, QKVW), lambda i, kc: (i, 0)),
        out_shape=jax.ShapeDtypeStruct((S, QKVW), jnp.float32),
        scratch_shapes=[pltpu.VMEM((BS1, QKVW), jnp.float32),
                        pltpu.VMEM((BS1, 1), jnp.float32)],
    )(x, wqkv)


def _k2_body(q_ref, k_ref, v_ref, cos_ref, sin_ref, qn_ref, kn_ref,
             o_ref, kr_ref):
    h = pl.program_id(0)
    iq = pl.program_id(1)

    @pl.when(jnp.logical_and(iq == 0, h % (H // KV) == 0))
    def _():
        km = _rms_in(k_ref[...], kn_ref[...])
        kr_ref[...] = km * cos_ref[...] + _rot_half(km) * sin_ref[...]

    qm = _rms_in(q_ref[...], qn_ref[...])
    cosq = cos_ref[pl.ds(iq * BQ, BQ), :]
    sinq = sin_ref[pl.ds(iq * BQ, BQ), :]
    q = (qm * cosq + _rot_half(qm) * sinq) * SCALE
    row = iq * BQ + lax.broadcasted_iota(jnp.int32, (BQ, BK), 0)

    def step(j, carry):
        m, l, acc = carry
        kj = kr_ref[pl.ds(j * BK, BK), :]
        vj = v_ref[pl.ds(j * BK, BK), :]
        s = lax.dot_general(q, kj, (((1,), (1,)), ((), ())),
                            preferred_element_type=jnp.float32)
        col = j * BK + lax.broadcasted_iota(jnp.int32, (BQ, BK), 1)
        s = jnp.where(col <= row, s, -1e30)
        mnew = jnp.maximum(m, jnp.max(s, axis=-1, keepdims=True))
        p = jnp.exp(s - mnew)
        corr = jnp.exp(m - mnew)
        l = l * corr + jnp.sum(p, axis=-1, keepdims=True)
        acc = acc * corr + jnp.dot(p, vj, preferred_element_type=jnp.float32)
        return m * 0 + mnew, l, acc

    m0 = jnp.full((BQ, 1), -1e30, jnp.float32)
    l0 = jnp.zeros((BQ, 1), jnp.float32)
    a0 = jnp.zeros((BQ, DH), jnp.float32)
    m, l, acc = lax.fori_loop(0, (iq * BQ) // BK + 1, step, (m0, l0, a0))
    o_ref[...] = acc / l


def _k2(qkv, cos, sin, qn_w, kn_w):
    nq = S // BQ
    grp = H // KV
    return pl.pallas_call(
        _k2_body,
        grid=(H, nq),
        in_specs=[
            pl.BlockSpec((BQ, DH), lambda h, i: (i, h)),
            pl.BlockSpec((S, DH), lambda h, i: (0, H + h // grp)),
            pl.BlockSpec((S, DH), lambda h, i: (0, H + KV + h // grp)),
            pl.BlockSpec((S, DH), lambda h, i: (0, 0)),
            pl.BlockSpec((S, DH), lambda h, i: (0, 0)),
            pl.BlockSpec((1, DH), lambda h, i: (0, 0)),
            pl.BlockSpec((1, DH), lambda h, i: (0, 0)),
        ],
        out_specs=pl.BlockSpec((BQ, DH), lambda h, i: (i, h)),
        out_shape=jax.ShapeDtypeStruct((S, H * DH), jnp.float32),
        scratch_shapes=[pltpu.VMEM((S, DH), jnp.float32)],
    )(qkv, qkv, qkv, cos, sin, qn_w, kn_w)


DKO = 512
KC3 = (H * DH) // DKO


def _k3_body(x_ref, o_ref, wo_ref, ln2_ref, gw_ref,
             x2_ref, h2_ref, ti_ref, tw_ref, cnt_ref, acc_ref):
    kc = pl.program_id(1)
    s = jnp.dot(o_ref[...], wo_ref[...], preferred_element_type=jnp.float32)

    @pl.when(kc == 0)
    def _():
        acc_ref[...] = x_ref[...] + s

    @pl.when(kc != 0)
    def _():
        acc_ref[...] = acc_ref[...] + s

    @pl.when(kc == KC3 - 1)
    def _():
        _k3_tail(acc_ref, ln2_ref, gw_ref, x2_ref, h2_ref,
                 ti_ref, tw_ref, cnt_ref)


def _k3_tail(acc_ref, ln2_ref, gw_ref, x2_ref, h2_ref, ti_ref, tw_ref, cnt_ref):
    acc = acc_ref[...]
    x2_ref[...] = acc
    h2 = _rms_in(acc, ln2_ref[...])
    h2_ref[...] = h2
    logits = jnp.dot(h2, gw_ref[...], preferred_element_type=jnp.float32)
    iot = lax.broadcasted_iota(jnp.int32, logits.shape, 1)
    m1 = jnp.max(logits, axis=-1, keepdims=True)
    i1 = jnp.min(jnp.where(logits == m1, iot, E), axis=-1, keepdims=True)
    l2m = jnp.where(iot == i1, -jnp.inf, logits)
    m2 = jnp.max(l2m, axis=-1, keepdims=True)
    i2 = jnp.min(jnp.where(l2m == m2, iot, E), axis=-1, keepdims=True)
    w1 = 1.0 / (1.0 + jnp.exp(m2 - m1))
    w2 = 1.0 - w1
    ti_ref[...] = jnp.concatenate([i1, i2], axis=1)
    tw_ref[...] = jnp.concatenate([w1, w2], axis=1)
    iot64 = lax.broadcasted_iota(jnp.int32, (BS3, 64), 1)
    oh = (iot64 == i1).astype(jnp.int32) + (iot64 == i2).astype(jnp.int32)
    cnt_ref[...] = jnp.sum(oh, axis=0, keepdims=True).reshape(1, 1, 64)


def _k3(x, o, wo, ln2_w, gate_w):
    n = S // BS3
    return pl.pallas_call(
        _k3_body,
        grid=(n, KC3),
        in_specs=[
            pl.BlockSpec((BS3, D), lambda i, kc: (i, 0)),
            pl.BlockSpec((BS3, DKO), lambda i, kc: (i, kc)),
            pl.BlockSpec((DKO, D), lambda i, kc: (kc, 0)),
            pl.BlockSpec((1, D), lambda i, kc: (0, 0)),
            pl.BlockSpec((D, E), lambda i, kc: (0, 0)),
        ],
        out_specs=[
            pl.BlockSpec((BS3, D), lambda i, kc: (i, 0)),
            pl.BlockSpec((BS3, D), lambda i, kc: (i, 0)),
            pl.BlockSpec((BS3, TK), lambda i, kc: (i, 0)),
            pl.BlockSpec((BS3, TK), lambda i, kc: (i, 0)),
            pl.BlockSpec((1, 1, 64), lambda i, kc: (i, 0, 0)),
        ],
        out_shape=[
            jax.ShapeDtypeStruct((S, D), jnp.float32),
            jax.ShapeDtypeStruct((S, D), jnp.float32),
            jax.ShapeDtypeStruct((S, TK), jnp.int32),
            jax.ShapeDtypeStruct((S, TK), jnp.float32),
            jax.ShapeDtypeStruct((S // BS3, 1, 64), jnp.int32),
        ],
        scratch_shapes=[pltpu.VMEM((BS3, D), jnp.float32)],
    )(x, o, wo, ln2_w, gate_w)


BLK = 256
NQ4 = 4
QTOK = S // NQ4
QPAIR = QTOK * TK
P = 6656
NB = P // BLK
NBP = 32
L = 16


def _extract(vec, lane):
    return jnp.sum(jnp.where(lax.iota(jnp.int32, L) == lane, vec, 0))


def _k4_kernel(ti_hbm, tw_hbm, h2_hbm, counts_hbm,
               hd_hbm, ws_hbm, pos_hbm, bexp_hbm,
               tiv, twv, posbuf, cmp_tok, cmp_w, cvm, rows, bev, sem):
    c = lax.axis_index("c")
    s = lax.axis_index("s")
    e = s % E
    ql = s // E
    q = 2 * c + ql

    pltpu.sync_copy(counts_hbm.at[:], cvm)
    qoff = pl.multiple_of(q * QPAIR, QPAIR)
    pltpu.sync_copy(ti_hbm.at[pl.ds(qoff, QPAIR)], tiv)
    pltpu.sync_copy(tw_hbm.at[pl.ds(qoff, QPAIR)], twv)

    crow = [cvm[blk, 0, pl.ds(0, L)] for blk in range(2 * NQ4)]
    cq = {}
    cnt = {}
    for ee in range(E):
        for qq in range(NQ4):
            cval = _extract(crow[2 * qq], ee) + _extract(crow[2 * qq + 1], ee)
            cnt[(ee, qq)] = cval
            cq[(ee, qq)] = ((cval + L - 1) // L) * L
    base = {}
    endblk = []
    running = jnp.int32(0)
    for ee in range(E):
        tot = jnp.int32(0)
        for qq in range(NQ4):
            base[(ee, qq)] = running * BLK + tot
            tot = tot + cq[(ee, qq)]
        running = running + (tot + BLK - 1) // BLK
        endblk.append(running)

    my_base = jnp.int32(0)
    my_cnt = jnp.int32(0)
    my_cq = jnp.int32(0)
    for ee in range(E):
        for qq in range(NQ4):
            sel = jnp.logical_and(e == ee, q == qq)
            my_base = jnp.where(sel, base[(ee, qq)], my_base)
            my_cnt = jnp.where(sel, cnt[(ee, qq)], my_cnt)
            my_cq = jnp.where(sel, cq[(ee, qq)], my_cq)

    @pl.when(jnp.logical_and(c == 0, s == 0))
    def _():
        for ch in range(NBP // L):
            bv = lax.iota(jnp.int32, L) + ch * L
            acc = jnp.zeros((L,), jnp.int32)
            for ee in range(E - 1):
                acc = acc + (bv >= endblk[ee]).astype(jnp.int32)
            bev[pl.ds(ch * L, L)] = acc
        pltpu.sync_copy(bev, bexp_hbm.at[:])

    zv = jnp.zeros((L,), jnp.int32)
    for i in range(QPAIR // L):
        cmp_tok[pl.ds(i * L, L)] = zv

    def pass2(i, cnt2):
        chunk = tiv[pl.ds(i * L, L)]
        mask = chunk == e
        mi = mask.astype(jnp.int32)
        within = plsc.cumsum(mi) - 1
        posv = my_base + cnt2 + within
        posbuf[pl.ds(i * L, L)] = jnp.where(mask, posv, 0)
        loc = cnt2 + within
        tok = (q * QPAIR + i * L + lax.iota(jnp.int32, L)) // TK
        plsc.store_scatter(cmp_tok, [loc], tok, mask=mask)
        plsc.store_scatter(cmp_w, [loc], twv[pl.ds(i * L, L)], mask=mask)
        return cnt2 + jnp.sum(mi)

    lax.fori_loop(0, QPAIR // L, pass2, jnp.int32(0))

    def gstep(j, carry):
        idxsl = cmp_tok.at[pl.ds(j * L, L)]
        pltpu.async_copy(h2_hbm.at[idxsl], rows, sem).wait()
        roff = pl.multiple_of(my_base + j * L, L)
        pltpu.sync_copy(rows, hd_hbm.at[pl.ds(roff, L)])
        pltpu.sync_copy(cmp_w.at[pl.ds(j * L, L)],
                        ws_hbm.at[pl.ds(roff, L)])
        return carry

    lax.fori_loop(0, my_cq // L, gstep, jnp.int32(0))

    pltpu.sync_copy(posbuf, pos_hbm.at[e, pl.ds(qoff, QPAIR)])


def _k4(ti_flat, tw_flat, h2, counts):
    mesh = plsc.VectorSubcoreMesh(core_axis_name="c", subcore_axis_name="s")
    kfn = pl.kernel(
        _k4_kernel,
        mesh=mesh,
        out_type=[
            jax.ShapeDtypeStruct((P, D), jnp.float32),
            jax.ShapeDtypeStruct((P,), jnp.float32),
            jax.ShapeDtypeStruct((E, S * TK), jnp.int32),
            jax.ShapeDtypeStruct((NBP,), jnp.int32),
        ],
        compiler_params=pltpu.CompilerParams(needs_layout_passes=False),
        scratch_types=[
            pltpu.VMEM((QPAIR,), jnp.int32),
            pltpu.VMEM((QPAIR,), jnp.float32),
            pltpu.VMEM((QPAIR,), jnp.int32),
            pltpu.VMEM((QPAIR,), jnp.int32),
            pltpu.VMEM((QPAIR,), jnp.float32),
            pltpu.VMEM((2 * NQ4, 1, 64), jnp.int32),
            pltpu.VMEM((L, D), jnp.float32),
            pltpu.VMEM((NBP,), jnp.int32),
            pltpu.SemaphoreType.DMA,
        ],
    )
    return kfn(ti_flat, tw_flat, h2, counts)


def _k5_body(bexp_ref, hd_ref, ws_ref, wg_ref, wu_ref, wd_ref, y_ref):
    hd = hd_ref[...]
    g = jnp.dot(hd, wg_ref[0], preferred_element_type=jnp.float32)
    u = jnp.dot(hd, wu_ref[0], preferred_element_type=jnp.float32)
    hh = (g * (1.0 / (1.0 + jnp.exp(-g)))) * u
    y = jnp.dot(hh, wd_ref[0], preferred_element_type=jnp.float32)
    y_ref[...] = y * ws_ref[...]


def _k5(hd, ws, bexp, w_gate, w_up, w_down):
    grid_spec = pltpu.PrefetchScalarGridSpec(
        num_scalar_prefetch=1,
        grid=(NB,),
        in_specs=[
            pl.BlockSpec((BLK, D), lambda b, be: (b, 0)),
            pl.BlockSpec((BLK, 1), lambda b, be: (b, 0)),
            pl.BlockSpec((1, D, F), lambda b, be: (be[b], 0, 0)),
            pl.BlockSpec((1, D, F), lambda b, be: (be[b], 0, 0)),
            pl.BlockSpec((1, F, D), lambda b, be: (be[b], 0, 0)),
        ],
        out_specs=pl.BlockSpec((BLK, D), lambda b, be: (b, 0)),
    )
    return pl.pallas_call(
        _k5_body,
        grid_spec=grid_spec,
        out_shape=jax.ShapeDtypeStruct((P, D), jnp.float32),
    )(bexp, hd, ws.reshape(P, 1), w_gate, w_up, w_down)


TPT = S // 32
CH6 = 8


def _k6_kernel(y_hbm, pos_hbm, x2_hbm, out_hbm,
               pidx, pparts, yb0, yb1, xv, ov, sem0, sem1):
    wid = lax.axis_index("c") * 16 + lax.axis_index("s")
    t0 = pl.multiple_of(wid * TPT, TPT)
    poff = pl.multiple_of(t0 * TK, TPT * TK)
    pltpu.sync_copy(pos_hbm.at[:, pl.ds(poff, TPT * TK)], pparts)
    npc = (TPT * TK) // L

    def sum_parts(i, carry):
        acc = pparts[0, pl.ds(i * L, L)]
        for ee in range(1, E):
            acc = acc + pparts[ee, pl.ds(i * L, L)]
        pidx[pl.ds(i * L, L)] = acc
        return carry

    lax.fori_loop(0, npc, sum_parts, jnp.int32(0))

    nch = TPT // CH6
    ybufs = [yb0, yb1]
    sems = [sem0, sem1]
    cps = [None, None]
    for ch in range(nch + 1):
        if ch < nch:
            idxsl = pidx.at[pl.ds(ch * CH6 * TK, L)]
            cps[ch % 2] = pltpu.async_copy(y_hbm.at[idxsl], ybufs[ch % 2],
                                           sems[ch % 2])
        if ch > 0:
            p = ch - 1
            yb = ybufs[p % 2]
            pltpu.sync_copy(
                x2_hbm.at[pl.ds(pl.multiple_of(t0 + p * CH6, CH6), CH6)], xv)
            cps[p % 2].wait()

            def body(j, carry):
                sl = pl.ds(j * L, L)
                for tt in range(CH6):
                    ov[tt, sl] = (xv[tt, sl] + yb[2 * tt, sl]
                                  + yb[2 * tt + 1, sl])
                return carry

            lax.fori_loop(0, D // L, body, jnp.int32(0))
            pltpu.sync_copy(
                ov, out_hbm.at[pl.ds(pl.multiple_of(t0 + p * CH6, CH6), CH6)])


def _k6(y, pos, x2):
    mesh = plsc.VectorSubcoreMesh(core_axis_name="c", subcore_axis_name="s")
    kfn = pl.kernel(
        _k6_kernel,
        mesh=mesh,
        out_type=jax.ShapeDtypeStruct((S, D), jnp.float32),
        compiler_params=pltpu.CompilerParams(needs_layout_passes=False),
        scratch_types=[
            pltpu.VMEM((TPT * TK,), jnp.int32),
            pltpu.VMEM((E, TPT * TK), jnp.int32),
            pltpu.VMEM((L, D), jnp.float32),
            pltpu.VMEM((L, D), jnp.float32),
            pltpu.VMEM((CH6, D), jnp.float32),
            pltpu.VMEM((CH6, D), jnp.float32),
            pltpu.SemaphoreType.DMA,
            pltpu.SemaphoreType.DMA,
        ],
    )
    return kfn(y, pos, x2)


def kernel(hidden_states, start_pos, position_embeddings, attention_mask,
           wq, wk, wv, wo, q_norm_w, k_norm_w, ln1_w, ln2_w,
           gate_w, w_gate, w_up, w_down):
    x = hidden_states.reshape(S, D)
    cos = position_embeddings[0]
    sin = position_embeddings[1]
    wqkv = ln1_w[:, None] * jnp.concatenate([wq, wk, wv], axis=1)
    qkv = _k1(x, wqkv)
    o = _k2(qkv, cos, sin, q_norm_w.reshape(1, DH), k_norm_w.reshape(1, DH))
    x2, h2, ti, tw, counts = _k3(x, o, wo, ln2_w.reshape(1, D), gate_w)
    hd, ws, pos, bexp = _k4(ti.reshape(S * TK), tw.reshape(S * TK),
                            h2, counts)
    y = _k5(hd, ws, bexp, w_gate, w_up, w_down)
    out = _k6(y, pos, x2)
    return out.reshape(B, S, D)
